# Initial kernel scaffold; baseline (speedup 1.0000x reference)
import jax
import jax.numpy as jnp
from jax.experimental import pallas as pl


def kernel(x, edge_index, batch, pos_index, pos_enc, pos_batch, zinit_W, zemb_bn1_g, zemb_bn1_bt, zemb_lin_W, zemb_lin_b, zemb_bn2_g, zemb_bn2_bt, xemb_l1_W, xemb_l1_b, xemb_bn1_g, xemb_bn1_bt, xemb_l2_W, xemb_l2_b, xemb_bn2_g, xemb_bn2_bt, c0_le_W, c0_le_b, c0_eps, c0_l1_W, c0_l1_b, c0_bn1_g, c0_bn1_bt, c0_l2_W, c0_l2_b, c0_bn2_g, c0_bn2_bt, c1_le_W, c1_le_b, c1_eps, c1_l1_W, c1_l1_b, c1_bn1_g, c1_bn1_bt, c1_l2_W, c1_l2_b, c1_bn2_g, c1_bn2_bt, c2_le_W, c2_le_b, c2_eps, c2_l1_W, c2_l1_b, c2_bn1_g, c2_bn1_bt, c2_l2_W, c2_l2_b, c2_bn2_g, c2_bn2_bt, lin1_W, lin1_b, bn_lin1_g, bn_lin1_bt, lin2_W, lin2_b):
    raise NotImplementedError("write your pallas kernel here")


# speedup vs baseline: 1.1086x; 1.1086x over previous
import functools

import jax
import jax.numpy as jnp
from jax.experimental import pallas as pl
from jax.experimental.pallas import tpu as pltpu

N = 10000
E = 320000
P = 640000
HID = 64
NL = 3
NC = 10
NG = 256
ZIN = 1800
IN_DIM = 10

BE = 6400
GRID_E = E // BE


def _stats_body(x_ref, o_ref):
    i = pl.program_id(0)

    @pl.when(i == 0)
    def _():
        o_ref[...] = jnp.zeros_like(o_ref)

    blk = x_ref[...]
    s1 = jnp.sum(blk, axis=0, keepdims=True)
    s2 = jnp.sum(blk * blk, axis=0, keepdims=True)
    o_ref[0:1, :] += s1
    o_ref[1:2, :] += s2


def _col_stats(x, bd):
    d = x.shape[1]
    return pl.pallas_call(
        _stats_body,
        grid=(x.shape[0] // bd,),
        in_specs=[pl.BlockSpec((bd, d), lambda i: (i, 0))],
        out_specs=pl.BlockSpec((8, d), lambda i: (0, 0)),
        out_shape=jax.ShapeDtypeStruct((8, d), jnp.float32),
    )(x)


def _bn_coeffs(stats, n, g, b):
    m = stats[0] / n
    v = stats[1] / n - m * m
    s = g / jnp.sqrt(v + 1e-5)
    t = b - m * s
    return s[None, :], t[None, :]


def _ztrans_body(z0_ref, s1_ref, t1_ref, w_ref, b_ref, zp_ref, st_ref):
    i = pl.program_id(0)

    @pl.when(i == 0)
    def _():
        st_ref[...] = jnp.zeros_like(st_ref)

    z1 = jnp.maximum(z0_ref[...] * s1_ref[...] + t1_ref[...], 0.0)
    zp = jnp.dot(z1, w_ref[...], preferred_element_type=jnp.float32) + b_ref[...]
    zp_ref[...] = zp
    st_ref[0:1, :] += jnp.sum(zp, axis=0, keepdims=True)
    st_ref[1:2, :] += jnp.sum(zp * zp, axis=0, keepdims=True)


def _ztrans(z0, s1, t1, w, b):
    return pl.pallas_call(
        _ztrans_body,
        grid=(GRID_E,),
        in_specs=[
            pl.BlockSpec((BE, HID), lambda i: (i, 0)),
            pl.BlockSpec((1, HID), lambda i: (0, 0)),
            pl.BlockSpec((1, HID), lambda i: (0, 0)),
            pl.BlockSpec((HID, HID), lambda i: (0, 0)),
            pl.BlockSpec((1, HID), lambda i: (0, 0)),
        ],
        out_specs=[
            pl.BlockSpec((BE, HID), lambda i: (i, 0)),
            pl.BlockSpec((8, HID), lambda i: (0, 0)),
        ],
        out_shape=[
            jax.ShapeDtypeStruct((E, HID), jnp.float32),
            jax.ShapeDtypeStruct((8, HID), jnp.float32),
        ],
    )(z0, s1, t1, w, b)


def _msg_body(zp_ref, hs_ref, s2_ref, t2_ref, wle_ref, ble_ref, m_ref):
    z2 = jnp.maximum(zp_ref[...] * s2_ref[...] + t2_ref[...], 0.0)
    e = jnp.dot(z2, wle_ref[...], preferred_element_type=jnp.float32) + ble_ref[...]
    m_ref[...] = jnp.maximum(hs_ref[...] + e, 0.0)


def _msg(z2pre, hs, s2, t2, wle, ble):
    din = wle.shape[1]
    return pl.pallas_call(
        _msg_body,
        grid=(GRID_E,),
        in_specs=[
            pl.BlockSpec((BE, HID), lambda i: (i, 0)),
            pl.BlockSpec((BE, din), lambda i: (i, 0)),
            pl.BlockSpec((1, HID), lambda i: (0, 0)),
            pl.BlockSpec((1, HID), lambda i: (0, 0)),
            pl.BlockSpec((HID, din), lambda i: (0, 0)),
            pl.BlockSpec((1, din), lambda i: (0, 0)),
        ],
        out_specs=pl.BlockSpec((BE, din), lambda i: (i, 0)),
        out_shape=jax.ShapeDtypeStruct((E, din), jnp.float32),
    )(z2pre, hs, s2, t2, wle, ble)


def _bn_full(x, g, b):
    m = jnp.mean(x, axis=0)
    v = jnp.var(x, axis=0)
    return (x - m) / jnp.sqrt(v + 1e-5) * g + b


def _mlp(h, p, pre):
    h = h @ p[pre + "_l1_W"] + p[pre + "_l1_b"]
    h = _bn_full(h, p[pre + "_bn1_g"], p[pre + "_bn1_bt"])
    h = jax.nn.relu(h)
    h = h @ p[pre + "_l2_W"] + p[pre + "_l2_b"]
    h = _bn_full(h, p[pre + "_bn2_g"], p[pre + "_bn2_bt"])
    return jax.nn.relu(h)


def kernel(x, edge_index, batch, pos_index, pos_enc, pos_batch, zinit_W, zemb_bn1_g, zemb_bn1_bt, zemb_lin_W, zemb_lin_b, zemb_bn2_g, zemb_bn2_bt, xemb_l1_W, xemb_l1_b, xemb_bn1_g, xemb_bn1_bt, xemb_l2_W, xemb_l2_b, xemb_bn2_g, xemb_bn2_bt, c0_le_W, c0_le_b, c0_eps, c0_l1_W, c0_l1_b, c0_bn1_g, c0_bn1_bt, c0_l2_W, c0_l2_b, c0_bn2_g, c0_bn2_bt, c1_le_W, c1_le_b, c1_eps, c1_l1_W, c1_l1_b, c1_bn1_g, c1_bn1_bt, c1_l2_W, c1_l2_b, c1_bn2_g, c1_bn2_bt, c2_le_W, c2_le_b, c2_eps, c2_l1_W, c2_l1_b, c2_bn1_g, c2_bn1_bt, c2_l2_W, c2_l2_b, c2_bn2_g, c2_bn2_bt, lin1_W, lin1_b, bn_lin1_g, bn_lin1_bt, lin2_W, lin2_b):
    p = dict(locals())
    src = edge_index[0]
    dst = edge_index[1]

    z0 = jax.ops.segment_sum(zinit_W[pos_index] * pos_enc[:, None], pos_batch,
                             num_segments=E)

    st1 = _col_stats(z0, BE)
    s1, t1 = _bn_coeffs(st1, float(E), zemb_bn1_g, zemb_bn1_bt)
    z2pre, st2 = _ztrans(z0, s1, t1, zemb_lin_W, zemb_lin_b[None, :])
    s2, t2 = _bn_coeffs(st2, float(E), zemb_bn2_g, zemb_bn2_bt)

    h = x
    xs = [_mlp(x, p, "xemb")]

    for i in range(NL):
        pre = "c%d" % i
        hs = h[src]
        m = _msg(z2pre, hs, s2, t2, p[pre + "_le_W"], p[pre + "_le_b"][None, :])
        agg = jax.ops.segment_sum(m, dst, num_segments=N)
        h = agg + (1.0 + p[pre + "_eps"]) * h
        h = _mlp(h, p, pre)
        xs.append(h)

    hc = jnp.concatenate(xs, axis=1)
    ones = jnp.ones((N,), dtype=jnp.float32)
    cnt = jax.ops.segment_sum(ones, batch, num_segments=NG)
    pooled = jax.ops.segment_sum(hc, batch, num_segments=NG) / jnp.maximum(cnt, 1.0)[:, None]
    o = pooled @ lin1_W + lin1_b
    o = _bn_full(o, bn_lin1_g, bn_lin1_bt)
    o = jax.nn.relu(o)
    o = o @ lin2_W + lin2_b
    return jax.nn.log_softmax(o, axis=-1)


# speedup vs baseline: 1.8470x; 1.6660x over previous
import dataclasses
import functools

import jax
import jax.numpy as jnp
from jax import lax
from jax.experimental import pallas as pl
from jax.experimental.pallas import tpu as pltpu
from jax.experimental.pallas import tpu_sc as plsc

N = 10000
E = 320000
P = 640000
HID = 64
NL = 3
NC = 10
NG = 256
ZIN = 1800
IN_DIM = 10

BE = 6400
GRID_E = E // BE

NWORK = 32
ES = E // NWORK
ZCH = 400
NZCH = ES // ZCH
ZK = 256


def _sc_compiler_params():
    cp = pltpu.CompilerParams()
    if "needs_layout_passes" in pltpu.CompilerParams.__dataclass_fields__:
        cp = dataclasses.replace(cp, needs_layout_passes=False)
    return cp


def _z0_body(zinit_hbm, pi_hbm, pe_hbm, pb_hbm, btab_hbm, z0_hbm,
             bvec, idxv, encv, segv, gbuf, outbuf, sem):
    wid = lax.axis_index("s") * 2 + lax.axis_index("c")
    pltpu.sync_copy(btab_hbm.at[wid], bvec)
    iota16 = lax.iota(jnp.int32, 16)
    bv0 = bvec[pl.ds(0, 16)]
    bv1 = bvec[pl.ds(16, 16)]
    zeros16 = jnp.zeros((16,), jnp.float32)

    def bound(j):
        half = bv0 if j < 16 else bv1
        return lax.reduce_max(jnp.where(iota16 == (j % 16), half, -1), (0,))

    for c in range(NZCH):
        lo = bound(c)
        hi = bound(c + 1)
        lo_al = lax.bitwise_and(lo, jnp.int32(~7))
        nblk = (hi - lo_al + (ZK - 1)) // ZK
        lo16 = jnp.full((16,), lo, jnp.int32)
        hi16 = jnp.full((16,), hi, jnp.int32)
        segbase16 = jnp.full((16,), wid * ES + c * ZCH, jnp.int32)

        @pl.loop(0, ZCH)
        def _(r):
            r16 = jnp.full((16,), r, jnp.int32)
            for j in range(4):
                plsc.store_scatter(outbuf, [r16, iota16 + 16 * j], zeros16)

        @pl.loop(0, nblk)
        def _(b):
            base = pl.multiple_of(lo_al + b * ZK, 8)
            pltpu.sync_copy(pi_hbm.at[pl.ds(base, ZK)], idxv)
            pltpu.sync_copy(pe_hbm.at[pl.ds(base, ZK)], encv)
            pltpu.sync_copy(pb_hbm.at[pl.ds(base, ZK)], segv)
            pltpu.async_copy(zinit_hbm.at[idxv], gbuf, sem).wait()
            base16 = jnp.full((16,), base, jnp.int32)

            @pl.loop(0, ZK)
            def _(k):
                k16 = jnp.full((16,), k, jnp.int32)
                seg16 = plsc.load_gather(segv, [k16])
                enc16 = plsc.load_gather(encv, [k16])
                gp16 = base16 + k16
                mask = (gp16 >= lo16) & (gp16 < hi16)
                enc_eff = jnp.where(mask, enc16, 0.0)
                row16 = jnp.where(mask, seg16 - segbase16, 0)
                for j in range(4):
                    col = iota16 + 16 * j
                    vals = plsc.load_gather(gbuf, [k16, col])
                    plsc.addupdate_scatter(outbuf, [row16, col], vals * enc_eff)

        pltpu.sync_copy(outbuf, z0_hbm.at[pl.ds(wid * ES + c * ZCH, ZCH)])


@jax.jit
def _z0_sc(zinit_W, pos_index, pos_enc, pos_batch):
    bnd = jnp.searchsorted(pos_batch, jnp.arange(0, E + 1, ZCH)).astype(jnp.int32)
    bnd = jnp.pad(bnd, (0, 32))
    rows = jnp.arange(NWORK)[:, None] * NZCH + jnp.arange(32)[None, :]
    btab = bnd[rows]
    zpad = jnp.pad(zinit_W, ((0, 0), (0, 64)))
    pi = jnp.pad(pos_index.astype(jnp.int32), (0, ZK))
    pe = jnp.pad(pos_enc, (0, ZK))
    pb = jnp.pad(pos_batch.astype(jnp.int32), (0, ZK))

    mesh = plsc.VectorSubcoreMesh(core_axis_name="c", subcore_axis_name="s",
                                  num_cores=2, num_subcores=16)
    kern = pl.kernel(
        _z0_body,
        out_type=jax.ShapeDtypeStruct((E, 128), jnp.float32),
        mesh=mesh,
        scratch_types=[
            pltpu.VMEM((32,), jnp.int32),
            pltpu.VMEM((ZK,), jnp.int32),
            pltpu.VMEM((ZK,), jnp.float32),
            pltpu.VMEM((ZK,), jnp.int32),
            pltpu.VMEM((ZK, 128), jnp.float32),
            pltpu.VMEM((ZCH, 128), jnp.float32),
            pltpu.SemaphoreType.DMA,
        ],
        compiler_params=_sc_compiler_params(),
    )
    return kern(zpad, pi, pe, pb, btab)


def _stats_body(x_ref, o_ref):
    i = pl.program_id(0)

    @pl.when(i == 0)
    def _():
        o_ref[...] = jnp.zeros_like(o_ref)

    blk = x_ref[...][:, :HID]
    s1 = jnp.sum(blk, axis=0, keepdims=True)
    s2 = jnp.sum(blk * blk, axis=0, keepdims=True)
    o_ref[0:1, :] += s1
    o_ref[1:2, :] += s2


def _col_stats(x, bd):
    d = x.shape[1]
    return pl.pallas_call(
        _stats_body,
        grid=(x.shape[0] // bd,),
        in_specs=[pl.BlockSpec((bd, d), lambda i: (i, 0))],
        out_specs=pl.BlockSpec((8, HID), lambda i: (0, 0)),
        out_shape=jax.ShapeDtypeStruct((8, HID), jnp.float32),
    )(x)


def _bn_coeffs(stats, n, g, b):
    m = stats[0] / n
    v = stats[1] / n - m * m
    s = g / jnp.sqrt(v + 1e-5)
    t = b - m * s
    return s[None, :], t[None, :]


def _ztrans_body(z0_ref, s1_ref, t1_ref, w_ref, b_ref, zp_ref, st_ref):
    i = pl.program_id(0)

    @pl.when(i == 0)
    def _():
        st_ref[...] = jnp.zeros_like(st_ref)

    z1 = jnp.maximum(z0_ref[...][:, :HID] * s1_ref[...] + t1_ref[...], 0.0)
    zp = jnp.dot(z1, w_ref[...], preferred_element_type=jnp.float32) + b_ref[...]
    zp_ref[...] = zp
    st_ref[0:1, :] += jnp.sum(zp, axis=0, keepdims=True)
    st_ref[1:2, :] += jnp.sum(zp * zp, axis=0, keepdims=True)


def _ztrans(z0, s1, t1, w, b):
    return pl.pallas_call(
        _ztrans_body,
        grid=(GRID_E,),
        in_specs=[
            pl.BlockSpec((BE, 128), lambda i: (i, 0)),
            pl.BlockSpec((1, HID), lambda i: (0, 0)),
            pl.BlockSpec((1, HID), lambda i: (0, 0)),
            pl.BlockSpec((HID, HID), lambda i: (0, 0)),
            pl.BlockSpec((1, HID), lambda i: (0, 0)),
        ],
        out_specs=[
            pl.BlockSpec((BE, HID), lambda i: (i, 0)),
            pl.BlockSpec((8, HID), lambda i: (0, 0)),
        ],
        out_shape=[
            jax.ShapeDtypeStruct((E, HID), jnp.float32),
            jax.ShapeDtypeStruct((8, HID), jnp.float32),
        ],
    )(z0, s1, t1, w, b)


def _msg_body(zp_ref, hs_ref, s2_ref, t2_ref, wle_ref, ble_ref, m_ref):
    z2 = jnp.maximum(zp_ref[...] * s2_ref[...] + t2_ref[...], 0.0)
    e = jnp.dot(z2, wle_ref[...], preferred_element_type=jnp.float32) + ble_ref[...]
    m_ref[...] = jnp.maximum(hs_ref[...] + e, 0.0)


def _msg(z2pre, hs, s2, t2, wle, ble):
    din = wle.shape[1]
    return pl.pallas_call(
        _msg_body,
        grid=(GRID_E,),
        in_specs=[
            pl.BlockSpec((BE, HID), lambda i: (i, 0)),
            pl.BlockSpec((BE, din), lambda i: (i, 0)),
            pl.BlockSpec((1, HID), lambda i: (0, 0)),
            pl.BlockSpec((1, HID), lambda i: (0, 0)),
            pl.BlockSpec((HID, din), lambda i: (0, 0)),
            pl.BlockSpec((1, din), lambda i: (0, 0)),
        ],
        out_specs=pl.BlockSpec((BE, din), lambda i: (i, 0)),
        out_shape=jax.ShapeDtypeStruct((E, din), jnp.float32),
    )(z2pre, hs, s2, t2, wle, ble)


def _bn_full(x, g, b):
    m = jnp.mean(x, axis=0)
    v = jnp.var(x, axis=0)
    return (x - m) / jnp.sqrt(v + 1e-5) * g + b


def _mlp(h, p, pre):
    h = h @ p[pre + "_l1_W"] + p[pre + "_l1_b"]
    h = _bn_full(h, p[pre + "_bn1_g"], p[pre + "_bn1_bt"])
    h = jax.nn.relu(h)
    h = h @ p[pre + "_l2_W"] + p[pre + "_l2_b"]
    h = _bn_full(h, p[pre + "_bn2_g"], p[pre + "_bn2_bt"])
    return jax.nn.relu(h)


def kernel(x, edge_index, batch, pos_index, pos_enc, pos_batch, zinit_W, zemb_bn1_g, zemb_bn1_bt, zemb_lin_W, zemb_lin_b, zemb_bn2_g, zemb_bn2_bt, xemb_l1_W, xemb_l1_b, xemb_bn1_g, xemb_bn1_bt, xemb_l2_W, xemb_l2_b, xemb_bn2_g, xemb_bn2_bt, c0_le_W, c0_le_b, c0_eps, c0_l1_W, c0_l1_b, c0_bn1_g, c0_bn1_bt, c0_l2_W, c0_l2_b, c0_bn2_g, c0_bn2_bt, c1_le_W, c1_le_b, c1_eps, c1_l1_W, c1_l1_b, c1_bn1_g, c1_bn1_bt, c1_l2_W, c1_l2_b, c1_bn2_g, c1_bn2_bt, c2_le_W, c2_le_b, c2_eps, c2_l1_W, c2_l1_b, c2_bn1_g, c2_bn1_bt, c2_l2_W, c2_l2_b, c2_bn2_g, c2_bn2_bt, lin1_W, lin1_b, bn_lin1_g, bn_lin1_bt, lin2_W, lin2_b):
    p = dict(locals())
    src = edge_index[0]
    dst = edge_index[1]

    z0 = _z0_sc(zinit_W, pos_index, pos_enc, pos_batch)

    st1 = _col_stats(z0, BE)
    s1, t1 = _bn_coeffs(st1, float(E), zemb_bn1_g, zemb_bn1_bt)
    z2pre, st2 = _ztrans(z0, s1, t1, zemb_lin_W, zemb_lin_b[None, :])
    s2, t2 = _bn_coeffs(st2, float(E), zemb_bn2_g, zemb_bn2_bt)

    h = x
    xs = [_mlp(x, p, "xemb")]

    for i in range(NL):
        pre = "c%d" % i
        hs = h[src]
        m = _msg(z2pre, hs, s2, t2, p[pre + "_le_W"], p[pre + "_le_b"][None, :])
        agg = jax.ops.segment_sum(m, dst, num_segments=N)
        h = agg + (1.0 + p[pre + "_eps"]) * h
        h = _mlp(h, p, pre)
        xs.append(h)

    hc = jnp.concatenate(xs, axis=1)
    ones = jnp.ones((N,), dtype=jnp.float32)
    cnt = jax.ops.segment_sum(ones, batch, num_segments=NG)
    pooled = jax.ops.segment_sum(hc, batch, num_segments=NG) / jnp.maximum(cnt, 1.0)[:, None]
    o = pooled @ lin1_W + lin1_b
    o = _bn_full(o, bn_lin1_g, bn_lin1_bt)
    o = jax.nn.relu(o)
    o = o @ lin2_W + lin2_b
    return jax.nn.log_softmax(o, axis=-1)


# speedup vs baseline: 2.2625x; 1.2250x over previous
import dataclasses
import functools

import jax
import jax.numpy as jnp
from jax import lax
from jax.experimental import pallas as pl
from jax.experimental.pallas import tpu as pltpu
from jax.experimental.pallas import tpu_sc as plsc

N = 10000
E = 320000
P = 640000
HID = 64
NL = 3
NC = 10
NG = 256
ZIN = 1800
IN_DIM = 10

BE = 6400
GRID_E = E // BE

NWORK = 32
ES = E // NWORK
ZCH = 400
NZCH = ES // ZCH
ZK = 256


def _sc_compiler_params():
    cp = pltpu.CompilerParams()
    if "needs_layout_passes" in pltpu.CompilerParams.__dataclass_fields__:
        cp = dataclasses.replace(cp, needs_layout_passes=False)
    return cp


def _z0_body(zinit_hbm, pi_hbm, pe_hbm, pb_hbm, btab_hbm, z0_hbm,
             bvec, idxv, encv, segv, gbuf, outbuf, sem):
    wid = lax.axis_index("s") * 2 + lax.axis_index("c")
    pltpu.sync_copy(btab_hbm.at[wid], bvec)
    iota16 = lax.iota(jnp.int32, 16)
    bv0 = bvec[pl.ds(0, 16)]
    bv1 = bvec[pl.ds(16, 16)]
    zeros16 = jnp.zeros((16,), jnp.float32)

    def bound(j):
        half = bv0 if j < 16 else bv1
        return lax.reduce_max(jnp.where(iota16 == (j % 16), half, -1), (0,))

    for c in range(NZCH):
        lo = bound(c)
        hi = bound(c + 1)
        lo_al = lax.bitwise_and(lo, jnp.int32(~7))
        nblk = (hi - lo_al + (ZK - 1)) // ZK
        lo16 = jnp.full((16,), lo, jnp.int32)
        hi16 = jnp.full((16,), hi, jnp.int32)
        segbase16 = jnp.full((16,), wid * ES + c * ZCH, jnp.int32)

        @pl.loop(0, ZCH)
        def _(r):
            r16 = jnp.full((16,), r, jnp.int32)
            for j in range(4):
                plsc.store_scatter(outbuf, [r16, iota16 + 16 * j], zeros16)

        @pl.loop(0, nblk)
        def _(b):
            base = pl.multiple_of(lo_al + b * ZK, 8)
            pltpu.sync_copy(pi_hbm.at[pl.ds(base, ZK)], idxv)
            pltpu.sync_copy(pe_hbm.at[pl.ds(base, ZK)], encv)
            pltpu.sync_copy(pb_hbm.at[pl.ds(base, ZK)], segv)
            pltpu.async_copy(zinit_hbm.at[idxv], gbuf, sem).wait()
            base16 = jnp.full((16,), base, jnp.int32)

            @pl.loop(0, ZK)
            def _(k):
                k16 = jnp.full((16,), k, jnp.int32)
                seg16 = plsc.load_gather(segv, [k16])
                enc16 = plsc.load_gather(encv, [k16])
                gp16 = base16 + k16
                mask = (gp16 >= lo16) & (gp16 < hi16)
                enc_eff = jnp.where(mask, enc16, 0.0)
                row16 = jnp.where(mask, seg16 - segbase16, 0)
                for j in range(4):
                    col = iota16 + 16 * j
                    vals = plsc.load_gather(gbuf, [k16, col])
                    plsc.addupdate_scatter(outbuf, [row16, col], vals * enc_eff)

        pltpu.sync_copy(outbuf, z0_hbm.at[pl.ds(wid * ES + c * ZCH, ZCH)])


@jax.jit
def _z0_sc(zinit_W, pos_index, pos_enc, pos_batch):
    bnd = jnp.searchsorted(pos_batch, jnp.arange(0, E + 1, ZCH)).astype(jnp.int32)
    bnd = jnp.pad(bnd, (0, 32))
    rows = jnp.arange(NWORK)[:, None] * NZCH + jnp.arange(32)[None, :]
    btab = bnd[rows]
    zpad = jnp.pad(zinit_W, ((0, 0), (0, 64)))
    pi = jnp.pad(pos_index.astype(jnp.int32), (0, ZK))
    pe = jnp.pad(pos_enc, (0, ZK))
    pb = jnp.pad(pos_batch.astype(jnp.int32), (0, ZK))

    mesh = plsc.VectorSubcoreMesh(core_axis_name="c", subcore_axis_name="s",
                                  num_cores=2, num_subcores=16)
    kern = pl.kernel(
        _z0_body,
        out_type=jax.ShapeDtypeStruct((E, 128), jnp.float32),
        mesh=mesh,
        scratch_types=[
            pltpu.VMEM((32,), jnp.int32),
            pltpu.VMEM((ZK,), jnp.int32),
            pltpu.VMEM((ZK,), jnp.float32),
            pltpu.VMEM((ZK,), jnp.int32),
            pltpu.VMEM((ZK, 128), jnp.float32),
            pltpu.VMEM((ZCH, 128), jnp.float32),
            pltpu.SemaphoreType.DMA,
        ],
        compiler_params=_sc_compiler_params(),
    )
    return kern(zpad, pi, pe, pb, btab)


NOWN = 5120
NACC = 5184
AK = 400
EPT2 = E // 16
NAB2 = EPT2 // AK


def _agg_body(m_hbm, dst_hbm, zeros_hbm, out_hbm, idxv, idxv2, mbuf, accum, sem):
    cid = lax.axis_index("c")
    sid = lax.axis_index("s")

    pltpu.sync_copy(zeros_hbm.at[pl.ds(0, 320)], accum.at[pl.ds(sid * 320, 320)])

    @pl.when(sid == 0)
    def _():
        pltpu.sync_copy(zeros_hbm.at[pl.ds(0, 64)], accum.at[pl.ds(NOWN, 64)])

    plsc.subcore_barrier()

    base_n16 = jnp.full((16,), cid * NOWN, jnp.int32)
    own16 = jnp.full((16,), NOWN, jnp.int32)
    t63 = jnp.full((16,), 63, jnp.int32)

    @pl.loop(0, NAB2)
    def _(b):
        base = pl.multiple_of(sid * EPT2 + b * AK, 8)
        pltpu.sync_copy(dst_hbm.at[pl.ds(base, AK)], idxv)
        pltpu.sync_copy(m_hbm.at[pl.ds(base, AK)], mbuf)
        for g in range(AK // 16):
            dv = idxv[pl.ds(g * 16, 16)]
            local = dv - base_n16
            owned = (local >= 0) & (local < own16)
            trash = own16 + (dv & t63)
            idxv2[pl.ds(g * 16, 16)] = jnp.where(owned, local, trash)
        pltpu.sync_copy(mbuf, accum.at[idxv2], add=True)

    plsc.subcore_barrier()
    pltpu.sync_copy(accum.at[pl.ds(sid * 320, 320)],
                    out_hbm.at[cid].at[pl.ds(sid * 320, 320)])


@jax.jit
def _agg_sc(m, dst):
    zeros = jnp.zeros((320, 128), jnp.float32)
    mesh = plsc.VectorSubcoreMesh(core_axis_name="c", subcore_axis_name="s",
                                  num_cores=2, num_subcores=16)
    kern = pl.kernel(
        _agg_body,
        out_type=jax.ShapeDtypeStruct((2, NOWN, 128), jnp.float32),
        mesh=mesh,
        scratch_types=[
            pltpu.VMEM((AK,), jnp.int32),
            pltpu.VMEM((AK,), jnp.int32),
            pltpu.VMEM((AK, 128), jnp.float32),
            pltpu.VMEM_SHARED((NACC, 128), jnp.float32),
            pltpu.SemaphoreType.DMA,
        ],
        compiler_params=_sc_compiler_params(),
    )
    return kern(m, dst.astype(jnp.int32), zeros)


def _stats_body(x_ref, o_ref):
    i = pl.program_id(0)

    @pl.when(i == 0)
    def _():
        o_ref[...] = jnp.zeros_like(o_ref)

    blk = x_ref[...][:, :HID]
    s1 = jnp.sum(blk, axis=0, keepdims=True)
    s2 = jnp.sum(blk * blk, axis=0, keepdims=True)
    o_ref[0:1, :] += s1
    o_ref[1:2, :] += s2


def _col_stats(x, bd):
    d = x.shape[1]
    return pl.pallas_call(
        _stats_body,
        grid=(x.shape[0] // bd,),
        in_specs=[pl.BlockSpec((bd, d), lambda i: (i, 0))],
        out_specs=pl.BlockSpec((8, HID), lambda i: (0, 0)),
        out_shape=jax.ShapeDtypeStruct((8, HID), jnp.float32),
    )(x)


def _bn_coeffs(stats, n, g, b):
    m = stats[0] / n
    v = stats[1] / n - m * m
    s = g / jnp.sqrt(v + 1e-5)
    t = b - m * s
    return s[None, :], t[None, :]


def _ztrans_body(z0_ref, s1_ref, t1_ref, w_ref, b_ref, zp_ref, st_ref):
    i = pl.program_id(0)

    @pl.when(i == 0)
    def _():
        st_ref[...] = jnp.zeros_like(st_ref)

    z1 = jnp.maximum(z0_ref[...][:, :HID] * s1_ref[...] + t1_ref[...], 0.0)
    zp = jnp.dot(z1, w_ref[...], preferred_element_type=jnp.float32) + b_ref[...]
    zp_ref[...] = zp
    st_ref[0:1, :] += jnp.sum(zp, axis=0, keepdims=True)
    st_ref[1:2, :] += jnp.sum(zp * zp, axis=0, keepdims=True)


def _ztrans(z0, s1, t1, w, b):
    return pl.pallas_call(
        _ztrans_body,
        grid=(GRID_E,),
        in_specs=[
            pl.BlockSpec((BE, 128), lambda i: (i, 0)),
            pl.BlockSpec((1, HID), lambda i: (0, 0)),
            pl.BlockSpec((1, HID), lambda i: (0, 0)),
            pl.BlockSpec((HID, HID), lambda i: (0, 0)),
            pl.BlockSpec((1, HID), lambda i: (0, 0)),
        ],
        out_specs=[
            pl.BlockSpec((BE, HID), lambda i: (i, 0)),
            pl.BlockSpec((8, HID), lambda i: (0, 0)),
        ],
        out_shape=[
            jax.ShapeDtypeStruct((E, HID), jnp.float32),
            jax.ShapeDtypeStruct((8, HID), jnp.float32),
        ],
    )(z0, s1, t1, w, b)


def _msg_body(zp_ref, hs_ref, s2_ref, t2_ref, wle_ref, ble_ref, m_ref):
    z2 = jnp.maximum(zp_ref[...] * s2_ref[...] + t2_ref[...], 0.0)
    e = jnp.dot(z2, wle_ref[...], preferred_element_type=jnp.float32) + ble_ref[...]
    m = jnp.maximum(hs_ref[...] + e, 0.0)
    m_ref[...] = jnp.concatenate([m, jnp.zeros_like(m)], axis=1)


def _msg(z2pre, hs, s2, t2, wle, ble):
    return pl.pallas_call(
        _msg_body,
        grid=(GRID_E,),
        in_specs=[
            pl.BlockSpec((BE, HID), lambda i: (i, 0)),
            pl.BlockSpec((BE, HID), lambda i: (i, 0)),
            pl.BlockSpec((1, HID), lambda i: (0, 0)),
            pl.BlockSpec((1, HID), lambda i: (0, 0)),
            pl.BlockSpec((HID, HID), lambda i: (0, 0)),
            pl.BlockSpec((1, HID), lambda i: (0, 0)),
        ],
        out_specs=pl.BlockSpec((BE, 128), lambda i: (i, 0)),
        out_shape=jax.ShapeDtypeStruct((E, 128), jnp.float32),
    )(z2pre, hs, s2, t2, wle, ble)


def _bn_full(x, g, b):
    m = jnp.mean(x, axis=0)
    v = jnp.var(x, axis=0)
    return (x - m) / jnp.sqrt(v + 1e-5) * g + b


def _mlp(h, p, pre):
    h = h @ p[pre + "_l1_W"] + p[pre + "_l1_b"]
    h = _bn_full(h, p[pre + "_bn1_g"], p[pre + "_bn1_bt"])
    h = jax.nn.relu(h)
    h = h @ p[pre + "_l2_W"] + p[pre + "_l2_b"]
    h = _bn_full(h, p[pre + "_bn2_g"], p[pre + "_bn2_bt"])
    return jax.nn.relu(h)


def kernel(x, edge_index, batch, pos_index, pos_enc, pos_batch, zinit_W, zemb_bn1_g, zemb_bn1_bt, zemb_lin_W, zemb_lin_b, zemb_bn2_g, zemb_bn2_bt, xemb_l1_W, xemb_l1_b, xemb_bn1_g, xemb_bn1_bt, xemb_l2_W, xemb_l2_b, xemb_bn2_g, xemb_bn2_bt, c0_le_W, c0_le_b, c0_eps, c0_l1_W, c0_l1_b, c0_bn1_g, c0_bn1_bt, c0_l2_W, c0_l2_b, c0_bn2_g, c0_bn2_bt, c1_le_W, c1_le_b, c1_eps, c1_l1_W, c1_l1_b, c1_bn1_g, c1_bn1_bt, c1_l2_W, c1_l2_b, c1_bn2_g, c1_bn2_bt, c2_le_W, c2_le_b, c2_eps, c2_l1_W, c2_l1_b, c2_bn1_g, c2_bn1_bt, c2_l2_W, c2_l2_b, c2_bn2_g, c2_bn2_bt, lin1_W, lin1_b, bn_lin1_g, bn_lin1_bt, lin2_W, lin2_b):
    p = dict(locals())
    src = edge_index[0]
    dst = edge_index[1]

    z0 = _z0_sc(zinit_W, pos_index, pos_enc, pos_batch)

    st1 = _col_stats(z0, BE)
    s1, t1 = _bn_coeffs(st1, float(E), zemb_bn1_g, zemb_bn1_bt)
    z2pre, st2 = _ztrans(z0, s1, t1, zemb_lin_W, zemb_lin_b[None, :])
    s2, t2 = _bn_coeffs(st2, float(E), zemb_bn2_g, zemb_bn2_bt)

    xs = [_mlp(x, p, "xemb")]

    h = jnp.pad(x, ((0, 0), (0, HID - IN_DIM)))
    p["c0_le_W"] = jnp.pad(c0_le_W, ((0, 0), (0, HID - IN_DIM)))
    p["c0_le_b"] = jnp.pad(c0_le_b, (0, HID - IN_DIM))
    p["c0_l1_W"] = jnp.pad(c0_l1_W, ((0, HID - IN_DIM), (0, 0)))
    for i in range(NL):
        pre = "c%d" % i
        hs = h[src]
        m = _msg(z2pre, hs, s2, t2, p[pre + "_le_W"], p[pre + "_le_b"][None, :])
        parts = _agg_sc(m, dst)
        agg = jnp.concatenate([parts[0], parts[1]], axis=0)[:N, :HID]
        h = agg + (1.0 + p[pre + "_eps"]) * h
        h = _mlp(h, p, pre)
        xs.append(h)

    hc = jnp.concatenate(xs, axis=1)
    ones = jnp.ones((N,), dtype=jnp.float32)
    cnt = jax.ops.segment_sum(ones, batch, num_segments=NG)
    pooled = jax.ops.segment_sum(hc, batch, num_segments=NG) / jnp.maximum(cnt, 1.0)[:, None]
    o = pooled @ lin1_W + lin1_b
    o = _bn_full(o, bn_lin1_g, bn_lin1_bt)
    o = jax.nn.relu(o)
    o = o @ lin2_W + lin2_b
    return jax.nn.log_softmax(o, axis=-1)


# speedup vs baseline: 2.9554x; 1.3062x over previous
import dataclasses
import functools

import jax
import jax.numpy as jnp
from jax import lax
from jax.experimental import pallas as pl
from jax.experimental.pallas import tpu as pltpu
from jax.experimental.pallas import tpu_sc as plsc

N = 10000
E = 320000
P = 640000
HID = 64
NL = 3
NC = 10
NG = 256
ZIN = 1800
IN_DIM = 10

BE = 6400
GRID_E = E // BE

NWORK = 32
ES = E // NWORK
ZCH = 400
NZCH = ES // ZCH
ZK = 256


def _sc_compiler_params():
    cp = pltpu.CompilerParams()
    if "needs_layout_passes" in pltpu.CompilerParams.__dataclass_fields__:
        cp = dataclasses.replace(cp, needs_layout_passes=False)
    return cp


def _z0_body(zinit_hbm, pi_hbm, pe_hbm, pb_hbm, btab_hbm, z0_hbm,
             bvec, idxv, encv, segv, gbuf, outbuf, sem):
    wid = lax.axis_index("s") * 2 + lax.axis_index("c")
    pltpu.sync_copy(btab_hbm.at[wid], bvec)
    iota16 = lax.iota(jnp.int32, 16)
    bv0 = bvec[pl.ds(0, 16)]
    bv1 = bvec[pl.ds(16, 16)]
    zeros16 = jnp.zeros((16,), jnp.float32)

    def bound(j):
        half = bv0 if j < 16 else bv1
        return lax.reduce_max(jnp.where(iota16 == (j % 16), half, -1), (0,))

    for c in range(NZCH):
        lo = bound(c)
        hi = bound(c + 1)
        lo_al = lax.bitwise_and(lo, jnp.int32(~7))
        nblk = (hi - lo_al + (ZK - 1)) // ZK
        lo16 = jnp.full((16,), lo, jnp.int32)
        hi16 = jnp.full((16,), hi, jnp.int32)
        segbase16 = jnp.full((16,), wid * ES + c * ZCH, jnp.int32)

        @pl.loop(0, ZCH)
        def _(r):
            r16 = jnp.full((16,), r, jnp.int32)
            for j in range(4):
                plsc.store_scatter(outbuf, [r16, iota16 + 16 * j], zeros16)

        @pl.loop(0, nblk)
        def _(b):
            base = pl.multiple_of(lo_al + b * ZK, 8)
            pltpu.sync_copy(pi_hbm.at[pl.ds(base, ZK)], idxv)
            pltpu.sync_copy(pe_hbm.at[pl.ds(base, ZK)], encv)
            pltpu.sync_copy(pb_hbm.at[pl.ds(base, ZK)], segv)
            pltpu.async_copy(zinit_hbm.at[idxv], gbuf, sem).wait()
            base16 = jnp.full((16,), base, jnp.int32)

            @pl.loop(0, ZK)
            def _(k):
                k16 = jnp.full((16,), k, jnp.int32)
                seg16 = plsc.load_gather(segv, [k16])
                enc16 = plsc.load_gather(encv, [k16])
                gp16 = base16 + k16
                mask = (gp16 >= lo16) & (gp16 < hi16)
                enc_eff = jnp.where(mask, enc16, 0.0)
                row16 = jnp.where(mask, seg16 - segbase16, 0)
                for j in range(4):
                    col = iota16 + 16 * j
                    vals = plsc.load_gather(gbuf, [k16, col])
                    plsc.addupdate_scatter(outbuf, [row16, col], vals * enc_eff)

        pltpu.sync_copy(outbuf, z0_hbm.at[pl.ds(wid * ES + c * ZCH, ZCH)])


@jax.jit
def _z0_sc(zinit_W, pos_index, pos_enc, pos_batch):
    bnd = jnp.searchsorted(pos_batch, jnp.arange(0, E + 1, ZCH)).astype(jnp.int32)
    bnd = jnp.pad(bnd, (0, 32))
    rows = jnp.arange(NWORK)[:, None] * NZCH + jnp.arange(32)[None, :]
    btab = bnd[rows]
    zpad = jnp.pad(zinit_W, ((0, 0), (0, 64)))
    pi = jnp.pad(pos_index.astype(jnp.int32), (0, ZK))
    pe = jnp.pad(pos_enc, (0, ZK))
    pb = jnp.pad(pos_batch.astype(jnp.int32), (0, ZK))

    mesh = plsc.VectorSubcoreMesh(core_axis_name="c", subcore_axis_name="s",
                                  num_cores=2, num_subcores=16)
    kern = pl.kernel(
        _z0_body,
        out_type=jax.ShapeDtypeStruct((E, 128), jnp.float32),
        mesh=mesh,
        scratch_types=[
            pltpu.VMEM((32,), jnp.int32),
            pltpu.VMEM((ZK,), jnp.int32),
            pltpu.VMEM((ZK,), jnp.float32),
            pltpu.VMEM((ZK,), jnp.int32),
            pltpu.VMEM((ZK, 128), jnp.float32),
            pltpu.VMEM((ZCH, 128), jnp.float32),
            pltpu.SemaphoreType.DMA,
        ],
        compiler_params=_sc_compiler_params(),
    )
    return kern(zpad, pi, pe, pb, btab)


NOWN = 5120
NACC = 5184
AK = 400
EPT2 = E // 16
NAB2 = EPT2 // AK


def _agg_body(m_hbm, dst_hbm, zeros_hbm, out_hbm, idxv, idxv2, mbuf, accum, sem):
    cid = lax.axis_index("c")
    sid = lax.axis_index("s")

    pltpu.sync_copy(zeros_hbm.at[pl.ds(0, 320)], accum.at[pl.ds(sid * 320, 320)])

    @pl.when(sid == 0)
    def _():
        pltpu.sync_copy(zeros_hbm.at[pl.ds(0, 64)], accum.at[pl.ds(NOWN, 64)])

    plsc.subcore_barrier()

    base_n16 = jnp.full((16,), cid * NOWN, jnp.int32)
    own16 = jnp.full((16,), NOWN, jnp.int32)
    t63 = jnp.full((16,), 63, jnp.int32)

    @pl.loop(0, NAB2)
    def _(b):
        base = pl.multiple_of(sid * EPT2 + b * AK, 8)
        pltpu.sync_copy(dst_hbm.at[pl.ds(base, AK)], idxv)
        pltpu.sync_copy(m_hbm.at[pl.ds(base, AK)], mbuf)
        for g in range(AK // 16):
            dv = idxv[pl.ds(g * 16, 16)]
            local = dv - base_n16
            owned = (local >= 0) & (local < own16)
            trash = own16 + (dv & t63)
            idxv2[pl.ds(g * 16, 16)] = jnp.where(owned, local, trash)
        pltpu.sync_copy(mbuf, accum.at[idxv2], add=True)

    plsc.subcore_barrier()
    pltpu.sync_copy(accum.at[pl.ds(sid * 320, 320)],
                    out_hbm.at[cid].at[pl.ds(sid * 320, 320)])


@jax.jit
def _agg_sc(m, dst):
    zeros = jnp.zeros((320, 128), jnp.float32)
    mesh = plsc.VectorSubcoreMesh(core_axis_name="c", subcore_axis_name="s",
                                  num_cores=2, num_subcores=16)
    kern = pl.kernel(
        _agg_body,
        out_type=jax.ShapeDtypeStruct((2, NOWN, 128), jnp.float32),
        mesh=mesh,
        scratch_types=[
            pltpu.VMEM((AK,), jnp.int32),
            pltpu.VMEM((AK,), jnp.int32),
            pltpu.VMEM((AK, 128), jnp.float32),
            pltpu.VMEM_SHARED((NACC, 128), jnp.float32),
            pltpu.SemaphoreType.DMA,
        ],
        compiler_params=_sc_compiler_params(),
    )
    return kern(m, dst.astype(jnp.int32), zeros)


GK = 400
NGB = (E // NWORK) // GK


def _gat_body(h_hbm, src_hbm, hs_hbm, idxv, gbuf, sem):
    wid = lax.axis_index("s") * 2 + lax.axis_index("c")

    @pl.loop(0, NGB)
    def _(b):
        base = pl.multiple_of(wid * (E // NWORK) + b * GK, 8)
        pltpu.sync_copy(src_hbm.at[pl.ds(base, GK)], idxv)
        pltpu.async_copy(h_hbm.at[idxv], gbuf, sem).wait()
        pltpu.sync_copy(gbuf, hs_hbm.at[pl.ds(base, GK)])


@jax.jit
def _gather_sc(h128, src):
    mesh = plsc.VectorSubcoreMesh(core_axis_name="c", subcore_axis_name="s",
                                  num_cores=2, num_subcores=16)
    kern = pl.kernel(
        _gat_body,
        out_type=jax.ShapeDtypeStruct((E, 128), jnp.float32),
        mesh=mesh,
        scratch_types=[
            pltpu.VMEM((GK,), jnp.int32),
            pltpu.VMEM((GK, 128), jnp.float32),
            pltpu.SemaphoreType.DMA,
        ],
        compiler_params=_sc_compiler_params(),
    )
    return kern(h128, src.astype(jnp.int32))


def _stats_body(x_ref, o_ref):
    i = pl.program_id(0)

    @pl.when(i == 0)
    def _():
        o_ref[...] = jnp.zeros_like(o_ref)

    blk = x_ref[...][:, :HID]
    s1 = jnp.sum(blk, axis=0, keepdims=True)
    s2 = jnp.sum(blk * blk, axis=0, keepdims=True)
    o_ref[0:1, :] += s1
    o_ref[1:2, :] += s2


def _col_stats(x, bd):
    d = x.shape[1]
    return pl.pallas_call(
        _stats_body,
        grid=(x.shape[0] // bd,),
        in_specs=[pl.BlockSpec((bd, d), lambda i: (i, 0))],
        out_specs=pl.BlockSpec((8, HID), lambda i: (0, 0)),
        out_shape=jax.ShapeDtypeStruct((8, HID), jnp.float32),
    )(x)


def _bn_coeffs(stats, n, g, b):
    m = stats[0] / n
    v = stats[1] / n - m * m
    s = g / jnp.sqrt(v + 1e-5)
    t = b - m * s
    return s[None, :], t[None, :]


def _ztrans_body(z0_ref, s1_ref, t1_ref, w_ref, b_ref, zp_ref, st_ref):
    i = pl.program_id(0)

    @pl.when(i == 0)
    def _():
        st_ref[...] = jnp.zeros_like(st_ref)

    z1 = jnp.maximum(z0_ref[...][:, :HID] * s1_ref[...] + t1_ref[...], 0.0)
    zp = jnp.dot(z1, w_ref[...], preferred_element_type=jnp.float32) + b_ref[...]
    zp_ref[...] = zp
    st_ref[0:1, :] += jnp.sum(zp, axis=0, keepdims=True)
    st_ref[1:2, :] += jnp.sum(zp * zp, axis=0, keepdims=True)


def _ztrans(z0, s1, t1, w, b):
    return pl.pallas_call(
        _ztrans_body,
        grid=(GRID_E,),
        in_specs=[
            pl.BlockSpec((BE, 128), lambda i: (i, 0)),
            pl.BlockSpec((1, HID), lambda i: (0, 0)),
            pl.BlockSpec((1, HID), lambda i: (0, 0)),
            pl.BlockSpec((HID, HID), lambda i: (0, 0)),
            pl.BlockSpec((1, HID), lambda i: (0, 0)),
        ],
        out_specs=[
            pl.BlockSpec((BE, HID), lambda i: (i, 0)),
            pl.BlockSpec((8, HID), lambda i: (0, 0)),
        ],
        out_shape=[
            jax.ShapeDtypeStruct((E, HID), jnp.float32),
            jax.ShapeDtypeStruct((8, HID), jnp.float32),
        ],
    )(z0, s1, t1, w, b)


def _msg_body(zp_ref, hs_ref, s2_ref, t2_ref, wle_ref, ble_ref, m_ref):
    z2 = jnp.maximum(zp_ref[...] * s2_ref[...] + t2_ref[...], 0.0)
    e = jnp.dot(z2, wle_ref[...], preferred_element_type=jnp.float32) + ble_ref[...]
    m = jnp.maximum(hs_ref[...][:, :HID] + e, 0.0)
    m_ref[...] = jnp.concatenate([m, jnp.zeros_like(m)], axis=1)


def _msg(z2pre, hs, s2, t2, wle, ble):
    return pl.pallas_call(
        _msg_body,
        grid=(GRID_E,),
        in_specs=[
            pl.BlockSpec((BE, HID), lambda i: (i, 0)),
            pl.BlockSpec((BE, 128), lambda i: (i, 0)),
            pl.BlockSpec((1, HID), lambda i: (0, 0)),
            pl.BlockSpec((1, HID), lambda i: (0, 0)),
            pl.BlockSpec((HID, HID), lambda i: (0, 0)),
            pl.BlockSpec((1, HID), lambda i: (0, 0)),
        ],
        out_specs=pl.BlockSpec((BE, 128), lambda i: (i, 0)),
        out_shape=jax.ShapeDtypeStruct((E, 128), jnp.float32),
    )(z2pre, hs, s2, t2, wle, ble)


def _bn_full(x, g, b):
    m = jnp.mean(x, axis=0)
    v = jnp.var(x, axis=0)
    return (x - m) / jnp.sqrt(v + 1e-5) * g + b


def _mlp(h, p, pre):
    h = h @ p[pre + "_l1_W"] + p[pre + "_l1_b"]
    h = _bn_full(h, p[pre + "_bn1_g"], p[pre + "_bn1_bt"])
    h = jax.nn.relu(h)
    h = h @ p[pre + "_l2_W"] + p[pre + "_l2_b"]
    h = _bn_full(h, p[pre + "_bn2_g"], p[pre + "_bn2_bt"])
    return jax.nn.relu(h)


def kernel(x, edge_index, batch, pos_index, pos_enc, pos_batch, zinit_W, zemb_bn1_g, zemb_bn1_bt, zemb_lin_W, zemb_lin_b, zemb_bn2_g, zemb_bn2_bt, xemb_l1_W, xemb_l1_b, xemb_bn1_g, xemb_bn1_bt, xemb_l2_W, xemb_l2_b, xemb_bn2_g, xemb_bn2_bt, c0_le_W, c0_le_b, c0_eps, c0_l1_W, c0_l1_b, c0_bn1_g, c0_bn1_bt, c0_l2_W, c0_l2_b, c0_bn2_g, c0_bn2_bt, c1_le_W, c1_le_b, c1_eps, c1_l1_W, c1_l1_b, c1_bn1_g, c1_bn1_bt, c1_l2_W, c1_l2_b, c1_bn2_g, c1_bn2_bt, c2_le_W, c2_le_b, c2_eps, c2_l1_W, c2_l1_b, c2_bn1_g, c2_bn1_bt, c2_l2_W, c2_l2_b, c2_bn2_g, c2_bn2_bt, lin1_W, lin1_b, bn_lin1_g, bn_lin1_bt, lin2_W, lin2_b):
    p = dict(locals())
    src = edge_index[0]
    dst = edge_index[1]

    z0 = _z0_sc(zinit_W, pos_index, pos_enc, pos_batch)

    st1 = _col_stats(z0, BE)
    s1, t1 = _bn_coeffs(st1, float(E), zemb_bn1_g, zemb_bn1_bt)
    z2pre, st2 = _ztrans(z0, s1, t1, zemb_lin_W, zemb_lin_b[None, :])
    s2, t2 = _bn_coeffs(st2, float(E), zemb_bn2_g, zemb_bn2_bt)

    xs = [_mlp(x, p, "xemb")]

    h = jnp.pad(x, ((0, 0), (0, HID - IN_DIM)))
    p["c0_le_W"] = jnp.pad(c0_le_W, ((0, 0), (0, HID - IN_DIM)))
    p["c0_le_b"] = jnp.pad(c0_le_b, (0, HID - IN_DIM))
    p["c0_l1_W"] = jnp.pad(c0_l1_W, ((0, HID - IN_DIM), (0, 0)))
    for i in range(NL):
        pre = "c%d" % i
        h128 = jnp.pad(h, ((0, 0), (0, 128 - HID)))
        hs = _gather_sc(h128, src)
        m = _msg(z2pre, hs, s2, t2, p[pre + "_le_W"], p[pre + "_le_b"][None, :])
        parts = _agg_sc(m, dst)
        agg = jnp.concatenate([parts[0], parts[1]], axis=0)[:N, :HID]
        h = agg + (1.0 + p[pre + "_eps"]) * h
        h = _mlp(h, p, pre)
        xs.append(h)

    hc = jnp.concatenate(xs, axis=1)
    ones = jnp.ones((N,), dtype=jnp.float32)
    cnt = jax.ops.segment_sum(ones, batch, num_segments=NG)
    pooled = jax.ops.segment_sum(hc, batch, num_segments=NG) / jnp.maximum(cnt, 1.0)[:, None]
    o = pooled @ lin1_W + lin1_b
    o = _bn_full(o, bn_lin1_g, bn_lin1_bt)
    o = jax.nn.relu(o)
    o = o @ lin2_W + lin2_b
    return jax.nn.log_softmax(o, axis=-1)


# speedup vs baseline: 3.1164x; 1.0545x over previous
import dataclasses
import functools

import jax
import jax.numpy as jnp
from jax import lax
from jax.experimental import pallas as pl
from jax.experimental.pallas import tpu as pltpu
from jax.experimental.pallas import tpu_sc as plsc

N = 10000
E = 320000
P = 640000
HID = 64
NL = 3
NC = 10
NG = 256
ZIN = 1800
IN_DIM = 10

BE = 6400
GRID_E = E // BE

NWORK = 32
ES = E // NWORK
ZCH = 400
NZCH = ES // ZCH
ZK = 512


def _sc_compiler_params():
    cp = pltpu.CompilerParams()
    if "needs_layout_passes" in pltpu.CompilerParams.__dataclass_fields__:
        cp = dataclasses.replace(cp, needs_layout_passes=False)
    return cp


def _z0_body(zinit_hbm, pi_hbm, pe_hbm, pb_hbm, btab_hbm, z0_hbm,
             bvec, idxv, encv, segv, gbuf, outbuf, sem):
    wid = lax.axis_index("s") * 2 + lax.axis_index("c")
    pltpu.sync_copy(btab_hbm.at[wid], bvec)
    iota16 = lax.iota(jnp.int32, 16)
    bv0 = bvec[pl.ds(0, 16)]
    bv1 = bvec[pl.ds(16, 16)]
    zeros16 = jnp.zeros((16,), jnp.float32)

    def bound(j):
        half = bv0 if j < 16 else bv1
        return lax.reduce_max(jnp.where(iota16 == (j % 16), half, -1), (0,))

    for c in range(NZCH):
        lo = bound(c)
        hi = bound(c + 1)
        lo_al = lax.bitwise_and(lo, jnp.int32(~7))
        nblk = (hi - lo_al + (ZK - 1)) // ZK
        lo16 = jnp.full((16,), lo, jnp.int32)
        hi16 = jnp.full((16,), hi, jnp.int32)
        segbase16 = jnp.full((16,), wid * ES + c * ZCH, jnp.int32)

        @pl.loop(0, ZCH)
        def _(r):
            r16 = jnp.full((16,), r, jnp.int32)
            for j in range(4):
                plsc.store_scatter(outbuf, [r16, iota16 + 16 * j], zeros16)

        @pl.loop(0, nblk)
        def _(b):
            base = pl.multiple_of(lo_al + b * ZK, 8)
            c1 = pltpu.async_copy(pi_hbm.at[pl.ds(base, ZK)], idxv, sem)
            c2 = pltpu.async_copy(pe_hbm.at[pl.ds(base, ZK)], encv, sem)
            c3 = pltpu.async_copy(pb_hbm.at[pl.ds(base, ZK)], segv, sem)
            c3.wait()
            c2.wait()
            c1.wait()
            pltpu.async_copy(zinit_hbm.at[idxv], gbuf, sem).wait()
            base16 = jnp.full((16,), base, jnp.int32)

            @pl.loop(0, ZK, unroll=4)
            def _(k):
                k16 = jnp.full((16,), k, jnp.int32)
                seg16 = plsc.load_gather(segv, [k16])
                enc16 = plsc.load_gather(encv, [k16])
                gp16 = base16 + k16
                mask = (gp16 >= lo16) & (gp16 < hi16)
                enc_eff = jnp.where(mask, enc16, 0.0)
                row16 = jnp.where(mask, seg16 - segbase16, 0)
                for j in range(4):
                    col = iota16 + 16 * j
                    vals = plsc.load_gather(gbuf, [k16, col])
                    plsc.addupdate_scatter(outbuf, [row16, col], vals * enc_eff)

        pltpu.sync_copy(outbuf, z0_hbm.at[pl.ds(wid * ES + c * ZCH, ZCH)])


@jax.jit
def _z0_sc(zinit_W, pos_index, pos_enc, pos_batch):
    bnd = jnp.searchsorted(pos_batch, jnp.arange(0, E + 1, ZCH)).astype(jnp.int32)
    bnd = jnp.pad(bnd, (0, 32))
    rows = jnp.arange(NWORK)[:, None] * NZCH + jnp.arange(32)[None, :]
    btab = bnd[rows]
    zpad = jnp.pad(zinit_W, ((0, 0), (0, 64)))
    pi = jnp.pad(pos_index.astype(jnp.int32), (0, ZK))
    pe = jnp.pad(pos_enc, (0, ZK))
    pb = jnp.pad(pos_batch.astype(jnp.int32), (0, ZK))

    mesh = plsc.VectorSubcoreMesh(core_axis_name="c", subcore_axis_name="s",
                                  num_cores=2, num_subcores=16)
    kern = pl.kernel(
        _z0_body,
        out_type=jax.ShapeDtypeStruct((E, 128), jnp.float32),
        mesh=mesh,
        scratch_types=[
            pltpu.VMEM((32,), jnp.int32),
            pltpu.VMEM((ZK,), jnp.int32),
            pltpu.VMEM((ZK,), jnp.float32),
            pltpu.VMEM((ZK,), jnp.int32),
            pltpu.VMEM((ZK, 128), jnp.float32),
            pltpu.VMEM((ZCH, 128), jnp.float32),
            pltpu.SemaphoreType.DMA,
        ],
        compiler_params=_sc_compiler_params(),
    )
    return kern(zpad, pi, pe, pb, btab)


NOWN = 5120
NACC = 5184
AK = 400
EPT2 = E // 16
NAB2 = EPT2 // AK


def _agg_body(m_hbm, dst_hbm, zeros_hbm, out_hbm, idxv, idxv2, mbuf, accum, sem):
    cid = lax.axis_index("c")
    sid = lax.axis_index("s")

    pltpu.sync_copy(zeros_hbm.at[pl.ds(0, 320)], accum.at[pl.ds(sid * 320, 320)])

    @pl.when(sid == 0)
    def _():
        pltpu.sync_copy(zeros_hbm.at[pl.ds(0, 64)], accum.at[pl.ds(NOWN, 64)])

    plsc.subcore_barrier()

    base_n16 = jnp.full((16,), cid * NOWN, jnp.int32)
    own16 = jnp.full((16,), NOWN, jnp.int32)
    t63 = jnp.full((16,), 63, jnp.int32)

    @pl.loop(0, NAB2)
    def _(b):
        base = pl.multiple_of(sid * EPT2 + b * AK, 8)
        pltpu.sync_copy(dst_hbm.at[pl.ds(base, AK)], idxv)
        pltpu.sync_copy(m_hbm.at[pl.ds(base, AK)], mbuf)
        for g in range(AK // 16):
            dv = idxv[pl.ds(g * 16, 16)]
            local = dv - base_n16
            owned = (local >= 0) & (local < own16)
            trash = own16 + (dv & t63)
            idxv2[pl.ds(g * 16, 16)] = jnp.where(owned, local, trash)
        pltpu.sync_copy(mbuf, accum.at[idxv2], add=True)

    plsc.subcore_barrier()
    pltpu.sync_copy(accum.at[pl.ds(sid * 320, 320)],
                    out_hbm.at[cid].at[pl.ds(sid * 320, 320)])


@jax.jit
def _agg_sc(m, dst):
    zeros = jnp.zeros((320, 128), jnp.float32)
    mesh = plsc.VectorSubcoreMesh(core_axis_name="c", subcore_axis_name="s",
                                  num_cores=2, num_subcores=16)
    kern = pl.kernel(
        _agg_body,
        out_type=jax.ShapeDtypeStruct((2, NOWN, 128), jnp.float32),
        mesh=mesh,
        scratch_types=[
            pltpu.VMEM((AK,), jnp.int32),
            pltpu.VMEM((AK,), jnp.int32),
            pltpu.VMEM((AK, 128), jnp.float32),
            pltpu.VMEM_SHARED((NACC, 128), jnp.float32),
            pltpu.SemaphoreType.DMA,
        ],
        compiler_params=_sc_compiler_params(),
    )
    return kern(m, dst.astype(jnp.int32), zeros)


GK = 400
NGB = (E // NWORK) // GK


def _gat_body(h_hbm, src_hbm, hs_hbm, idxv, gbuf, sem):
    wid = lax.axis_index("s") * 2 + lax.axis_index("c")

    @pl.loop(0, NGB)
    def _(b):
        base = pl.multiple_of(wid * (E // NWORK) + b * GK, 8)
        pltpu.sync_copy(src_hbm.at[pl.ds(base, GK)], idxv)
        pltpu.async_copy(h_hbm.at[idxv], gbuf, sem).wait()
        pltpu.sync_copy(gbuf, hs_hbm.at[pl.ds(base, GK)])


@jax.jit
def _gather_sc(h128, src):
    mesh = plsc.VectorSubcoreMesh(core_axis_name="c", subcore_axis_name="s",
                                  num_cores=2, num_subcores=16)
    kern = pl.kernel(
        _gat_body,
        out_type=jax.ShapeDtypeStruct((E, 128), jnp.float32),
        mesh=mesh,
        scratch_types=[
            pltpu.VMEM((GK,), jnp.int32),
            pltpu.VMEM((GK, 128), jnp.float32),
            pltpu.SemaphoreType.DMA,
        ],
        compiler_params=_sc_compiler_params(),
    )
    return kern(h128, src.astype(jnp.int32))


def _stats_body(x_ref, o_ref):
    i = pl.program_id(0)

    @pl.when(i == 0)
    def _():
        o_ref[...] = jnp.zeros_like(o_ref)

    blk = x_ref[...][:, :HID]
    s1 = jnp.sum(blk, axis=0, keepdims=True)
    s2 = jnp.sum(blk * blk, axis=0, keepdims=True)
    o_ref[0:1, :] += s1
    o_ref[1:2, :] += s2


def _col_stats(x, bd):
    d = x.shape[1]
    return pl.pallas_call(
        _stats_body,
        grid=(x.shape[0] // bd,),
        in_specs=[pl.BlockSpec((bd, d), lambda i: (i, 0))],
        out_specs=pl.BlockSpec((8, HID), lambda i: (0, 0)),
        out_shape=jax.ShapeDtypeStruct((8, HID), jnp.float32),
    )(x)


def _bn_coeffs(stats, n, g, b):
    m = stats[0] / n
    v = stats[1] / n - m * m
    s = g / jnp.sqrt(v + 1e-5)
    t = b - m * s
    return s[None, :], t[None, :]


def _ztrans_body(z0_ref, s1_ref, t1_ref, w_ref, b_ref, zp_ref, st_ref):
    i = pl.program_id(0)

    @pl.when(i == 0)
    def _():
        st_ref[...] = jnp.zeros_like(st_ref)

    z1 = jnp.maximum(z0_ref[...][:, :HID] * s1_ref[...] + t1_ref[...], 0.0)
    zp = jnp.dot(z1, w_ref[...], preferred_element_type=jnp.float32) + b_ref[...]
    zp_ref[...] = zp
    st_ref[0:1, :] += jnp.sum(zp, axis=0, keepdims=True)
    st_ref[1:2, :] += jnp.sum(zp * zp, axis=0, keepdims=True)


def _ztrans(z0, s1, t1, w, b):
    return pl.pallas_call(
        _ztrans_body,
        grid=(GRID_E,),
        in_specs=[
            pl.BlockSpec((BE, 128), lambda i: (i, 0)),
            pl.BlockSpec((1, HID), lambda i: (0, 0)),
            pl.BlockSpec((1, HID), lambda i: (0, 0)),
            pl.BlockSpec((HID, HID), lambda i: (0, 0)),
            pl.BlockSpec((1, HID), lambda i: (0, 0)),
        ],
        out_specs=[
            pl.BlockSpec((BE, HID), lambda i: (i, 0)),
            pl.BlockSpec((8, HID), lambda i: (0, 0)),
        ],
        out_shape=[
            jax.ShapeDtypeStruct((E, HID), jnp.float32),
            jax.ShapeDtypeStruct((8, HID), jnp.float32),
        ],
    )(z0, s1, t1, w, b)


def _msg_body(zp_ref, hs_ref, s2_ref, t2_ref, wle_ref, ble_ref, m_ref):
    z2 = jnp.maximum(zp_ref[...] * s2_ref[...] + t2_ref[...], 0.0)
    e = jnp.dot(z2, wle_ref[...], preferred_element_type=jnp.float32) + ble_ref[...]
    m = jnp.maximum(hs_ref[...][:, :HID] + e, 0.0)
    m_ref[...] = jnp.concatenate([m, jnp.zeros_like(m)], axis=1)


def _msg(z2pre, hs, s2, t2, wle, ble):
    return pl.pallas_call(
        _msg_body,
        grid=(GRID_E,),
        in_specs=[
            pl.BlockSpec((BE, HID), lambda i: (i, 0)),
            pl.BlockSpec((BE, 128), lambda i: (i, 0)),
            pl.BlockSpec((1, HID), lambda i: (0, 0)),
            pl.BlockSpec((1, HID), lambda i: (0, 0)),
            pl.BlockSpec((HID, HID), lambda i: (0, 0)),
            pl.BlockSpec((1, HID), lambda i: (0, 0)),
        ],
        out_specs=pl.BlockSpec((BE, 128), lambda i: (i, 0)),
        out_shape=jax.ShapeDtypeStruct((E, 128), jnp.float32),
    )(z2pre, hs, s2, t2, wle, ble)


def _bn_full(x, g, b):
    m = jnp.mean(x, axis=0)
    v = jnp.var(x, axis=0)
    return (x - m) / jnp.sqrt(v + 1e-5) * g + b


def _mlp(h, p, pre):
    h = h @ p[pre + "_l1_W"] + p[pre + "_l1_b"]
    h = _bn_full(h, p[pre + "_bn1_g"], p[pre + "_bn1_bt"])
    h = jax.nn.relu(h)
    h = h @ p[pre + "_l2_W"] + p[pre + "_l2_b"]
    h = _bn_full(h, p[pre + "_bn2_g"], p[pre + "_bn2_bt"])
    return jax.nn.relu(h)


def kernel(x, edge_index, batch, pos_index, pos_enc, pos_batch, zinit_W, zemb_bn1_g, zemb_bn1_bt, zemb_lin_W, zemb_lin_b, zemb_bn2_g, zemb_bn2_bt, xemb_l1_W, xemb_l1_b, xemb_bn1_g, xemb_bn1_bt, xemb_l2_W, xemb_l2_b, xemb_bn2_g, xemb_bn2_bt, c0_le_W, c0_le_b, c0_eps, c0_l1_W, c0_l1_b, c0_bn1_g, c0_bn1_bt, c0_l2_W, c0_l2_b, c0_bn2_g, c0_bn2_bt, c1_le_W, c1_le_b, c1_eps, c1_l1_W, c1_l1_b, c1_bn1_g, c1_bn1_bt, c1_l2_W, c1_l2_b, c1_bn2_g, c1_bn2_bt, c2_le_W, c2_le_b, c2_eps, c2_l1_W, c2_l1_b, c2_bn1_g, c2_bn1_bt, c2_l2_W, c2_l2_b, c2_bn2_g, c2_bn2_bt, lin1_W, lin1_b, bn_lin1_g, bn_lin1_bt, lin2_W, lin2_b):
    p = dict(locals())
    src = edge_index[0]
    dst = edge_index[1]

    z0 = _z0_sc(zinit_W, pos_index, pos_enc, pos_batch)

    st1 = _col_stats(z0, BE)
    s1, t1 = _bn_coeffs(st1, float(E), zemb_bn1_g, zemb_bn1_bt)
    z2pre, st2 = _ztrans(z0, s1, t1, zemb_lin_W, zemb_lin_b[None, :])
    s2, t2 = _bn_coeffs(st2, float(E), zemb_bn2_g, zemb_bn2_bt)

    xs = [_mlp(x, p, "xemb")]

    h = jnp.pad(x, ((0, 0), (0, HID - IN_DIM)))
    p["c0_le_W"] = jnp.pad(c0_le_W, ((0, 0), (0, HID - IN_DIM)))
    p["c0_le_b"] = jnp.pad(c0_le_b, (0, HID - IN_DIM))
    p["c0_l1_W"] = jnp.pad(c0_l1_W, ((0, HID - IN_DIM), (0, 0)))
    for i in range(NL):
        pre = "c%d" % i
        h128 = jnp.pad(h, ((0, 0), (0, 128 - HID)))
        hs = _gather_sc(h128, src)
        m = _msg(z2pre, hs, s2, t2, p[pre + "_le_W"], p[pre + "_le_b"][None, :])
        parts = _agg_sc(m, dst)
        agg = jnp.concatenate([parts[0], parts[1]], axis=0)[:N, :HID]
        h = agg + (1.0 + p[pre + "_eps"]) * h
        h = _mlp(h, p, pre)
        xs.append(h)

    hc = jnp.concatenate(xs, axis=1)
    ones = jnp.ones((N,), dtype=jnp.float32)
    cnt = jax.ops.segment_sum(ones, batch, num_segments=NG)
    pooled = jax.ops.segment_sum(hc, batch, num_segments=NG) / jnp.maximum(cnt, 1.0)[:, None]
    o = pooled @ lin1_W + lin1_b
    o = _bn_full(o, bn_lin1_g, bn_lin1_bt)
    o = jax.nn.relu(o)
    o = o @ lin2_W + lin2_b
    return jax.nn.log_softmax(o, axis=-1)


# speedup vs baseline: 3.1194x; 1.0010x over previous
import dataclasses
import functools

import jax
import jax.numpy as jnp
from jax import lax
from jax.experimental import pallas as pl
from jax.experimental.pallas import tpu as pltpu
from jax.experimental.pallas import tpu_sc as plsc

N = 10000
E = 320000
P = 640000
HID = 64
NL = 3
NC = 10
NG = 256
ZIN = 1800
IN_DIM = 10

BE = 6400
GRID_E = E // BE

NWORK = 32
ES = E // NWORK
ZCH = 400
NZCH = ES // ZCH
ZK = 512


def _sc_compiler_params():
    cp = pltpu.CompilerParams()
    if "needs_layout_passes" in pltpu.CompilerParams.__dataclass_fields__:
        cp = dataclasses.replace(cp, needs_layout_passes=False)
    return cp


def _z0_body(zinit_hbm, pi_hbm, pe_hbm, pb_hbm, btab_hbm, z0_hbm,
             bvec, idxv, encv, segv, gbuf, outbuf, sem):
    wid = lax.axis_index("s") * 2 + lax.axis_index("c")
    pltpu.sync_copy(btab_hbm.at[wid], bvec)
    iota16 = lax.iota(jnp.int32, 16)
    bv0 = bvec[pl.ds(0, 16)]
    bv1 = bvec[pl.ds(16, 16)]
    zeros16 = jnp.zeros((16,), jnp.float32)

    def bound(j):
        j16 = jnp.full((16,), j, jnp.int32)
        a = lax.reduce_max(jnp.where(iota16 == j16, bv0, -1), (0,))
        b = lax.reduce_max(jnp.where(iota16 == j16 - 16, bv1, -1), (0,))
        return lax.max(a, b)

    @pl.loop(0, NZCH)
    def _(c):
        lo = bound(c)
        hi = bound(c + 1)
        lo_al = lax.bitwise_and(lo, jnp.int32(~7))
        nblk = (hi - lo_al + (ZK - 1)) // ZK
        lo16 = jnp.full((16,), lo, jnp.int32)
        hi16 = jnp.full((16,), hi, jnp.int32)
        segbase16 = jnp.full((16,), wid * ES + c * ZCH, jnp.int32)

        @pl.loop(0, ZCH)
        def _(r):
            r16 = jnp.full((16,), r, jnp.int32)
            for j in range(4):
                plsc.store_scatter(outbuf, [r16, iota16 + 16 * j], zeros16)

        def load_block(base):
            c1 = pltpu.async_copy(pi_hbm.at[pl.ds(base, ZK)], idxv, sem)
            c2 = pltpu.async_copy(pe_hbm.at[pl.ds(base, ZK)], encv, sem)
            c3 = pltpu.async_copy(pb_hbm.at[pl.ds(base, ZK)], segv, sem)
            c3.wait()
            c2.wait()
            c1.wait()
            pltpu.async_copy(zinit_hbm.at[idxv], gbuf, sem).wait()

        def masked_block(b):
            base = pl.multiple_of(lo_al + b * ZK, 8)
            load_block(base)
            base16 = jnp.full((16,), base, jnp.int32)

            @pl.loop(0, ZK, unroll=4)
            def _(k):
                k16 = jnp.full((16,), k, jnp.int32)
                seg16 = plsc.load_gather(segv, [k16])
                enc16 = plsc.load_gather(encv, [k16])
                gp16 = base16 + k16
                mask = (gp16 >= lo16) & (gp16 < hi16)
                enc_eff = jnp.where(mask, enc16, 0.0)
                row16 = jnp.where(mask, seg16 - segbase16, 0)
                for j in range(4):
                    col = iota16 + 16 * j
                    vals = plsc.load_gather(gbuf, [k16, col])
                    plsc.addupdate_scatter(outbuf, [row16, col], vals * enc_eff)

        masked_block(jnp.int32(0))

        @pl.loop(1, nblk - 1)
        def _(b):
            base = pl.multiple_of(lo_al + b * ZK, 8)
            load_block(base)

            @pl.loop(0, ZK, unroll=4)
            def _(k):
                k16 = jnp.full((16,), k, jnp.int32)
                seg16 = plsc.load_gather(segv, [k16])
                enc16 = plsc.load_gather(encv, [k16])
                row16 = seg16 - segbase16
                for j in range(4):
                    col = iota16 + 16 * j
                    vals = plsc.load_gather(gbuf, [k16, col])
                    plsc.addupdate_scatter(outbuf, [row16, col], vals * enc16)

        @pl.when(nblk >= 2)
        def _():
            masked_block(nblk - 1)

        pltpu.sync_copy(outbuf, z0_hbm.at[pl.ds(pl.multiple_of(wid * ES + c * ZCH, 8), ZCH)])


@jax.jit
def _z0_sc(zinit_W, pos_index, pos_enc, pos_batch):
    bnd = jnp.searchsorted(pos_batch, jnp.arange(0, E + 1, ZCH)).astype(jnp.int32)
    bnd = jnp.pad(bnd, (0, 32))
    rows = jnp.arange(NWORK)[:, None] * NZCH + jnp.arange(32)[None, :]
    btab = bnd[rows]
    zpad = jnp.pad(zinit_W, ((0, 0), (0, 64)))
    pi = jnp.pad(pos_index.astype(jnp.int32), (0, ZK))
    pe = jnp.pad(pos_enc, (0, ZK))
    pb = jnp.pad(pos_batch.astype(jnp.int32), (0, ZK))

    mesh = plsc.VectorSubcoreMesh(core_axis_name="c", subcore_axis_name="s",
                                  num_cores=2, num_subcores=16)
    kern = pl.kernel(
        _z0_body,
        out_type=jax.ShapeDtypeStruct((E, 128), jnp.float32),
        mesh=mesh,
        scratch_types=[
            pltpu.VMEM((32,), jnp.int32),
            pltpu.VMEM((ZK,), jnp.int32),
            pltpu.VMEM((ZK,), jnp.float32),
            pltpu.VMEM((ZK,), jnp.int32),
            pltpu.VMEM((ZK, 128), jnp.float32),
            pltpu.VMEM((ZCH, 128), jnp.float32),
            pltpu.SemaphoreType.DMA,
        ],
        compiler_params=_sc_compiler_params(),
    )
    return kern(zpad, pi, pe, pb, btab)


NOWN = 5120
NACC = 5184
AK = 400
EPT2 = E // 16
NAB2 = EPT2 // AK


def _agg_body(m_hbm, dst_hbm, zeros_hbm, out_hbm, idxv, idxv2, mbuf, accum, sem):
    cid = lax.axis_index("c")
    sid = lax.axis_index("s")

    pltpu.sync_copy(zeros_hbm.at[pl.ds(0, 320)], accum.at[pl.ds(sid * 320, 320)])

    @pl.when(sid == 0)
    def _():
        pltpu.sync_copy(zeros_hbm.at[pl.ds(0, 64)], accum.at[pl.ds(NOWN, 64)])

    plsc.subcore_barrier()

    base_n16 = jnp.full((16,), cid * NOWN, jnp.int32)
    own16 = jnp.full((16,), NOWN, jnp.int32)
    t63 = jnp.full((16,), 63, jnp.int32)

    @pl.loop(0, NAB2)
    def _(b):
        base = pl.multiple_of(sid * EPT2 + b * AK, 8)
        pltpu.sync_copy(dst_hbm.at[pl.ds(base, AK)], idxv)
        pltpu.sync_copy(m_hbm.at[pl.ds(base, AK)], mbuf)
        for g in range(AK // 16):
            dv = idxv[pl.ds(g * 16, 16)]
            local = dv - base_n16
            owned = (local >= 0) & (local < own16)
            trash = own16 + (dv & t63)
            idxv2[pl.ds(g * 16, 16)] = jnp.where(owned, local, trash)
        pltpu.sync_copy(mbuf, accum.at[idxv2], add=True)

    plsc.subcore_barrier()
    pltpu.sync_copy(accum.at[pl.ds(sid * 320, 320)],
                    out_hbm.at[cid].at[pl.ds(sid * 320, 320)])


@jax.jit
def _agg_sc(m, dst):
    zeros = jnp.zeros((320, 128), jnp.float32)
    mesh = plsc.VectorSubcoreMesh(core_axis_name="c", subcore_axis_name="s",
                                  num_cores=2, num_subcores=16)
    kern = pl.kernel(
        _agg_body,
        out_type=jax.ShapeDtypeStruct((2, NOWN, 128), jnp.float32),
        mesh=mesh,
        scratch_types=[
            pltpu.VMEM((AK,), jnp.int32),
            pltpu.VMEM((AK,), jnp.int32),
            pltpu.VMEM((AK, 128), jnp.float32),
            pltpu.VMEM_SHARED((NACC, 128), jnp.float32),
            pltpu.SemaphoreType.DMA,
        ],
        compiler_params=_sc_compiler_params(),
    )
    return kern(m, dst.astype(jnp.int32), zeros)


GK = 400
NGB = (E // NWORK) // GK


def _gat_body(h_hbm, src_hbm, hs_hbm, idxv, gbuf, sem):
    wid = lax.axis_index("s") * 2 + lax.axis_index("c")

    @pl.loop(0, NGB)
    def _(b):
        base = pl.multiple_of(wid * (E // NWORK) + b * GK, 8)
        pltpu.sync_copy(src_hbm.at[pl.ds(base, GK)], idxv)
        pltpu.async_copy(h_hbm.at[idxv], gbuf, sem).wait()
        pltpu.sync_copy(gbuf, hs_hbm.at[pl.ds(base, GK)])


@jax.jit
def _gather_sc(h128, src):
    mesh = plsc.VectorSubcoreMesh(core_axis_name="c", subcore_axis_name="s",
                                  num_cores=2, num_subcores=16)
    kern = pl.kernel(
        _gat_body,
        out_type=jax.ShapeDtypeStruct((E, 128), jnp.float32),
        mesh=mesh,
        scratch_types=[
            pltpu.VMEM((GK,), jnp.int32),
            pltpu.VMEM((GK, 128), jnp.float32),
            pltpu.SemaphoreType.DMA,
        ],
        compiler_params=_sc_compiler_params(),
    )
    return kern(h128, src.astype(jnp.int32))


def _stats_body(x_ref, o_ref):
    i = pl.program_id(0)

    @pl.when(i == 0)
    def _():
        o_ref[...] = jnp.zeros_like(o_ref)

    blk = x_ref[...][:, :HID]
    s1 = jnp.sum(blk, axis=0, keepdims=True)
    s2 = jnp.sum(blk * blk, axis=0, keepdims=True)
    o_ref[0:1, :] += s1
    o_ref[1:2, :] += s2


def _col_stats(x, bd):
    d = x.shape[1]
    return pl.pallas_call(
        _stats_body,
        grid=(x.shape[0] // bd,),
        in_specs=[pl.BlockSpec((bd, d), lambda i: (i, 0))],
        out_specs=pl.BlockSpec((8, HID), lambda i: (0, 0)),
        out_shape=jax.ShapeDtypeStruct((8, HID), jnp.float32),
    )(x)


def _bn_coeffs(stats, n, g, b):
    m = stats[0] / n
    v = stats[1] / n - m * m
    s = g / jnp.sqrt(v + 1e-5)
    t = b - m * s
    return s[None, :], t[None, :]


def _ztrans_body(z0_ref, s1_ref, t1_ref, w_ref, b_ref, zp_ref, st_ref):
    i = pl.program_id(0)

    @pl.when(i == 0)
    def _():
        st_ref[...] = jnp.zeros_like(st_ref)

    z1 = jnp.maximum(z0_ref[...][:, :HID] * s1_ref[...] + t1_ref[...], 0.0)
    zp = jnp.dot(z1, w_ref[...], preferred_element_type=jnp.float32) + b_ref[...]
    zp_ref[...] = zp
    st_ref[0:1, :] += jnp.sum(zp, axis=0, keepdims=True)
    st_ref[1:2, :] += jnp.sum(zp * zp, axis=0, keepdims=True)


def _ztrans(z0, s1, t1, w, b):
    return pl.pallas_call(
        _ztrans_body,
        grid=(GRID_E,),
        in_specs=[
            pl.BlockSpec((BE, 128), lambda i: (i, 0)),
            pl.BlockSpec((1, HID), lambda i: (0, 0)),
            pl.BlockSpec((1, HID), lambda i: (0, 0)),
            pl.BlockSpec((HID, HID), lambda i: (0, 0)),
            pl.BlockSpec((1, HID), lambda i: (0, 0)),
        ],
        out_specs=[
            pl.BlockSpec((BE, HID), lambda i: (i, 0)),
            pl.BlockSpec((8, HID), lambda i: (0, 0)),
        ],
        out_shape=[
            jax.ShapeDtypeStruct((E, HID), jnp.float32),
            jax.ShapeDtypeStruct((8, HID), jnp.float32),
        ],
    )(z0, s1, t1, w, b)


def _msg_body(zp_ref, hs_ref, s2_ref, t2_ref, wle_ref, ble_ref, m_ref):
    z2 = jnp.maximum(zp_ref[...] * s2_ref[...] + t2_ref[...], 0.0)
    e = jnp.dot(z2, wle_ref[...], preferred_element_type=jnp.float32) + ble_ref[...]
    m = jnp.maximum(hs_ref[...][:, :HID] + e, 0.0)
    m_ref[...] = jnp.concatenate([m, jnp.zeros_like(m)], axis=1)


def _msg(z2pre, hs, s2, t2, wle, ble):
    return pl.pallas_call(
        _msg_body,
        grid=(GRID_E,),
        in_specs=[
            pl.BlockSpec((BE, HID), lambda i: (i, 0)),
            pl.BlockSpec((BE, 128), lambda i: (i, 0)),
            pl.BlockSpec((1, HID), lambda i: (0, 0)),
            pl.BlockSpec((1, HID), lambda i: (0, 0)),
            pl.BlockSpec((HID, HID), lambda i: (0, 0)),
            pl.BlockSpec((1, HID), lambda i: (0, 0)),
        ],
        out_specs=pl.BlockSpec((BE, 128), lambda i: (i, 0)),
        out_shape=jax.ShapeDtypeStruct((E, 128), jnp.float32),
    )(z2pre, hs, s2, t2, wle, ble)


def _bn_full(x, g, b):
    m = jnp.mean(x, axis=0)
    v = jnp.var(x, axis=0)
    return (x - m) / jnp.sqrt(v + 1e-5) * g + b


def _mlp(h, p, pre):
    h = h @ p[pre + "_l1_W"] + p[pre + "_l1_b"]
    h = _bn_full(h, p[pre + "_bn1_g"], p[pre + "_bn1_bt"])
    h = jax.nn.relu(h)
    h = h @ p[pre + "_l2_W"] + p[pre + "_l2_b"]
    h = _bn_full(h, p[pre + "_bn2_g"], p[pre + "_bn2_bt"])
    return jax.nn.relu(h)


def kernel(x, edge_index, batch, pos_index, pos_enc, pos_batch, zinit_W, zemb_bn1_g, zemb_bn1_bt, zemb_lin_W, zemb_lin_b, zemb_bn2_g, zemb_bn2_bt, xemb_l1_W, xemb_l1_b, xemb_bn1_g, xemb_bn1_bt, xemb_l2_W, xemb_l2_b, xemb_bn2_g, xemb_bn2_bt, c0_le_W, c0_le_b, c0_eps, c0_l1_W, c0_l1_b, c0_bn1_g, c0_bn1_bt, c0_l2_W, c0_l2_b, c0_bn2_g, c0_bn2_bt, c1_le_W, c1_le_b, c1_eps, c1_l1_W, c1_l1_b, c1_bn1_g, c1_bn1_bt, c1_l2_W, c1_l2_b, c1_bn2_g, c1_bn2_bt, c2_le_W, c2_le_b, c2_eps, c2_l1_W, c2_l1_b, c2_bn1_g, c2_bn1_bt, c2_l2_W, c2_l2_b, c2_bn2_g, c2_bn2_bt, lin1_W, lin1_b, bn_lin1_g, bn_lin1_bt, lin2_W, lin2_b):
    p = dict(locals())
    src = edge_index[0]
    dst = edge_index[1]

    z0 = _z0_sc(zinit_W, pos_index, pos_enc, pos_batch)

    st1 = _col_stats(z0, BE)
    s1, t1 = _bn_coeffs(st1, float(E), zemb_bn1_g, zemb_bn1_bt)
    z2pre, st2 = _ztrans(z0, s1, t1, zemb_lin_W, zemb_lin_b[None, :])
    s2, t2 = _bn_coeffs(st2, float(E), zemb_bn2_g, zemb_bn2_bt)

    xs = [_mlp(x, p, "xemb")]

    h = jnp.pad(x, ((0, 0), (0, HID - IN_DIM)))
    p["c0_le_W"] = jnp.pad(c0_le_W, ((0, 0), (0, HID - IN_DIM)))
    p["c0_le_b"] = jnp.pad(c0_le_b, (0, HID - IN_DIM))
    p["c0_l1_W"] = jnp.pad(c0_l1_W, ((0, HID - IN_DIM), (0, 0)))
    for i in range(NL):
        pre = "c%d" % i
        h128 = jnp.pad(h, ((0, 0), (0, 128 - HID)))
        hs = _gather_sc(h128, src)
        m = _msg(z2pre, hs, s2, t2, p[pre + "_le_W"], p[pre + "_le_b"][None, :])
        parts = _agg_sc(m, dst)
        agg = jnp.concatenate([parts[0], parts[1]], axis=0)[:N, :HID]
        h = agg + (1.0 + p[pre + "_eps"]) * h
        h = _mlp(h, p, pre)
        xs.append(h)

    hc = jnp.concatenate(xs, axis=1)
    ones = jnp.ones((N,), dtype=jnp.float32)
    cnt = jax.ops.segment_sum(ones, batch, num_segments=NG)
    pooled = jax.ops.segment_sum(hc, batch, num_segments=NG) / jnp.maximum(cnt, 1.0)[:, None]
    o = pooled @ lin1_W + lin1_b
    o = _bn_full(o, bn_lin1_g, bn_lin1_bt)
    o = jax.nn.relu(o)
    o = o @ lin2_W + lin2_b
    return jax.nn.log_softmax(o, axis=-1)


# speedup vs baseline: 3.6192x; 1.1602x over previous
import dataclasses
import functools

import jax
import jax.numpy as jnp
from jax import lax
from jax.experimental import pallas as pl
from jax.experimental.pallas import tpu as pltpu
from jax.experimental.pallas import tpu_sc as plsc

N = 10000
E = 320000
P = 640000
HID = 64
NL = 3
NC = 10
NG = 256
ZIN = 1800
IN_DIM = 10

BE = 6400
GRID_E = E // BE

NWORK = 32
ES = E // NWORK
ZCH = 400
NZCH = ES // ZCH
ZK = 512


def _sc_compiler_params():
    cp = pltpu.CompilerParams()
    if "needs_layout_passes" in pltpu.CompilerParams.__dataclass_fields__:
        cp = dataclasses.replace(cp, needs_layout_passes=False)
    return cp


def _z0_body(zinit_hbm, pi_hbm, pe_hbm, pb_hbm, btab_hbm, z0_hbm,
             bvec, idxv, encv, segv, gbuf, outbuf, sem):
    wid = lax.axis_index("s") * 2 + lax.axis_index("c")
    pltpu.sync_copy(btab_hbm.at[wid], bvec)
    iota16 = lax.iota(jnp.int32, 16)
    bv0 = bvec[pl.ds(0, 16)]
    bv1 = bvec[pl.ds(16, 16)]
    zeros16 = jnp.zeros((16,), jnp.float32)

    def bound(j):
        j16 = jnp.full((16,), j, jnp.int32)
        a = lax.reduce_max(jnp.where(iota16 == j16, bv0, -1), (0,))
        b = lax.reduce_max(jnp.where(iota16 == j16 - 16, bv1, -1), (0,))
        return lax.max(a, b)

    @pl.loop(0, NZCH)
    def _(c):
        lo = bound(c)
        hi = bound(c + 1)
        lo_al = lax.bitwise_and(lo, jnp.int32(~7))
        nblk = (hi - lo_al + (ZK - 1)) // ZK
        lo16 = jnp.full((16,), lo, jnp.int32)
        hi16 = jnp.full((16,), hi, jnp.int32)
        segbase16 = jnp.full((16,), wid * ES + c * ZCH, jnp.int32)

        @pl.loop(0, ZCH)
        def _(r):
            r16 = jnp.full((16,), r, jnp.int32)
            for j in range(4):
                plsc.store_scatter(outbuf, [r16, iota16 + 16 * j], zeros16)

        def load_block(base):
            c1 = pltpu.async_copy(pi_hbm.at[pl.ds(base, ZK)], idxv, sem)
            c2 = pltpu.async_copy(pe_hbm.at[pl.ds(base, ZK)], encv, sem)
            c3 = pltpu.async_copy(pb_hbm.at[pl.ds(base, ZK)], segv, sem)
            c3.wait()
            c2.wait()
            c1.wait()
            pltpu.async_copy(zinit_hbm.at[idxv], gbuf, sem).wait()

        def masked_block(b):
            base = pl.multiple_of(lo_al + b * ZK, 8)
            load_block(base)
            base16 = jnp.full((16,), base, jnp.int32)

            @plsc.parallel_loop(0, ZK, unroll=4)
            def _(k):
                k16 = jnp.full((16,), k, jnp.int32)
                seg16 = plsc.load_gather(segv, [k16])
                enc16 = plsc.load_gather(encv, [k16])
                gp16 = base16 + k16
                mask = (gp16 >= lo16) & (gp16 < hi16)
                enc_eff = jnp.where(mask, enc16, 0.0)
                row16 = jnp.where(mask, seg16 - segbase16, 0)
                for j in range(4):
                    col = iota16 + 16 * j
                    vals = plsc.load_gather(gbuf, [k16, col])
                    plsc.addupdate_scatter(outbuf, [row16, col], vals * enc_eff)

        masked_block(jnp.int32(0))

        @pl.loop(1, nblk - 1)
        def _(b):
            base = pl.multiple_of(lo_al + b * ZK, 8)
            load_block(base)

            @plsc.parallel_loop(0, ZK, unroll=4)
            def _(k):
                k16 = jnp.full((16,), k, jnp.int32)
                seg16 = plsc.load_gather(segv, [k16])
                enc16 = plsc.load_gather(encv, [k16])
                row16 = seg16 - segbase16
                for j in range(4):
                    col = iota16 + 16 * j
                    vals = plsc.load_gather(gbuf, [k16, col])
                    plsc.addupdate_scatter(outbuf, [row16, col], vals * enc16)

        @pl.when(nblk >= 2)
        def _():
            masked_block(nblk - 1)

        pltpu.sync_copy(outbuf, z0_hbm.at[pl.ds(pl.multiple_of(wid * ES + c * ZCH, 8), ZCH)])


@jax.jit
def _z0_sc(zinit_W, pos_index, pos_enc, pos_batch):
    bnd = jnp.searchsorted(pos_batch, jnp.arange(0, E + 1, ZCH)).astype(jnp.int32)
    bnd = jnp.pad(bnd, (0, 32))
    rows = jnp.arange(NWORK)[:, None] * NZCH + jnp.arange(32)[None, :]
    btab = bnd[rows]
    zpad = jnp.pad(zinit_W, ((0, 0), (0, 64)))
    pi = jnp.pad(pos_index.astype(jnp.int32), (0, ZK))
    pe = jnp.pad(pos_enc, (0, ZK))
    pb = jnp.pad(pos_batch.astype(jnp.int32), (0, ZK))

    mesh = plsc.VectorSubcoreMesh(core_axis_name="c", subcore_axis_name="s",
                                  num_cores=2, num_subcores=16)
    kern = pl.kernel(
        _z0_body,
        out_type=jax.ShapeDtypeStruct((E, 128), jnp.float32),
        mesh=mesh,
        scratch_types=[
            pltpu.VMEM((32,), jnp.int32),
            pltpu.VMEM((ZK,), jnp.int32),
            pltpu.VMEM((ZK,), jnp.float32),
            pltpu.VMEM((ZK,), jnp.int32),
            pltpu.VMEM((ZK, 128), jnp.float32),
            pltpu.VMEM((ZCH, 128), jnp.float32),
            pltpu.SemaphoreType.DMA,
        ],
        compiler_params=_sc_compiler_params(),
    )
    return kern(zpad, pi, pe, pb, btab)


NOWN = 5120
NACC = 5184
AK = 400
EPT2 = E // 16
NAB2 = EPT2 // AK


def _agg_body(m_hbm, dst_hbm, zeros_hbm, out_hbm, idxv, idxv2, mbuf, accum, sem):
    cid = lax.axis_index("c")
    sid = lax.axis_index("s")

    pltpu.sync_copy(zeros_hbm.at[pl.ds(0, 320)], accum.at[pl.ds(sid * 320, 320)])

    @pl.when(sid == 0)
    def _():
        pltpu.sync_copy(zeros_hbm.at[pl.ds(0, 64)], accum.at[pl.ds(NOWN, 64)])

    plsc.subcore_barrier()

    base_n16 = jnp.full((16,), cid * NOWN, jnp.int32)
    own16 = jnp.full((16,), NOWN, jnp.int32)
    t63 = jnp.full((16,), 63, jnp.int32)

    @pl.loop(0, NAB2)
    def _(b):
        base = pl.multiple_of(sid * EPT2 + b * AK, 8)
        pltpu.sync_copy(dst_hbm.at[pl.ds(base, AK)], idxv)
        pltpu.sync_copy(m_hbm.at[pl.ds(base, AK)], mbuf)
        for g in range(AK // 16):
            dv = idxv[pl.ds(g * 16, 16)]
            local = dv - base_n16
            owned = (local >= 0) & (local < own16)
            trash = own16 + (dv & t63)
            idxv2[pl.ds(g * 16, 16)] = jnp.where(owned, local, trash)
        pltpu.sync_copy(mbuf, accum.at[idxv2], add=True)

    plsc.subcore_barrier()
    pltpu.sync_copy(accum.at[pl.ds(sid * 320, 320)],
                    out_hbm.at[cid].at[pl.ds(sid * 320, 320)])


@jax.jit
def _agg_sc(m, dst):
    zeros = jnp.zeros((320, 128), jnp.float32)
    mesh = plsc.VectorSubcoreMesh(core_axis_name="c", subcore_axis_name="s",
                                  num_cores=2, num_subcores=16)
    kern = pl.kernel(
        _agg_body,
        out_type=jax.ShapeDtypeStruct((2, NOWN, 128), jnp.float32),
        mesh=mesh,
        scratch_types=[
            pltpu.VMEM((AK,), jnp.int32),
            pltpu.VMEM((AK,), jnp.int32),
            pltpu.VMEM((AK, 128), jnp.float32),
            pltpu.VMEM_SHARED((NACC, 128), jnp.float32),
            pltpu.SemaphoreType.DMA,
        ],
        compiler_params=_sc_compiler_params(),
    )
    return kern(m, dst.astype(jnp.int32), zeros)


GK = 400
NGB = (E // NWORK) // GK


def _gat_body(h_hbm, src_hbm, hs_hbm, idxv, gbuf, sem):
    wid = lax.axis_index("s") * 2 + lax.axis_index("c")

    @pl.loop(0, NGB)
    def _(b):
        base = pl.multiple_of(wid * (E // NWORK) + b * GK, 8)
        pltpu.sync_copy(src_hbm.at[pl.ds(base, GK)], idxv)
        pltpu.async_copy(h_hbm.at[idxv], gbuf, sem).wait()
        pltpu.sync_copy(gbuf, hs_hbm.at[pl.ds(base, GK)])


@jax.jit
def _gather_sc(h128, src):
    mesh = plsc.VectorSubcoreMesh(core_axis_name="c", subcore_axis_name="s",
                                  num_cores=2, num_subcores=16)
    kern = pl.kernel(
        _gat_body,
        out_type=jax.ShapeDtypeStruct((E, 128), jnp.float32),
        mesh=mesh,
        scratch_types=[
            pltpu.VMEM((GK,), jnp.int32),
            pltpu.VMEM((GK, 128), jnp.float32),
            pltpu.SemaphoreType.DMA,
        ],
        compiler_params=_sc_compiler_params(),
    )
    return kern(h128, src.astype(jnp.int32))


def _stats_body(x_ref, o_ref):
    i = pl.program_id(0)

    @pl.when(i == 0)
    def _():
        o_ref[...] = jnp.zeros_like(o_ref)

    blk = x_ref[...][:, :HID]
    s1 = jnp.sum(blk, axis=0, keepdims=True)
    s2 = jnp.sum(blk * blk, axis=0, keepdims=True)
    o_ref[0:1, :] += s1
    o_ref[1:2, :] += s2


def _col_stats(x, bd):
    d = x.shape[1]
    return pl.pallas_call(
        _stats_body,
        grid=(x.shape[0] // bd,),
        in_specs=[pl.BlockSpec((bd, d), lambda i: (i, 0))],
        out_specs=pl.BlockSpec((8, HID), lambda i: (0, 0)),
        out_shape=jax.ShapeDtypeStruct((8, HID), jnp.float32),
    )(x)


def _bn_coeffs(stats, n, g, b):
    m = stats[0] / n
    v = stats[1] / n - m * m
    s = g / jnp.sqrt(v + 1e-5)
    t = b - m * s
    return s[None, :], t[None, :]


def _ztrans_body(z0_ref, s1_ref, t1_ref, w_ref, b_ref, zp_ref, st_ref):
    i = pl.program_id(0)

    @pl.when(i == 0)
    def _():
        st_ref[...] = jnp.zeros_like(st_ref)

    z1 = jnp.maximum(z0_ref[...][:, :HID] * s1_ref[...] + t1_ref[...], 0.0)
    zp = jnp.dot(z1, w_ref[...], preferred_element_type=jnp.float32) + b_ref[...]
    zp_ref[...] = zp
    st_ref[0:1, :] += jnp.sum(zp, axis=0, keepdims=True)
    st_ref[1:2, :] += jnp.sum(zp * zp, axis=0, keepdims=True)


def _ztrans(z0, s1, t1, w, b):
    return pl.pallas_call(
        _ztrans_body,
        grid=(GRID_E,),
        in_specs=[
            pl.BlockSpec((BE, 128), lambda i: (i, 0)),
            pl.BlockSpec((1, HID), lambda i: (0, 0)),
            pl.BlockSpec((1, HID), lambda i: (0, 0)),
            pl.BlockSpec((HID, HID), lambda i: (0, 0)),
            pl.BlockSpec((1, HID), lambda i: (0, 0)),
        ],
        out_specs=[
            pl.BlockSpec((BE, HID), lambda i: (i, 0)),
            pl.BlockSpec((8, HID), lambda i: (0, 0)),
        ],
        out_shape=[
            jax.ShapeDtypeStruct((E, HID), jnp.float32),
            jax.ShapeDtypeStruct((8, HID), jnp.float32),
        ],
    )(z0, s1, t1, w, b)


def _msg_body(zp_ref, hs_ref, s2_ref, t2_ref, wle_ref, ble_ref, m_ref):
    z2 = jnp.maximum(zp_ref[...] * s2_ref[...] + t2_ref[...], 0.0)
    e = jnp.dot(z2, wle_ref[...], preferred_element_type=jnp.float32) + ble_ref[...]
    m = jnp.maximum(hs_ref[...][:, :HID] + e, 0.0)
    m_ref[...] = jnp.concatenate([m, jnp.zeros_like(m)], axis=1)


def _msg(z2pre, hs, s2, t2, wle, ble):
    return pl.pallas_call(
        _msg_body,
        grid=(GRID_E,),
        in_specs=[
            pl.BlockSpec((BE, HID), lambda i: (i, 0)),
            pl.BlockSpec((BE, 128), lambda i: (i, 0)),
            pl.BlockSpec((1, HID), lambda i: (0, 0)),
            pl.BlockSpec((1, HID), lambda i: (0, 0)),
            pl.BlockSpec((HID, HID), lambda i: (0, 0)),
            pl.BlockSpec((1, HID), lambda i: (0, 0)),
        ],
        out_specs=pl.BlockSpec((BE, 128), lambda i: (i, 0)),
        out_shape=jax.ShapeDtypeStruct((E, 128), jnp.float32),
    )(z2pre, hs, s2, t2, wle, ble)


def _bn_full(x, g, b):
    m = jnp.mean(x, axis=0)
    v = jnp.var(x, axis=0)
    return (x - m) / jnp.sqrt(v + 1e-5) * g + b


def _mlp(h, p, pre):
    h = h @ p[pre + "_l1_W"] + p[pre + "_l1_b"]
    h = _bn_full(h, p[pre + "_bn1_g"], p[pre + "_bn1_bt"])
    h = jax.nn.relu(h)
    h = h @ p[pre + "_l2_W"] + p[pre + "_l2_b"]
    h = _bn_full(h, p[pre + "_bn2_g"], p[pre + "_bn2_bt"])
    return jax.nn.relu(h)


def kernel(x, edge_index, batch, pos_index, pos_enc, pos_batch, zinit_W, zemb_bn1_g, zemb_bn1_bt, zemb_lin_W, zemb_lin_b, zemb_bn2_g, zemb_bn2_bt, xemb_l1_W, xemb_l1_b, xemb_bn1_g, xemb_bn1_bt, xemb_l2_W, xemb_l2_b, xemb_bn2_g, xemb_bn2_bt, c0_le_W, c0_le_b, c0_eps, c0_l1_W, c0_l1_b, c0_bn1_g, c0_bn1_bt, c0_l2_W, c0_l2_b, c0_bn2_g, c0_bn2_bt, c1_le_W, c1_le_b, c1_eps, c1_l1_W, c1_l1_b, c1_bn1_g, c1_bn1_bt, c1_l2_W, c1_l2_b, c1_bn2_g, c1_bn2_bt, c2_le_W, c2_le_b, c2_eps, c2_l1_W, c2_l1_b, c2_bn1_g, c2_bn1_bt, c2_l2_W, c2_l2_b, c2_bn2_g, c2_bn2_bt, lin1_W, lin1_b, bn_lin1_g, bn_lin1_bt, lin2_W, lin2_b):
    p = dict(locals())
    src = edge_index[0]
    dst = edge_index[1]

    z0 = _z0_sc(zinit_W, pos_index, pos_enc, pos_batch)

    st1 = _col_stats(z0, BE)
    s1, t1 = _bn_coeffs(st1, float(E), zemb_bn1_g, zemb_bn1_bt)
    z2pre, st2 = _ztrans(z0, s1, t1, zemb_lin_W, zemb_lin_b[None, :])
    s2, t2 = _bn_coeffs(st2, float(E), zemb_bn2_g, zemb_bn2_bt)

    xs = [_mlp(x, p, "xemb")]

    h = jnp.pad(x, ((0, 0), (0, HID - IN_DIM)))
    p["c0_le_W"] = jnp.pad(c0_le_W, ((0, 0), (0, HID - IN_DIM)))
    p["c0_le_b"] = jnp.pad(c0_le_b, (0, HID - IN_DIM))
    p["c0_l1_W"] = jnp.pad(c0_l1_W, ((0, HID - IN_DIM), (0, 0)))
    for i in range(NL):
        pre = "c%d" % i
        h128 = jnp.pad(h, ((0, 0), (0, 128 - HID)))
        hs = _gather_sc(h128, src)
        m = _msg(z2pre, hs, s2, t2, p[pre + "_le_W"], p[pre + "_le_b"][None, :])
        parts = _agg_sc(m, dst)
        agg = jnp.concatenate([parts[0], parts[1]], axis=0)[:N, :HID]
        h = agg + (1.0 + p[pre + "_eps"]) * h
        h = _mlp(h, p, pre)
        xs.append(h)

    hc = jnp.concatenate(xs, axis=1)
    ones = jnp.ones((N,), dtype=jnp.float32)
    cnt = jax.ops.segment_sum(ones, batch, num_segments=NG)
    pooled = jax.ops.segment_sum(hc, batch, num_segments=NG) / jnp.maximum(cnt, 1.0)[:, None]
    o = pooled @ lin1_W + lin1_b
    o = _bn_full(o, bn_lin1_g, bn_lin1_bt)
    o = jax.nn.relu(o)
    o = o @ lin2_W + lin2_b
    return jax.nn.log_softmax(o, axis=-1)


# speedup vs baseline: 3.8233x; 1.0564x over previous
import dataclasses
import functools

import jax
import jax.numpy as jnp
from jax import lax
from jax.experimental import pallas as pl
from jax.experimental.pallas import tpu as pltpu
from jax.experimental.pallas import tpu_sc as plsc

N = 10000
E = 320000
P = 640000
HID = 64
NL = 3
NC = 10
NG = 256
ZIN = 1800
IN_DIM = 10

BE = 6400
GRID_E = E // BE

NWORK = 32
ES = E // NWORK
ZCH = 400
NZCH = ES // ZCH
ZK = 512


def _sc_compiler_params():
    cp = pltpu.CompilerParams()
    if "needs_layout_passes" in pltpu.CompilerParams.__dataclass_fields__:
        cp = dataclasses.replace(cp, needs_layout_passes=False)
    return cp


def _z0_body(zinit_hbm, pi_hbm, pe_hbm, pb_hbm, btab_hbm, z0_hbm,
             bvec, idxv, encv, segv, gbuf, outbuf, sem):
    wid = lax.axis_index("s") * 2 + lax.axis_index("c")
    pltpu.sync_copy(btab_hbm.at[wid], bvec)
    iota16 = lax.iota(jnp.int32, 16)
    bv0 = bvec[pl.ds(0, 16)]
    bv1 = bvec[pl.ds(16, 16)]
    zeros16 = jnp.zeros((16,), jnp.float32)

    def bound(j):
        j16 = jnp.full((16,), j, jnp.int32)
        a = lax.reduce_max(jnp.where(iota16 == j16, bv0, -1), (0,))
        b = lax.reduce_max(jnp.where(iota16 == j16 - 16, bv1, -1), (0,))
        return lax.max(a, b)

    @pl.loop(0, NZCH)
    def _(c):
        lo = bound(c)
        hi = bound(c + 1)
        lo_al = lax.bitwise_and(lo, jnp.int32(~7))
        nblk = (hi - lo_al + (ZK - 1)) // ZK
        lo16 = jnp.full((16,), lo, jnp.int32)
        hi16 = jnp.full((16,), hi, jnp.int32)
        segbase16 = jnp.full((16,), wid * ES + c * ZCH, jnp.int32)

        @pl.loop(0, ZCH)
        def _(r):
            r16 = jnp.full((16,), r, jnp.int32)
            for j in range(4):
                plsc.store_scatter(outbuf, [r16, iota16 + 16 * j], zeros16)

        def load_block(base):
            c1 = pltpu.async_copy(pi_hbm.at[pl.ds(base, ZK)], idxv, sem)
            c2 = pltpu.async_copy(pe_hbm.at[pl.ds(base, ZK)], encv, sem)
            c3 = pltpu.async_copy(pb_hbm.at[pl.ds(base, ZK)], segv, sem)
            c3.wait()
            c2.wait()
            c1.wait()
            pltpu.async_copy(zinit_hbm.at[idxv], gbuf, sem).wait()

        def masked_block(b):
            base = pl.multiple_of(lo_al + b * ZK, 8)
            load_block(base)
            base16 = jnp.full((16,), base, jnp.int32)

            @plsc.parallel_loop(0, ZK, unroll=4)
            def _(k):
                k16 = jnp.full((16,), k, jnp.int32)
                seg16 = plsc.load_gather(segv, [k16])
                enc16 = plsc.load_gather(encv, [k16])
                gp16 = base16 + k16
                mask = (gp16 >= lo16) & (gp16 < hi16)
                enc_eff = jnp.where(mask, enc16, 0.0)
                row16 = jnp.where(mask, seg16 - segbase16, 0)
                for j in range(4):
                    col = iota16 + 16 * j
                    vals = plsc.load_gather(gbuf, [k16, col])
                    plsc.addupdate_scatter(outbuf, [row16, col], vals * enc_eff)

        masked_block(jnp.int32(0))

        @pl.loop(1, nblk - 1)
        def _(b):
            base = pl.multiple_of(lo_al + b * ZK, 8)
            load_block(base)

            @plsc.parallel_loop(0, ZK, unroll=4)
            def _(k):
                k16 = jnp.full((16,), k, jnp.int32)
                seg16 = plsc.load_gather(segv, [k16])
                enc16 = plsc.load_gather(encv, [k16])
                row16 = seg16 - segbase16
                for j in range(4):
                    col = iota16 + 16 * j
                    vals = plsc.load_gather(gbuf, [k16, col])
                    plsc.addupdate_scatter(outbuf, [row16, col], vals * enc16)

        @pl.when(nblk >= 2)
        def _():
            masked_block(nblk - 1)

        pltpu.sync_copy(outbuf, z0_hbm.at[pl.ds(pl.multiple_of(wid * ES + c * ZCH, 8), ZCH)])


@jax.jit
def _z0_sc(zinit_W, pos_index, pos_enc, pos_batch):
    bnd = jnp.searchsorted(pos_batch, jnp.arange(0, E + 1, ZCH)).astype(jnp.int32)
    bnd = jnp.pad(bnd, (0, 32))
    rows = jnp.arange(NWORK)[:, None] * NZCH + jnp.arange(32)[None, :]
    btab = bnd[rows]
    zpad = jnp.pad(zinit_W, ((0, 0), (0, 64)))
    pi = jnp.pad(pos_index.astype(jnp.int32), (0, ZK))
    pe = jnp.pad(pos_enc, (0, ZK))
    pb = jnp.pad(pos_batch.astype(jnp.int32), (0, ZK))

    mesh = plsc.VectorSubcoreMesh(core_axis_name="c", subcore_axis_name="s",
                                  num_cores=2, num_subcores=16)
    kern = pl.kernel(
        _z0_body,
        out_type=jax.ShapeDtypeStruct((E, 128), jnp.float32),
        mesh=mesh,
        scratch_types=[
            pltpu.VMEM((32,), jnp.int32),
            pltpu.VMEM((ZK,), jnp.int32),
            pltpu.VMEM((ZK,), jnp.float32),
            pltpu.VMEM((ZK,), jnp.int32),
            pltpu.VMEM((ZK, 128), jnp.float32),
            pltpu.VMEM((ZCH, 128), jnp.float32),
            pltpu.SemaphoreType.DMA,
        ],
        compiler_params=_sc_compiler_params(),
    )
    return kern(zpad, pi, pe, pb, btab)


NOWN = 5120
NACC = 5184
AK = 400
EPT2 = E // 16
NAB2 = EPT2 // AK


def _agg_body(m_hbm, dst_hbm, zeros_hbm, out_hbm, idxv, idxv2, mbuf, accum, sem):
    cid = lax.axis_index("c")
    sid = lax.axis_index("s")

    pltpu.sync_copy(zeros_hbm.at[pl.ds(0, 320)], accum.at[pl.ds(sid * 320, 320)])

    @pl.when(sid == 0)
    def _():
        pltpu.sync_copy(zeros_hbm.at[pl.ds(0, 64)], accum.at[pl.ds(NOWN, 64)])

    plsc.subcore_barrier()

    base_n16 = jnp.full((16,), cid * NOWN, jnp.int32)
    own16 = jnp.full((16,), NOWN, jnp.int32)
    t63 = jnp.full((16,), 63, jnp.int32)

    @pl.loop(0, NAB2)
    def _(b):
        base = pl.multiple_of(sid * EPT2 + b * AK, 8)
        pltpu.sync_copy(dst_hbm.at[pl.ds(base, AK)], idxv)
        pltpu.sync_copy(m_hbm.at[pl.ds(base, AK)], mbuf)
        for g in range(AK // 16):
            dv = idxv[pl.ds(g * 16, 16)]
            local = dv - base_n16
            owned = (local >= 0) & (local < own16)
            trash = own16 + (dv & t63)
            idxv2[pl.ds(g * 16, 16)] = jnp.where(owned, local, trash)
        pltpu.sync_copy(mbuf, accum.at[idxv2], add=True)

    plsc.subcore_barrier()
    pltpu.sync_copy(accum.at[pl.ds(sid * 320, 320)],
                    out_hbm.at[cid].at[pl.ds(sid * 320, 320)])


@jax.jit
def _agg_sc(m, dst):
    zeros = jnp.zeros((320, 128), jnp.float32)
    mesh = plsc.VectorSubcoreMesh(core_axis_name="c", subcore_axis_name="s",
                                  num_cores=2, num_subcores=16)
    kern = pl.kernel(
        _agg_body,
        out_type=jax.ShapeDtypeStruct((2, NOWN, 128), jnp.float32),
        mesh=mesh,
        scratch_types=[
            pltpu.VMEM((AK,), jnp.int32),
            pltpu.VMEM((AK,), jnp.int32),
            pltpu.VMEM((AK, 128), jnp.float32),
            pltpu.VMEM_SHARED((NACC, 128), jnp.float32),
            pltpu.SemaphoreType.DMA,
        ],
        compiler_params=_sc_compiler_params(),
    )
    return kern(m, dst.astype(jnp.int32), zeros)


GK = 400
NGB = (E // NWORK) // GK


def _gat_body(h_hbm, src_hbm, hs_hbm, idxv, gbuf, sem):
    wid = lax.axis_index("s") * 2 + lax.axis_index("c")

    @pl.loop(0, NGB)
    def _(b):
        base = pl.multiple_of(wid * (E // NWORK) + b * GK, 8)
        pltpu.sync_copy(src_hbm.at[pl.ds(base, GK)], idxv)
        pltpu.async_copy(h_hbm.at[idxv], gbuf, sem).wait()
        pltpu.sync_copy(gbuf, hs_hbm.at[pl.ds(base, GK)])


@jax.jit
def _gather_sc(h128, src):
    mesh = plsc.VectorSubcoreMesh(core_axis_name="c", subcore_axis_name="s",
                                  num_cores=2, num_subcores=16)
    kern = pl.kernel(
        _gat_body,
        out_type=jax.ShapeDtypeStruct((E, 128), jnp.float32),
        mesh=mesh,
        scratch_types=[
            pltpu.VMEM((GK,), jnp.int32),
            pltpu.VMEM((GK, 128), jnp.float32),
            pltpu.SemaphoreType.DMA,
        ],
        compiler_params=_sc_compiler_params(),
    )
    return kern(h128, src.astype(jnp.int32))


def _stats_body(x_ref, o_ref):
    i = pl.program_id(0)

    @pl.when(i == 0)
    def _():
        o_ref[...] = jnp.zeros_like(o_ref)

    blk = x_ref[...][:, :HID]
    s1 = jnp.sum(blk, axis=0, keepdims=True)
    s2 = jnp.sum(blk * blk, axis=0, keepdims=True)
    o_ref[0:1, :] += s1
    o_ref[1:2, :] += s2


def _col_stats(x, bd):
    d = x.shape[1]
    return pl.pallas_call(
        _stats_body,
        grid=(x.shape[0] // bd,),
        in_specs=[pl.BlockSpec((bd, d), lambda i: (i, 0))],
        out_specs=pl.BlockSpec((8, HID), lambda i: (0, 0)),
        out_shape=jax.ShapeDtypeStruct((8, HID), jnp.float32),
    )(x)


def _bn_coeffs(stats, n, g, b):
    m = stats[0] / n
    v = stats[1] / n - m * m
    s = g / jnp.sqrt(v + 1e-5)
    t = b - m * s
    return s[None, :], t[None, :]


def _ztrans_body(z0_ref, s1_ref, t1_ref, w_ref, b_ref, zp_ref, st_ref):
    i = pl.program_id(0)

    @pl.when(i == 0)
    def _():
        st_ref[...] = jnp.zeros_like(st_ref)

    z1 = jnp.maximum(z0_ref[...][:, :HID] * s1_ref[...] + t1_ref[...], 0.0)
    zp = jnp.dot(z1, w_ref[...], preferred_element_type=jnp.float32) + b_ref[...]
    zp_ref[...] = zp
    st_ref[0:1, :] += jnp.sum(zp, axis=0, keepdims=True)
    st_ref[1:2, :] += jnp.sum(zp * zp, axis=0, keepdims=True)


def _ztrans(z0, s1, t1, w, b):
    return pl.pallas_call(
        _ztrans_body,
        grid=(GRID_E,),
        in_specs=[
            pl.BlockSpec((BE, 128), lambda i: (i, 0)),
            pl.BlockSpec((1, HID), lambda i: (0, 0)),
            pl.BlockSpec((1, HID), lambda i: (0, 0)),
            pl.BlockSpec((HID, HID), lambda i: (0, 0)),
            pl.BlockSpec((1, HID), lambda i: (0, 0)),
        ],
        out_specs=[
            pl.BlockSpec((BE, HID), lambda i: (i, 0)),
            pl.BlockSpec((8, HID), lambda i: (0, 0)),
        ],
        out_shape=[
            jax.ShapeDtypeStruct((E, HID), jnp.float32),
            jax.ShapeDtypeStruct((8, HID), jnp.float32),
        ],
    )(z0, s1, t1, w, b)


def _msg_body(zp_ref, hs_ref, s2_ref, t2_ref, wle_ref, ble_ref, m_ref):
    z2 = jnp.maximum(zp_ref[...] * s2_ref[...] + t2_ref[...], 0.0)
    e = jnp.dot(z2, wle_ref[...], preferred_element_type=jnp.float32) + ble_ref[...]
    m = jnp.maximum(hs_ref[...][:, :HID] + e, 0.0)
    m_ref[...] = jnp.concatenate([m, jnp.zeros_like(m)], axis=1)


def _msg(z2pre, hs, s2, t2, wle, ble):
    return pl.pallas_call(
        _msg_body,
        grid=(GRID_E,),
        in_specs=[
            pl.BlockSpec((BE, HID), lambda i: (i, 0)),
            pl.BlockSpec((BE, 128), lambda i: (i, 0)),
            pl.BlockSpec((1, HID), lambda i: (0, 0)),
            pl.BlockSpec((1, HID), lambda i: (0, 0)),
            pl.BlockSpec((HID, HID), lambda i: (0, 0)),
            pl.BlockSpec((1, HID), lambda i: (0, 0)),
        ],
        out_specs=pl.BlockSpec((BE, 128), lambda i: (i, 0)),
        out_shape=jax.ShapeDtypeStruct((E, 128), jnp.float32),
    )(z2pre, hs, s2, t2, wle, ble)


def _bn_in(x, g, b):
    m = jnp.mean(x, axis=0, keepdims=True)
    v = jnp.mean(x * x, axis=0, keepdims=True) - m * m
    return (x - m) / jnp.sqrt(v + 1e-5) * g + b


def _mlp_in(h, w1, b1, g1, t1, w2, b2, g2, t2):
    h = jnp.dot(h, w1, preferred_element_type=jnp.float32) + b1
    h = jnp.maximum(_bn_in(h, g1, t1), 0.0)
    h = jnp.dot(h, w2, preferred_element_type=jnp.float32) + b2
    return jnp.maximum(_bn_in(h, g2, t2), 0.0)


def _xemb_body(x_ref, w1, b1, g1, t1, w2, b2, g2, t2, o_ref):
    o_ref[...] = _mlp_in(x_ref[...], w1[...], b1[...], g1[...], t1[...],
                         w2[...], b2[...], g2[...], t2[...])


def _xemb(x, p):
    return pl.pallas_call(
        _xemb_body,
        out_shape=jax.ShapeDtypeStruct((N, HID), jnp.float32),
    )(x, p["xemb_l1_W"], p["xemb_l1_b"][None, :], p["xemb_bn1_g"][None, :],
      p["xemb_bn1_bt"][None, :], p["xemb_l2_W"], p["xemb_l2_b"][None, :],
      p["xemb_bn2_g"][None, :], p["xemb_bn2_bt"][None, :])


def _node_body(parts_ref, h_ref, sc_ref, w1, b1, g1, t1, w2, b2, g2, t2, o_ref):
    agg = jnp.concatenate([parts_ref[0], parts_ref[1]], axis=0)[:N, :HID]
    h = agg + sc_ref[...] * h_ref[...]
    o_ref[...] = _mlp_in(h, w1[...], b1[...], g1[...], t1[...],
                         w2[...], b2[...], g2[...], t2[...])


def _node_update(parts, h, eps, p, pre):
    return pl.pallas_call(
        _node_body,
        out_shape=jax.ShapeDtypeStruct((N, HID), jnp.float32),
    )(parts, h, (1.0 + eps).reshape(1, 1),
      p[pre + "_l1_W"], p[pre + "_l1_b"][None, :], p[pre + "_bn1_g"][None, :],
      p[pre + "_bn1_bt"][None, :], p[pre + "_l2_W"], p[pre + "_l2_b"][None, :],
      p[pre + "_bn2_g"][None, :], p[pre + "_bn2_bt"][None, :])


def _head_body(x0, x1, x2, x3, b_ref, w1, b1, g1, t1, w2, b2, o_ref):
    hc = jnp.concatenate([x0[...], x1[...], x2[...], x3[...]], axis=1)
    onehot = (b_ref[...] == lax.broadcasted_iota(jnp.int32, (1, NG), 1)
              ).astype(jnp.float32)
    pooled = lax.dot_general(onehot, hc, (((0,), (0,)), ((), ())),
                             preferred_element_type=jnp.float32)
    cnt = jnp.sum(onehot, axis=0)[:, None]
    pooled = pooled / jnp.maximum(cnt, 1.0)
    o = jnp.dot(pooled, w1[...], preferred_element_type=jnp.float32) + b1[...]
    o = jnp.maximum(_bn_in(o, g1[...], t1[...]), 0.0)
    o = jnp.dot(o, w2[...], preferred_element_type=jnp.float32) + b2[...]
    o_ref[...] = jax.nn.log_softmax(o, axis=-1)


def _head(xs, batch, p):
    return pl.pallas_call(
        _head_body,
        out_shape=jax.ShapeDtypeStruct((NG, NC), jnp.float32),
    )(xs[0], xs[1], xs[2], xs[3], batch.astype(jnp.int32)[:, None],
      p["lin1_W"], p["lin1_b"][None, :], p["bn_lin1_g"][None, :],
      p["bn_lin1_bt"][None, :], p["lin2_W"], p["lin2_b"][None, :])


def kernel(x, edge_index, batch, pos_index, pos_enc, pos_batch, zinit_W, zemb_bn1_g, zemb_bn1_bt, zemb_lin_W, zemb_lin_b, zemb_bn2_g, zemb_bn2_bt, xemb_l1_W, xemb_l1_b, xemb_bn1_g, xemb_bn1_bt, xemb_l2_W, xemb_l2_b, xemb_bn2_g, xemb_bn2_bt, c0_le_W, c0_le_b, c0_eps, c0_l1_W, c0_l1_b, c0_bn1_g, c0_bn1_bt, c0_l2_W, c0_l2_b, c0_bn2_g, c0_bn2_bt, c1_le_W, c1_le_b, c1_eps, c1_l1_W, c1_l1_b, c1_bn1_g, c1_bn1_bt, c1_l2_W, c1_l2_b, c1_bn2_g, c1_bn2_bt, c2_le_W, c2_le_b, c2_eps, c2_l1_W, c2_l1_b, c2_bn1_g, c2_bn1_bt, c2_l2_W, c2_l2_b, c2_bn2_g, c2_bn2_bt, lin1_W, lin1_b, bn_lin1_g, bn_lin1_bt, lin2_W, lin2_b):
    p = dict(locals())
    src = edge_index[0]
    dst = edge_index[1]

    z0 = _z0_sc(zinit_W, pos_index, pos_enc, pos_batch)

    st1 = _col_stats(z0, BE)
    s1, t1 = _bn_coeffs(st1, float(E), zemb_bn1_g, zemb_bn1_bt)
    z2pre, st2 = _ztrans(z0, s1, t1, zemb_lin_W, zemb_lin_b[None, :])
    s2, t2 = _bn_coeffs(st2, float(E), zemb_bn2_g, zemb_bn2_bt)

    xs = [_xemb(x, p)]

    h = jnp.pad(x, ((0, 0), (0, HID - IN_DIM)))
    p["c0_le_W"] = jnp.pad(c0_le_W, ((0, 0), (0, HID - IN_DIM)))
    p["c0_le_b"] = jnp.pad(c0_le_b, (0, HID - IN_DIM))
    p["c0_l1_W"] = jnp.pad(c0_l1_W, ((0, HID - IN_DIM), (0, 0)))
    for i in range(NL):
        pre = "c%d" % i
        h128 = jnp.pad(h, ((0, 0), (0, 128 - HID)))
        hs = _gather_sc(h128, src)
        m = _msg(z2pre, hs, s2, t2, p[pre + "_le_W"], p[pre + "_le_b"][None, :])
        parts = _agg_sc(m, dst)
        h = _node_update(parts, h, p[pre + "_eps"], p, pre)
        xs.append(h)

    return _head(xs, batch, p)


# speedup vs baseline: 4.5091x; 1.1794x over previous
import dataclasses
import functools

import jax
import jax.numpy as jnp
from jax import lax
from jax.experimental import pallas as pl
from jax.experimental.pallas import tpu as pltpu
from jax.experimental.pallas import tpu_sc as plsc

N = 10000
E = 320000
P = 640000
HID = 64
NL = 3
NC = 10
NG = 256
ZIN = 1800
IN_DIM = 10

BE = 6400
GRID_E = E // BE

NWORK = 32
ES = E // NWORK
ZCH = 400
NZCH = ES // ZCH
ZK = 512


def _sc_compiler_params():
    cp = pltpu.CompilerParams()
    if "needs_layout_passes" in pltpu.CompilerParams.__dataclass_fields__:
        cp = dataclasses.replace(cp, needs_layout_passes=False)
    return cp


def _z0_body(zinit_hbm, pi_hbm, pe_hbm, pb_hbm, btab_hbm, z0_hbm,
             bvec, idxv, encv, segv, gbuf, outbuf, sem):
    wid = lax.axis_index("s") * 2 + lax.axis_index("c")
    pltpu.sync_copy(btab_hbm.at[wid], bvec)
    iota16 = lax.iota(jnp.int32, 16)
    bv0 = bvec[pl.ds(0, 16)]
    bv1 = bvec[pl.ds(16, 16)]
    zeros16 = jnp.zeros((16,), jnp.float32)

    def bound(j):
        j16 = jnp.full((16,), j, jnp.int32)
        a = lax.reduce_max(jnp.where(iota16 == j16, bv0, -1), (0,))
        b = lax.reduce_max(jnp.where(iota16 == j16 - 16, bv1, -1), (0,))
        return lax.max(a, b)

    @pl.loop(0, NZCH)
    def _(c):
        lo = bound(c)
        hi = bound(c + 1)
        lo_al = lax.bitwise_and(lo, jnp.int32(~7))
        nblk = (hi - lo_al + (ZK - 1)) // ZK
        lo16 = jnp.full((16,), lo, jnp.int32)
        hi16 = jnp.full((16,), hi, jnp.int32)
        segbase16 = jnp.full((16,), wid * ES + c * ZCH, jnp.int32)

        @pl.loop(0, ZCH)
        def _(r):
            r16 = jnp.full((16,), r, jnp.int32)
            for j in range(4):
                plsc.store_scatter(outbuf, [r16, iota16 + 16 * j], zeros16)

        def load_block(base):
            c1 = pltpu.async_copy(pi_hbm.at[pl.ds(base, ZK)], idxv, sem)
            c2 = pltpu.async_copy(pe_hbm.at[pl.ds(base, ZK)], encv, sem)
            c3 = pltpu.async_copy(pb_hbm.at[pl.ds(base, ZK)], segv, sem)
            c3.wait()
            c2.wait()
            c1.wait()
            pltpu.async_copy(zinit_hbm.at[idxv], gbuf, sem).wait()

        def masked_block(b):
            base = pl.multiple_of(lo_al + b * ZK, 8)
            load_block(base)
            base16 = jnp.full((16,), base, jnp.int32)

            @plsc.parallel_loop(0, ZK, unroll=4)
            def _(k):
                k16 = jnp.full((16,), k, jnp.int32)
                seg16 = plsc.load_gather(segv, [k16])
                enc16 = plsc.load_gather(encv, [k16])
                gp16 = base16 + k16
                mask = (gp16 >= lo16) & (gp16 < hi16)
                enc_eff = jnp.where(mask, enc16, 0.0)
                row16 = jnp.where(mask, seg16 - segbase16, 0)
                for j in range(4):
                    col = iota16 + 16 * j
                    vals = plsc.load_gather(gbuf, [k16, col])
                    plsc.addupdate_scatter(outbuf, [row16, col], vals * enc_eff)

        masked_block(jnp.int32(0))

        @pl.loop(1, nblk - 1)
        def _(b):
            base = pl.multiple_of(lo_al + b * ZK, 8)
            load_block(base)

            @plsc.parallel_loop(0, ZK, unroll=4)
            def _(k):
                k16 = jnp.full((16,), k, jnp.int32)
                seg16 = plsc.load_gather(segv, [k16])
                enc16 = plsc.load_gather(encv, [k16])
                row16 = seg16 - segbase16
                for j in range(4):
                    col = iota16 + 16 * j
                    vals = plsc.load_gather(gbuf, [k16, col])
                    plsc.addupdate_scatter(outbuf, [row16, col], vals * enc16)

        @pl.when(nblk >= 2)
        def _():
            masked_block(nblk - 1)

        pltpu.sync_copy(outbuf, z0_hbm.at[pl.ds(pl.multiple_of(wid * ES + c * ZCH, 8), ZCH)])


@jax.jit
def _z0_sc(zinit_W, pos_index, pos_enc, pos_batch):
    bnd = jnp.searchsorted(pos_batch, jnp.arange(0, E + 1, ZCH)).astype(jnp.int32)
    bnd = jnp.pad(bnd, (0, 32))
    rows = jnp.arange(NWORK)[:, None] * NZCH + jnp.arange(32)[None, :]
    btab = bnd[rows]
    zpad = jnp.pad(zinit_W, ((0, 0), (0, 64)))
    pi = jnp.pad(pos_index.astype(jnp.int32), (0, ZK))
    pe = jnp.pad(pos_enc, (0, ZK))
    pb = jnp.pad(pos_batch.astype(jnp.int32), (0, ZK))

    mesh = plsc.VectorSubcoreMesh(core_axis_name="c", subcore_axis_name="s",
                                  num_cores=2, num_subcores=16)
    kern = pl.kernel(
        _z0_body,
        out_type=jax.ShapeDtypeStruct((E, 128), jnp.float32),
        mesh=mesh,
        scratch_types=[
            pltpu.VMEM((32,), jnp.int32),
            pltpu.VMEM((ZK,), jnp.int32),
            pltpu.VMEM((ZK,), jnp.float32),
            pltpu.VMEM((ZK,), jnp.int32),
            pltpu.VMEM((ZK, 128), jnp.float32),
            pltpu.VMEM((ZCH, 128), jnp.float32),
            pltpu.SemaphoreType.DMA,
        ],
        compiler_params=_sc_compiler_params(),
    )
    return kern(zpad, pi, pe, pb, btab)


NP = 10240
AK = 200
EPT = E // NWORK
NAB = EPT // AK


def _agg_body(m_hbm, dst_hbm, zeros_hbm, out_hbm, idxv, mbuf, accum, sem):
    cid = lax.axis_index("c")
    sid = lax.axis_index("s")
    wid = cid * 16 + sid

    pltpu.sync_copy(zeros_hbm.at[pl.ds(0, 640)], accum.at[pl.ds(sid * 640, 640)])
    plsc.subcore_barrier()

    @pl.loop(0, NAB)
    def _(b):
        base = pl.multiple_of(wid * EPT + b * AK, 8)
        c1 = pltpu.async_copy(dst_hbm.at[pl.ds(base, AK)], idxv, sem)
        c2 = pltpu.async_copy(m_hbm.at[pl.ds(base, AK)], mbuf, sem)
        c2.wait()
        c1.wait()
        pltpu.sync_copy(mbuf, accum.at[idxv], add=True)

    plsc.subcore_barrier()
    pltpu.sync_copy(accum.at[pl.ds(sid * 640, 640)],
                    out_hbm.at[cid].at[pl.ds(sid * 640, 640)])


@jax.jit
def _agg_sc(m, dst):
    zeros = jnp.zeros((640, 128), jnp.float32)
    mesh = plsc.VectorSubcoreMesh(core_axis_name="c", subcore_axis_name="s",
                                  num_cores=2, num_subcores=16)
    kern = pl.kernel(
        _agg_body,
        out_type=jax.ShapeDtypeStruct((2, NP, 128), jnp.float32),
        mesh=mesh,
        scratch_types=[
            pltpu.VMEM((AK,), jnp.int32),
            pltpu.VMEM((AK, 128), jnp.float32),
            pltpu.VMEM_SHARED((NP, 128), jnp.float32),
            pltpu.SemaphoreType.DMA,
        ],
        compiler_params=_sc_compiler_params(),
    )
    return kern(m, dst.astype(jnp.int32), zeros)


GK = 400
NGB = (E // NWORK) // GK


def _gat_body(h_hbm, src_hbm, hs_hbm, idxv, gbuf, sem):
    wid = lax.axis_index("s") * 2 + lax.axis_index("c")

    @pl.loop(0, NGB)
    def _(b):
        base = pl.multiple_of(wid * (E // NWORK) + b * GK, 8)
        pltpu.sync_copy(src_hbm.at[pl.ds(base, GK)], idxv)
        pltpu.async_copy(h_hbm.at[idxv], gbuf, sem).wait()
        pltpu.sync_copy(gbuf, hs_hbm.at[pl.ds(base, GK)])


@jax.jit
def _gather_sc(h128, src):
    mesh = plsc.VectorSubcoreMesh(core_axis_name="c", subcore_axis_name="s",
                                  num_cores=2, num_subcores=16)
    kern = pl.kernel(
        _gat_body,
        out_type=jax.ShapeDtypeStruct((E, 128), jnp.float32),
        mesh=mesh,
        scratch_types=[
            pltpu.VMEM((GK,), jnp.int32),
            pltpu.VMEM((GK, 128), jnp.float32),
            pltpu.SemaphoreType.DMA,
        ],
        compiler_params=_sc_compiler_params(),
    )
    return kern(h128, src.astype(jnp.int32))


def _stats_body(x_ref, o_ref):
    i = pl.program_id(0)

    @pl.when(i == 0)
    def _():
        o_ref[...] = jnp.zeros_like(o_ref)

    blk = x_ref[...][:, :HID]
    s1 = jnp.sum(blk, axis=0, keepdims=True)
    s2 = jnp.sum(blk * blk, axis=0, keepdims=True)
    o_ref[0:1, :] += s1
    o_ref[1:2, :] += s2


def _col_stats(x, bd):
    d = x.shape[1]
    return pl.pallas_call(
        _stats_body,
        grid=(x.shape[0] // bd,),
        in_specs=[pl.BlockSpec((bd, d), lambda i: (i, 0))],
        out_specs=pl.BlockSpec((8, HID), lambda i: (0, 0)),
        out_shape=jax.ShapeDtypeStruct((8, HID), jnp.float32),
    )(x)


def _bn_coeffs(stats, n, g, b):
    m = stats[0] / n
    v = stats[1] / n - m * m
    s = g / jnp.sqrt(v + 1e-5)
    t = b - m * s
    return s[None, :], t[None, :]


def _ztrans_body(z0_ref, s1_ref, t1_ref, w_ref, b_ref, zp_ref, st_ref):
    i = pl.program_id(0)

    @pl.when(i == 0)
    def _():
        st_ref[...] = jnp.zeros_like(st_ref)

    z1 = jnp.maximum(z0_ref[...][:, :HID] * s1_ref[...] + t1_ref[...], 0.0)
    zp = jnp.dot(z1, w_ref[...], preferred_element_type=jnp.float32) + b_ref[...]
    zp_ref[...] = zp
    st_ref[0:1, :] += jnp.sum(zp, axis=0, keepdims=True)
    st_ref[1:2, :] += jnp.sum(zp * zp, axis=0, keepdims=True)


def _ztrans(z0, s1, t1, w, b):
    return pl.pallas_call(
        _ztrans_body,
        grid=(GRID_E,),
        in_specs=[
            pl.BlockSpec((BE, 128), lambda i: (i, 0)),
            pl.BlockSpec((1, HID), lambda i: (0, 0)),
            pl.BlockSpec((1, HID), lambda i: (0, 0)),
            pl.BlockSpec((HID, HID), lambda i: (0, 0)),
            pl.BlockSpec((1, HID), lambda i: (0, 0)),
        ],
        out_specs=[
            pl.BlockSpec((BE, HID), lambda i: (i, 0)),
            pl.BlockSpec((8, HID), lambda i: (0, 0)),
        ],
        out_shape=[
            jax.ShapeDtypeStruct((E, HID), jnp.float32),
            jax.ShapeDtypeStruct((8, HID), jnp.float32),
        ],
    )(z0, s1, t1, w, b)


def _msg_body(zp_ref, hs_ref, s2_ref, t2_ref, wle_ref, ble_ref, m_ref):
    z2 = jnp.maximum(zp_ref[...] * s2_ref[...] + t2_ref[...], 0.0)
    e = jnp.dot(z2, wle_ref[...], preferred_element_type=jnp.float32) + ble_ref[...]
    m = jnp.maximum(hs_ref[...][:, :HID] + e, 0.0)
    m_ref[...] = jnp.concatenate([m, jnp.zeros_like(m)], axis=1)


def _msg(z2pre, hs, s2, t2, wle, ble):
    return pl.pallas_call(
        _msg_body,
        grid=(GRID_E,),
        in_specs=[
            pl.BlockSpec((BE, HID), lambda i: (i, 0)),
            pl.BlockSpec((BE, 128), lambda i: (i, 0)),
            pl.BlockSpec((1, HID), lambda i: (0, 0)),
            pl.BlockSpec((1, HID), lambda i: (0, 0)),
            pl.BlockSpec((HID, HID), lambda i: (0, 0)),
            pl.BlockSpec((1, HID), lambda i: (0, 0)),
        ],
        out_specs=pl.BlockSpec((BE, 128), lambda i: (i, 0)),
        out_shape=jax.ShapeDtypeStruct((E, 128), jnp.float32),
    )(z2pre, hs, s2, t2, wle, ble)


def _bn_in(x, g, b):
    m = jnp.mean(x, axis=0, keepdims=True)
    v = jnp.mean(x * x, axis=0, keepdims=True) - m * m
    return (x - m) / jnp.sqrt(v + 1e-5) * g + b


def _mlp_in(h, w1, b1, g1, t1, w2, b2, g2, t2):
    h = jnp.dot(h, w1, preferred_element_type=jnp.float32) + b1
    h = jnp.maximum(_bn_in(h, g1, t1), 0.0)
    h = jnp.dot(h, w2, preferred_element_type=jnp.float32) + b2
    return jnp.maximum(_bn_in(h, g2, t2), 0.0)


def _xemb_body(x_ref, w1, b1, g1, t1, w2, b2, g2, t2, o_ref):
    o_ref[...] = _mlp_in(x_ref[...], w1[...], b1[...], g1[...], t1[...],
                         w2[...], b2[...], g2[...], t2[...])


def _xemb(x, p):
    return pl.pallas_call(
        _xemb_body,
        out_shape=jax.ShapeDtypeStruct((N, HID), jnp.float32),
    )(x, p["xemb_l1_W"], p["xemb_l1_b"][None, :], p["xemb_bn1_g"][None, :],
      p["xemb_bn1_bt"][None, :], p["xemb_l2_W"], p["xemb_l2_b"][None, :],
      p["xemb_bn2_g"][None, :], p["xemb_bn2_bt"][None, :])


def _node_body(parts_ref, h_ref, sc_ref, w1, b1, g1, t1, w2, b2, g2, t2, o_ref):
    agg = (parts_ref[0] + parts_ref[1])[:N, :HID]
    h = agg + sc_ref[...] * h_ref[...]
    o_ref[...] = _mlp_in(h, w1[...], b1[...], g1[...], t1[...],
                         w2[...], b2[...], g2[...], t2[...])


def _node_update(parts, h, eps, p, pre):
    return pl.pallas_call(
        _node_body,
        out_shape=jax.ShapeDtypeStruct((N, HID), jnp.float32),
    )(parts, h, (1.0 + eps).reshape(1, 1),
      p[pre + "_l1_W"], p[pre + "_l1_b"][None, :], p[pre + "_bn1_g"][None, :],
      p[pre + "_bn1_bt"][None, :], p[pre + "_l2_W"], p[pre + "_l2_b"][None, :],
      p[pre + "_bn2_g"][None, :], p[pre + "_bn2_bt"][None, :])


def _head_body(x0, x1, x2, x3, b_ref, w1, b1, g1, t1, w2, b2, o_ref):
    hc = jnp.concatenate([x0[...], x1[...], x2[...], x3[...]], axis=1)
    onehot = (b_ref[...] == lax.broadcasted_iota(jnp.int32, (1, NG), 1)
              ).astype(jnp.float32)
    pooled = lax.dot_general(onehot, hc, (((0,), (0,)), ((), ())),
                             preferred_element_type=jnp.float32)
    cnt = jnp.sum(onehot, axis=0)[:, None]
    pooled = pooled / jnp.maximum(cnt, 1.0)
    o = jnp.dot(pooled, w1[...], preferred_element_type=jnp.float32) + b1[...]
    o = jnp.maximum(_bn_in(o, g1[...], t1[...]), 0.0)
    o = jnp.dot(o, w2[...], preferred_element_type=jnp.float32) + b2[...]
    o_ref[...] = jax.nn.log_softmax(o, axis=-1)


def _head(xs, batch, p):
    return pl.pallas_call(
        _head_body,
        out_shape=jax.ShapeDtypeStruct((NG, NC), jnp.float32),
    )(xs[0], xs[1], xs[2], xs[3], batch.astype(jnp.int32)[:, None],
      p["lin1_W"], p["lin1_b"][None, :], p["bn_lin1_g"][None, :],
      p["bn_lin1_bt"][None, :], p["lin2_W"], p["lin2_b"][None, :])


def kernel(x, edge_index, batch, pos_index, pos_enc, pos_batch, zinit_W, zemb_bn1_g, zemb_bn1_bt, zemb_lin_W, zemb_lin_b, zemb_bn2_g, zemb_bn2_bt, xemb_l1_W, xemb_l1_b, xemb_bn1_g, xemb_bn1_bt, xemb_l2_W, xemb_l2_b, xemb_bn2_g, xemb_bn2_bt, c0_le_W, c0_le_b, c0_eps, c0_l1_W, c0_l1_b, c0_bn1_g, c0_bn1_bt, c0_l2_W, c0_l2_b, c0_bn2_g, c0_bn2_bt, c1_le_W, c1_le_b, c1_eps, c1_l1_W, c1_l1_b, c1_bn1_g, c1_bn1_bt, c1_l2_W, c1_l2_b, c1_bn2_g, c1_bn2_bt, c2_le_W, c2_le_b, c2_eps, c2_l1_W, c2_l1_b, c2_bn1_g, c2_bn1_bt, c2_l2_W, c2_l2_b, c2_bn2_g, c2_bn2_bt, lin1_W, lin1_b, bn_lin1_g, bn_lin1_bt, lin2_W, lin2_b):
    p = dict(locals())
    src = edge_index[0]
    dst = edge_index[1]

    z0 = _z0_sc(zinit_W, pos_index, pos_enc, pos_batch)

    st1 = _col_stats(z0, BE)
    s1, t1 = _bn_coeffs(st1, float(E), zemb_bn1_g, zemb_bn1_bt)
    z2pre, st2 = _ztrans(z0, s1, t1, zemb_lin_W, zemb_lin_b[None, :])
    s2, t2 = _bn_coeffs(st2, float(E), zemb_bn2_g, zemb_bn2_bt)

    xs = [_xemb(x, p)]

    h = jnp.pad(x, ((0, 0), (0, HID - IN_DIM)))
    p["c0_le_W"] = jnp.pad(c0_le_W, ((0, 0), (0, HID - IN_DIM)))
    p["c0_le_b"] = jnp.pad(c0_le_b, (0, HID - IN_DIM))
    p["c0_l1_W"] = jnp.pad(c0_l1_W, ((0, HID - IN_DIM), (0, 0)))
    for i in range(NL):
        pre = "c%d" % i
        h128 = jnp.pad(h, ((0, 0), (0, 128 - HID)))
        hs = _gather_sc(h128, src)
        m = _msg(z2pre, hs, s2, t2, p[pre + "_le_W"], p[pre + "_le_b"][None, :])
        parts = _agg_sc(m, dst)
        h = _node_update(parts, h, p[pre + "_eps"], p, pre)
        xs.append(h)

    return _head(xs, batch, p)


# speedup vs baseline: 4.5369x; 1.0062x over previous
import dataclasses
import functools

import jax
import jax.numpy as jnp
from jax import lax
from jax.experimental import pallas as pl
from jax.experimental.pallas import tpu as pltpu
from jax.experimental.pallas import tpu_sc as plsc

N = 10000
E = 320000
P = 640000
HID = 64
NL = 3
NC = 10
NG = 256
ZIN = 1800
IN_DIM = 10

BE = 6400
GRID_E = E // BE

NWORK = 32
ES = E // NWORK
ZCH = 400
NZCH = ES // ZCH
ZK = 512


def _sc_compiler_params():
    cp = pltpu.CompilerParams()
    if "needs_layout_passes" in pltpu.CompilerParams.__dataclass_fields__:
        cp = dataclasses.replace(cp, needs_layout_passes=False)
    return cp


def _z0_body(zinit_hbm, pi_hbm, pe_hbm, pb_hbm, btab_hbm, z0_hbm,
             bvec, idxv, encv, segv, gbuf, outbuf, sem):
    wid = lax.axis_index("s") * 2 + lax.axis_index("c")
    pltpu.sync_copy(btab_hbm.at[wid], bvec)
    iota16 = lax.iota(jnp.int32, 16)
    bv0 = bvec[pl.ds(0, 16)]
    bv1 = bvec[pl.ds(16, 16)]
    zeros16 = jnp.zeros((16,), jnp.float32)

    def bound(j):
        j16 = jnp.full((16,), j, jnp.int32)
        a = lax.reduce_max(jnp.where(iota16 == j16, bv0, -1), (0,))
        b = lax.reduce_max(jnp.where(iota16 == j16 - 16, bv1, -1), (0,))
        return lax.max(a, b)

    @pl.loop(0, NZCH)
    def _(c):
        lo = bound(c)
        hi = bound(c + 1)
        lo_al = lax.bitwise_and(lo, jnp.int32(~7))
        nblk = (hi - lo_al + (ZK - 1)) // ZK
        lo16 = jnp.full((16,), lo, jnp.int32)
        hi16 = jnp.full((16,), hi, jnp.int32)
        segbase16 = jnp.full((16,), wid * ES + c * ZCH, jnp.int32)

        @pl.loop(0, ZCH)
        def _(r):
            r16 = jnp.full((16,), r, jnp.int32)
            for j in range(4):
                plsc.store_scatter(outbuf, [r16, iota16 + 16 * j], zeros16)

        def load_block(base):
            c1 = pltpu.async_copy(pi_hbm.at[pl.ds(base, ZK)], idxv, sem)
            c2 = pltpu.async_copy(pe_hbm.at[pl.ds(base, ZK)], encv, sem)
            c3 = pltpu.async_copy(pb_hbm.at[pl.ds(base, ZK)], segv, sem)
            c3.wait()
            c2.wait()
            c1.wait()
            pltpu.async_copy(zinit_hbm.at[idxv], gbuf, sem).wait()

        def masked_block(b):
            base = pl.multiple_of(lo_al + b * ZK, 8)
            load_block(base)
            base16 = jnp.full((16,), base, jnp.int32)

            @plsc.parallel_loop(0, ZK, unroll=4)
            def _(k):
                k16 = jnp.full((16,), k, jnp.int32)
                seg16 = plsc.load_gather(segv, [k16])
                enc16 = plsc.load_gather(encv, [k16])
                gp16 = base16 + k16
                mask = (gp16 >= lo16) & (gp16 < hi16)
                enc_eff = jnp.where(mask, enc16, 0.0)
                row16 = jnp.where(mask, seg16 - segbase16, 0)
                for j in range(4):
                    col = iota16 + 16 * j
                    vals = plsc.load_gather(gbuf, [k16, col])
                    plsc.addupdate_scatter(outbuf, [row16, col], vals * enc_eff)

        masked_block(jnp.int32(0))

        @pl.loop(1, nblk - 1)
        def _(b):
            base = pl.multiple_of(lo_al + b * ZK, 8)
            load_block(base)

            @plsc.parallel_loop(0, ZK, unroll=4)
            def _(k):
                k16 = jnp.full((16,), k, jnp.int32)
                seg16 = plsc.load_gather(segv, [k16])
                enc16 = plsc.load_gather(encv, [k16])
                row16 = seg16 - segbase16
                for j in range(4):
                    col = iota16 + 16 * j
                    vals = plsc.load_gather(gbuf, [k16, col])
                    plsc.addupdate_scatter(outbuf, [row16, col], vals * enc16)

        @pl.when(nblk >= 2)
        def _():
            masked_block(nblk - 1)

        pltpu.sync_copy(outbuf, z0_hbm.at[pl.ds(pl.multiple_of(wid * ES + c * ZCH, 8), ZCH)])


@jax.jit
def _z0_sc(zinit_W, pos_index, pos_enc, pos_batch):
    bnd = jnp.searchsorted(pos_batch, jnp.arange(0, E + 1, ZCH)).astype(jnp.int32)
    bnd = jnp.pad(bnd, (0, 32))
    rows = jnp.arange(NWORK)[:, None] * NZCH + jnp.arange(32)[None, :]
    btab = bnd[rows]
    zpad = jnp.pad(zinit_W, ((0, 0), (0, 64)))
    pi = jnp.pad(pos_index.astype(jnp.int32), (0, ZK))
    pe = jnp.pad(pos_enc, (0, ZK))
    pb = jnp.pad(pos_batch.astype(jnp.int32), (0, ZK))

    mesh = plsc.VectorSubcoreMesh(core_axis_name="c", subcore_axis_name="s",
                                  num_cores=2, num_subcores=16)
    kern = pl.kernel(
        _z0_body,
        out_type=jax.ShapeDtypeStruct((E, 128), jnp.float32),
        mesh=mesh,
        scratch_types=[
            pltpu.VMEM((32,), jnp.int32),
            pltpu.VMEM((ZK,), jnp.int32),
            pltpu.VMEM((ZK,), jnp.float32),
            pltpu.VMEM((ZK,), jnp.int32),
            pltpu.VMEM((ZK, 128), jnp.float32),
            pltpu.VMEM((ZCH, 128), jnp.float32),
            pltpu.SemaphoreType.DMA,
        ],
        compiler_params=_sc_compiler_params(),
    )
    return kern(zpad, pi, pe, pb, btab)


NP = 10240
AK = 200
EPT = E // NWORK
NAB = EPT // AK


def _agg_body(m_hbm, dst_hbm, zeros_hbm, out_hbm, idxv, mbuf, accum, sem):
    cid = lax.axis_index("c")
    sid = lax.axis_index("s")
    wid = cid * 16 + sid

    pltpu.sync_copy(zeros_hbm.at[pl.ds(0, 640)], accum.at[pl.ds(sid * 640, 640)])
    plsc.subcore_barrier()

    @pl.loop(0, NAB)
    def _(b):
        base = pl.multiple_of(wid * EPT + b * AK, 8)
        c1 = pltpu.async_copy(dst_hbm.at[pl.ds(base, AK)], idxv, sem)
        c2 = pltpu.async_copy(m_hbm.at[pl.ds(base, AK)], mbuf, sem)
        c2.wait()
        c1.wait()
        pltpu.sync_copy(mbuf, accum.at[idxv], add=True)

    plsc.subcore_barrier()
    pltpu.sync_copy(accum.at[pl.ds(sid * 640, 640)],
                    out_hbm.at[cid].at[pl.ds(sid * 640, 640)])


@jax.jit
def _agg_sc(m, dst):
    zeros = jnp.zeros((640, 128), jnp.float32)
    mesh = plsc.VectorSubcoreMesh(core_axis_name="c", subcore_axis_name="s",
                                  num_cores=2, num_subcores=16)
    kern = pl.kernel(
        _agg_body,
        out_type=jax.ShapeDtypeStruct((2, NP, 128), jnp.float32),
        mesh=mesh,
        scratch_types=[
            pltpu.VMEM((AK,), jnp.int32),
            pltpu.VMEM((AK, 128), jnp.float32),
            pltpu.VMEM_SHARED((NP, 128), jnp.float32),
            pltpu.SemaphoreType.DMA,
        ],
        compiler_params=_sc_compiler_params(),
    )
    return kern(m, dst.astype(jnp.int32), zeros)


GK = 400
NGB = (E // NWORK) // GK


def _gat_body(h_hbm, src_hbm, hs_hbm, idx0, idx1, gb0, gb1, sem, semw):
    wid = lax.axis_index("s") * 2 + lax.axis_index("c")
    tbase = wid * (E // NWORK)

    @pl.loop(0, NGB // 2)
    def _(i):
        b0 = pl.multiple_of(tbase + (2 * i) * GK, 8)
        b1 = pl.multiple_of(tbase + (2 * i + 1) * GK, 8)
        pltpu.sync_copy(src_hbm.at[pl.ds(b0, GK)], idx0)
        g0 = pltpu.async_copy(h_hbm.at[idx0], gb0, sem)
        pltpu.sync_copy(src_hbm.at[pl.ds(b1, GK)], idx1)
        g0.wait()
        w0 = pltpu.async_copy(gb0, hs_hbm.at[pl.ds(b0, GK)], semw)
        g1 = pltpu.async_copy(h_hbm.at[idx1], gb1, sem)
        g1.wait()
        w1 = pltpu.async_copy(gb1, hs_hbm.at[pl.ds(b1, GK)], semw)
        w0.wait()
        w1.wait()

    b0 = pl.multiple_of(tbase + (NGB - 1) * GK, 8)
    pltpu.sync_copy(src_hbm.at[pl.ds(b0, GK)], idx0)
    pltpu.async_copy(h_hbm.at[idx0], gb0, sem).wait()
    pltpu.sync_copy(gb0, hs_hbm.at[pl.ds(b0, GK)])


@jax.jit
def _gather_sc(h128, src):
    mesh = plsc.VectorSubcoreMesh(core_axis_name="c", subcore_axis_name="s",
                                  num_cores=2, num_subcores=16)
    kern = pl.kernel(
        _gat_body,
        out_type=jax.ShapeDtypeStruct((E, 128), jnp.float32),
        mesh=mesh,
        scratch_types=[
            pltpu.VMEM((GK,), jnp.int32),
            pltpu.VMEM((GK,), jnp.int32),
            pltpu.VMEM((GK, 128), jnp.float32),
            pltpu.VMEM((GK, 128), jnp.float32),
            pltpu.SemaphoreType.DMA,
            pltpu.SemaphoreType.DMA,
        ],
        compiler_params=_sc_compiler_params(),
    )
    return kern(h128, src.astype(jnp.int32))


def _stats_body(x_ref, o_ref):
    i = pl.program_id(0)

    @pl.when(i == 0)
    def _():
        o_ref[...] = jnp.zeros_like(o_ref)

    blk = x_ref[...][:, :HID]
    s1 = jnp.sum(blk, axis=0, keepdims=True)
    s2 = jnp.sum(blk * blk, axis=0, keepdims=True)
    o_ref[0:1, :] += s1
    o_ref[1:2, :] += s2


def _col_stats(x, bd):
    d = x.shape[1]
    return pl.pallas_call(
        _stats_body,
        grid=(x.shape[0] // bd,),
        in_specs=[pl.BlockSpec((bd, d), lambda i: (i, 0))],
        out_specs=pl.BlockSpec((8, HID), lambda i: (0, 0)),
        out_shape=jax.ShapeDtypeStruct((8, HID), jnp.float32),
    )(x)


def _bn_coeffs(stats, n, g, b):
    m = stats[0] / n
    v = stats[1] / n - m * m
    s = g / jnp.sqrt(v + 1e-5)
    t = b - m * s
    return s[None, :], t[None, :]


def _ztrans_body(z0_ref, s1_ref, t1_ref, w_ref, b_ref, zp_ref, st_ref):
    i = pl.program_id(0)

    @pl.when(i == 0)
    def _():
        st_ref[...] = jnp.zeros_like(st_ref)

    z1 = jnp.maximum(z0_ref[...][:, :HID] * s1_ref[...] + t1_ref[...], 0.0)
    zp = jnp.dot(z1, w_ref[...], preferred_element_type=jnp.float32) + b_ref[...]
    zp_ref[...] = zp
    st_ref[0:1, :] += jnp.sum(zp, axis=0, keepdims=True)
    st_ref[1:2, :] += jnp.sum(zp * zp, axis=0, keepdims=True)


def _ztrans(z0, s1, t1, w, b):
    return pl.pallas_call(
        _ztrans_body,
        grid=(GRID_E,),
        in_specs=[
            pl.BlockSpec((BE, 128), lambda i: (i, 0)),
            pl.BlockSpec((1, HID), lambda i: (0, 0)),
            pl.BlockSpec((1, HID), lambda i: (0, 0)),
            pl.BlockSpec((HID, HID), lambda i: (0, 0)),
            pl.BlockSpec((1, HID), lambda i: (0, 0)),
        ],
        out_specs=[
            pl.BlockSpec((BE, HID), lambda i: (i, 0)),
            pl.BlockSpec((8, HID), lambda i: (0, 0)),
        ],
        out_shape=[
            jax.ShapeDtypeStruct((E, HID), jnp.float32),
            jax.ShapeDtypeStruct((8, HID), jnp.float32),
        ],
    )(z0, s1, t1, w, b)


def _msg_body(zp_ref, hs_ref, s2_ref, t2_ref, wle_ref, ble_ref, m_ref):
    z2 = jnp.maximum(zp_ref[...] * s2_ref[...] + t2_ref[...], 0.0)
    e = jnp.dot(z2, wle_ref[...], preferred_element_type=jnp.float32) + ble_ref[...]
    m = jnp.maximum(hs_ref[...][:, :HID] + e, 0.0)
    m_ref[...] = jnp.concatenate([m, jnp.zeros_like(m)], axis=1)


def _msg(z2pre, hs, s2, t2, wle, ble):
    return pl.pallas_call(
        _msg_body,
        grid=(GRID_E,),
        in_specs=[
            pl.BlockSpec((BE, HID), lambda i: (i, 0)),
            pl.BlockSpec((BE, 128), lambda i: (i, 0)),
            pl.BlockSpec((1, HID), lambda i: (0, 0)),
            pl.BlockSpec((1, HID), lambda i: (0, 0)),
            pl.BlockSpec((HID, HID), lambda i: (0, 0)),
            pl.BlockSpec((1, HID), lambda i: (0, 0)),
        ],
        out_specs=pl.BlockSpec((BE, 128), lambda i: (i, 0)),
        out_shape=jax.ShapeDtypeStruct((E, 128), jnp.float32),
    )(z2pre, hs, s2, t2, wle, ble)


def _bn_in(x, g, b):
    m = jnp.mean(x, axis=0, keepdims=True)
    v = jnp.mean(x * x, axis=0, keepdims=True) - m * m
    return (x - m) / jnp.sqrt(v + 1e-5) * g + b


def _mlp_in(h, w1, b1, g1, t1, w2, b2, g2, t2):
    h = jnp.dot(h, w1, preferred_element_type=jnp.float32) + b1
    h = jnp.maximum(_bn_in(h, g1, t1), 0.0)
    h = jnp.dot(h, w2, preferred_element_type=jnp.float32) + b2
    return jnp.maximum(_bn_in(h, g2, t2), 0.0)


def _xemb_body(x_ref, w1, b1, g1, t1, w2, b2, g2, t2, o_ref):
    o_ref[...] = _mlp_in(x_ref[...], w1[...], b1[...], g1[...], t1[...],
                         w2[...], b2[...], g2[...], t2[...])


def _xemb(x, p):
    return pl.pallas_call(
        _xemb_body,
        out_shape=jax.ShapeDtypeStruct((N, HID), jnp.float32),
    )(x, p["xemb_l1_W"], p["xemb_l1_b"][None, :], p["xemb_bn1_g"][None, :],
      p["xemb_bn1_bt"][None, :], p["xemb_l2_W"], p["xemb_l2_b"][None, :],
      p["xemb_bn2_g"][None, :], p["xemb_bn2_bt"][None, :])


def _node_body(parts_ref, h_ref, sc_ref, w1, b1, g1, t1, w2, b2, g2, t2, o_ref):
    agg = (parts_ref[0] + parts_ref[1])[:N, :HID]
    h = agg + sc_ref[...] * h_ref[...]
    o_ref[...] = _mlp_in(h, w1[...], b1[...], g1[...], t1[...],
                         w2[...], b2[...], g2[...], t2[...])


def _node_update(parts, h, eps, p, pre):
    return pl.pallas_call(
        _node_body,
        out_shape=jax.ShapeDtypeStruct((N, HID), jnp.float32),
    )(parts, h, (1.0 + eps).reshape(1, 1),
      p[pre + "_l1_W"], p[pre + "_l1_b"][None, :], p[pre + "_bn1_g"][None, :],
      p[pre + "_bn1_bt"][None, :], p[pre + "_l2_W"], p[pre + "_l2_b"][None, :],
      p[pre + "_bn2_g"][None, :], p[pre + "_bn2_bt"][None, :])


def _head_body(x0, x1, x2, x3, b_ref, w1, b1, g1, t1, w2, b2, o_ref):
    hc = jnp.concatenate([x0[...], x1[...], x2[...], x3[...]], axis=1)
    onehot = (b_ref[...] == lax.broadcasted_iota(jnp.int32, (1, NG), 1)
              ).astype(jnp.float32)
    pooled = lax.dot_general(onehot, hc, (((0,), (0,)), ((), ())),
                             preferred_element_type=jnp.float32)
    cnt = jnp.sum(onehot, axis=0)[:, None]
    pooled = pooled / jnp.maximum(cnt, 1.0)
    o = jnp.dot(pooled, w1[...], preferred_element_type=jnp.float32) + b1[...]
    o = jnp.maximum(_bn_in(o, g1[...], t1[...]), 0.0)
    o = jnp.dot(o, w2[...], preferred_element_type=jnp.float32) + b2[...]
    o_ref[...] = jax.nn.log_softmax(o, axis=-1)


def _head(xs, batch, p):
    return pl.pallas_call(
        _head_body,
        out_shape=jax.ShapeDtypeStruct((NG, NC), jnp.float32),
    )(xs[0], xs[1], xs[2], xs[3], batch.astype(jnp.int32)[:, None],
      p["lin1_W"], p["lin1_b"][None, :], p["bn_lin1_g"][None, :],
      p["bn_lin1_bt"][None, :], p["lin2_W"], p["lin2_b"][None, :])


def kernel(x, edge_index, batch, pos_index, pos_enc, pos_batch, zinit_W, zemb_bn1_g, zemb_bn1_bt, zemb_lin_W, zemb_lin_b, zemb_bn2_g, zemb_bn2_bt, xemb_l1_W, xemb_l1_b, xemb_bn1_g, xemb_bn1_bt, xemb_l2_W, xemb_l2_b, xemb_bn2_g, xemb_bn2_bt, c0_le_W, c0_le_b, c0_eps, c0_l1_W, c0_l1_b, c0_bn1_g, c0_bn1_bt, c0_l2_W, c0_l2_b, c0_bn2_g, c0_bn2_bt, c1_le_W, c1_le_b, c1_eps, c1_l1_W, c1_l1_b, c1_bn1_g, c1_bn1_bt, c1_l2_W, c1_l2_b, c1_bn2_g, c1_bn2_bt, c2_le_W, c2_le_b, c2_eps, c2_l1_W, c2_l1_b, c2_bn1_g, c2_bn1_bt, c2_l2_W, c2_l2_b, c2_bn2_g, c2_bn2_bt, lin1_W, lin1_b, bn_lin1_g, bn_lin1_bt, lin2_W, lin2_b):
    p = dict(locals())
    src = edge_index[0]
    dst = edge_index[1]

    z0 = _z0_sc(zinit_W, pos_index, pos_enc, pos_batch)

    st1 = _col_stats(z0, BE)
    s1, t1 = _bn_coeffs(st1, float(E), zemb_bn1_g, zemb_bn1_bt)
    z2pre, st2 = _ztrans(z0, s1, t1, zemb_lin_W, zemb_lin_b[None, :])
    s2, t2 = _bn_coeffs(st2, float(E), zemb_bn2_g, zemb_bn2_bt)

    xs = [_xemb(x, p)]

    h = jnp.pad(x, ((0, 0), (0, HID - IN_DIM)))
    p["c0_le_W"] = jnp.pad(c0_le_W, ((0, 0), (0, HID - IN_DIM)))
    p["c0_le_b"] = jnp.pad(c0_le_b, (0, HID - IN_DIM))
    p["c0_l1_W"] = jnp.pad(c0_l1_W, ((0, HID - IN_DIM), (0, 0)))
    for i in range(NL):
        pre = "c%d" % i
        h128 = jnp.pad(h, ((0, 0), (0, 128 - HID)))
        hs = _gather_sc(h128, src)
        m = _msg(z2pre, hs, s2, t2, p[pre + "_le_W"], p[pre + "_le_b"][None, :])
        parts = _agg_sc(m, dst)
        h = _node_update(parts, h, p[pre + "_eps"], p, pre)
        xs.append(h)

    return _head(xs, batch, p)


# speedup vs baseline: 4.7390x; 1.0445x over previous
import dataclasses
import functools

import jax
import jax.numpy as jnp
from jax import lax
from jax.experimental import pallas as pl
from jax.experimental.pallas import tpu as pltpu
from jax.experimental.pallas import tpu_sc as plsc

N = 10000
E = 320000
P = 640000
HID = 64
NL = 3
NC = 10
NG = 256
ZIN = 1800
IN_DIM = 10

BE = 6400
GRID_E = E // BE

NWORK = 32
ES = E // NWORK
ZCH = 400
NZCH = ES // ZCH
ZK = 512


def _sc_compiler_params():
    cp = pltpu.CompilerParams()
    if "needs_layout_passes" in pltpu.CompilerParams.__dataclass_fields__:
        cp = dataclasses.replace(cp, needs_layout_passes=False)
    return cp


def _z0_body(zinit_hbm, pi_hbm, pe_hbm, pb_hbm, btab_hbm, z0_hbm,
             bvec, idxv, encv, segv, gbuf, outbuf, sem):
    wid = lax.axis_index("s") * 2 + lax.axis_index("c")
    pltpu.sync_copy(btab_hbm.at[wid], bvec)
    iota16 = lax.iota(jnp.int32, 16)
    bv0 = bvec[pl.ds(0, 16)]
    bv1 = bvec[pl.ds(16, 16)]
    zeros16 = jnp.zeros((16,), jnp.float32)

    def bound(j):
        j16 = jnp.full((16,), j, jnp.int32)
        a = lax.reduce_max(jnp.where(iota16 == j16, bv0, -1), (0,))
        b = lax.reduce_max(jnp.where(iota16 == j16 - 16, bv1, -1), (0,))
        return lax.max(a, b)

    @pl.loop(0, NZCH)
    def _(c):
        lo = bound(c)
        hi = bound(c + 1)
        lo_al = lax.bitwise_and(lo, jnp.int32(~7))
        nblk = (hi - lo_al + (ZK - 1)) // ZK
        lo16 = jnp.full((16,), lo, jnp.int32)
        hi16 = jnp.full((16,), hi, jnp.int32)
        segbase16 = jnp.full((16,), wid * ES + c * ZCH, jnp.int32)

        @pl.loop(0, ZCH)
        def _(r):
            r16 = jnp.full((16,), r, jnp.int32)
            for j in range(4):
                plsc.store_scatter(outbuf, [r16, iota16 + 16 * j], zeros16)

        def load_block(base):
            c1 = pltpu.async_copy(pi_hbm.at[pl.ds(base, ZK)], idxv, sem)
            c2 = pltpu.async_copy(pe_hbm.at[pl.ds(base, ZK)], encv, sem)
            c3 = pltpu.async_copy(pb_hbm.at[pl.ds(base, ZK)], segv, sem)
            c3.wait()
            c2.wait()
            c1.wait()
            pltpu.async_copy(zinit_hbm.at[idxv], gbuf, sem).wait()

        def masked_block(b):
            base = pl.multiple_of(lo_al + b * ZK, 8)
            load_block(base)
            base16 = jnp.full((16,), base, jnp.int32)

            @plsc.parallel_loop(0, ZK, unroll=4)
            def _(k):
                k16 = jnp.full((16,), k, jnp.int32)
                seg16 = plsc.load_gather(segv, [k16])
                enc16 = plsc.load_gather(encv, [k16])
                gp16 = base16 + k16
                mask = (gp16 >= lo16) & (gp16 < hi16)
                enc_eff = jnp.where(mask, enc16, 0.0)
                row16 = jnp.where(mask, seg16 - segbase16, 0)
                for j in range(4):
                    col = iota16 + 16 * j
                    vals = plsc.load_gather(gbuf, [k16, col])
                    plsc.addupdate_scatter(outbuf, [row16, col], vals * enc_eff)

        masked_block(jnp.int32(0))

        @pl.loop(1, nblk - 1)
        def _(b):
            base = pl.multiple_of(lo_al + b * ZK, 8)
            load_block(base)

            @plsc.parallel_loop(0, ZK, unroll=4)
            def _(k):
                k16 = jnp.full((16,), k, jnp.int32)
                seg16 = plsc.load_gather(segv, [k16])
                enc16 = plsc.load_gather(encv, [k16])
                row16 = seg16 - segbase16
                for j in range(4):
                    col = iota16 + 16 * j
                    vals = plsc.load_gather(gbuf, [k16, col])
                    plsc.addupdate_scatter(outbuf, [row16, col], vals * enc16)

        @pl.when(nblk >= 2)
        def _():
            masked_block(nblk - 1)

        pltpu.sync_copy(outbuf, z0_hbm.at[pl.ds(pl.multiple_of(wid * ES + c * ZCH, 8), ZCH)])


@jax.jit
def _z0_sc(zinit_W, pos_index, pos_enc, pos_batch):
    bnd = jnp.searchsorted(pos_batch, jnp.arange(0, E + 1, ZCH)).astype(jnp.int32)
    bnd = jnp.pad(bnd, (0, 32))
    rows = jnp.arange(NWORK)[:, None] * NZCH + jnp.arange(32)[None, :]
    btab = bnd[rows]
    zpad = jnp.pad(zinit_W, ((0, 0), (0, 64)))
    pi = jnp.pad(pos_index.astype(jnp.int32), (0, ZK))
    pe = jnp.pad(pos_enc, (0, ZK))
    pb = jnp.pad(pos_batch.astype(jnp.int32), (0, ZK))

    mesh = plsc.VectorSubcoreMesh(core_axis_name="c", subcore_axis_name="s",
                                  num_cores=2, num_subcores=16)
    kern = pl.kernel(
        _z0_body,
        out_type=jax.ShapeDtypeStruct((E, 128), jnp.float32),
        mesh=mesh,
        scratch_types=[
            pltpu.VMEM((32,), jnp.int32),
            pltpu.VMEM((ZK,), jnp.int32),
            pltpu.VMEM((ZK,), jnp.float32),
            pltpu.VMEM((ZK,), jnp.int32),
            pltpu.VMEM((ZK, 128), jnp.float32),
            pltpu.VMEM((ZCH, 128), jnp.float32),
            pltpu.SemaphoreType.DMA,
        ],
        compiler_params=_sc_compiler_params(),
    )
    return kern(zpad, pi, pe, pb, btab)


NP = 10240
AK = 200
EPT = E // NWORK
NAB = EPT // AK


def _agg_body(m_hbm, dst_hbm, zeros_hbm, out_hbm, idxv, mbuf, accum, sem):
    cid = lax.axis_index("c")
    sid = lax.axis_index("s")
    wid = cid * 16 + sid

    pltpu.sync_copy(zeros_hbm.at[pl.ds(0, 640)], accum.at[pl.ds(sid * 640, 640)])
    plsc.subcore_barrier()

    @pl.loop(0, NAB)
    def _(b):
        base = pl.multiple_of(wid * EPT + b * AK, 8)
        c1 = pltpu.async_copy(dst_hbm.at[pl.ds(base, AK)], idxv, sem)
        c2 = pltpu.async_copy(m_hbm.at[pl.ds(base, AK)], mbuf, sem)
        c2.wait()
        c1.wait()
        pltpu.sync_copy(mbuf, accum.at[idxv], add=True)

    plsc.subcore_barrier()
    pltpu.sync_copy(accum.at[pl.ds(sid * 640, 640)],
                    out_hbm.at[cid].at[pl.ds(sid * 640, 640)])


@jax.jit
def _agg_sc(m, dst):
    zeros = jnp.zeros((640, 128), jnp.float32)
    mesh = plsc.VectorSubcoreMesh(core_axis_name="c", subcore_axis_name="s",
                                  num_cores=2, num_subcores=16)
    kern = pl.kernel(
        _agg_body,
        out_type=jax.ShapeDtypeStruct((2, NP, 128), jnp.float32),
        mesh=mesh,
        scratch_types=[
            pltpu.VMEM((AK,), jnp.int32),
            pltpu.VMEM((AK, 128), jnp.float32),
            pltpu.VMEM_SHARED((NP, 128), jnp.float32),
            pltpu.SemaphoreType.DMA,
        ],
        compiler_params=_sc_compiler_params(),
    )
    return kern(m, dst.astype(jnp.int32), zeros)


GK = 400
NGB = (E // NWORK) // GK


def _gat_body(h_hbm, src_hbm, hs_hbm, idx0, idx1, gb0, gb1, sem, semw):
    wid = lax.axis_index("s") * 2 + lax.axis_index("c")
    tbase = wid * (E // NWORK)

    @pl.loop(0, NGB // 2)
    def _(i):
        b0 = pl.multiple_of(tbase + (2 * i) * GK, 8)
        b1 = pl.multiple_of(tbase + (2 * i + 1) * GK, 8)
        pltpu.sync_copy(src_hbm.at[pl.ds(b0, GK)], idx0)
        g0 = pltpu.async_copy(h_hbm.at[idx0], gb0, sem)
        pltpu.sync_copy(src_hbm.at[pl.ds(b1, GK)], idx1)
        g0.wait()
        w0 = pltpu.async_copy(gb0, hs_hbm.at[pl.ds(b0, GK)], semw)
        g1 = pltpu.async_copy(h_hbm.at[idx1], gb1, sem)
        g1.wait()
        w1 = pltpu.async_copy(gb1, hs_hbm.at[pl.ds(b1, GK)], semw)
        w0.wait()
        w1.wait()

    b0 = pl.multiple_of(tbase + (NGB - 1) * GK, 8)
    pltpu.sync_copy(src_hbm.at[pl.ds(b0, GK)], idx0)
    pltpu.async_copy(h_hbm.at[idx0], gb0, sem).wait()
    pltpu.sync_copy(gb0, hs_hbm.at[pl.ds(b0, GK)])


@jax.jit
def _gather_sc(h128, src):
    mesh = plsc.VectorSubcoreMesh(core_axis_name="c", subcore_axis_name="s",
                                  num_cores=2, num_subcores=16)
    kern = pl.kernel(
        _gat_body,
        out_type=jax.ShapeDtypeStruct((E, 128), jnp.float32),
        mesh=mesh,
        scratch_types=[
            pltpu.VMEM((GK,), jnp.int32),
            pltpu.VMEM((GK,), jnp.int32),
            pltpu.VMEM((GK, 128), jnp.float32),
            pltpu.VMEM((GK, 128), jnp.float32),
            pltpu.SemaphoreType.DMA,
            pltpu.SemaphoreType.DMA,
        ],
        compiler_params=_sc_compiler_params(),
    )
    return kern(h128, src.astype(jnp.int32))


def _stats_body(x_ref, o_ref):
    i = pl.program_id(0)

    @pl.when(i == 0)
    def _():
        o_ref[...] = jnp.zeros_like(o_ref)

    blk = x_ref[...][:, :HID]
    s1 = jnp.sum(blk, axis=0, keepdims=True)
    s2 = jnp.sum(blk * blk, axis=0, keepdims=True)
    o_ref[0:1, :] += s1
    o_ref[1:2, :] += s2


def _col_stats(x, bd):
    d = x.shape[1]
    return pl.pallas_call(
        _stats_body,
        grid=(x.shape[0] // bd,),
        in_specs=[pl.BlockSpec((bd, d), lambda i: (i, 0))],
        out_specs=pl.BlockSpec((8, HID), lambda i: (0, 0)),
        out_shape=jax.ShapeDtypeStruct((8, HID), jnp.float32),
    )(x)


def _bn_coeffs(stats, n, g, b):
    m = stats[0] / n
    v = stats[1] / n - m * m
    s = g / jnp.sqrt(v + 1e-5)
    t = b - m * s
    return s[None, :], t[None, :]


def _ztrans_body(z0_ref, s1_ref, t1_ref, w_ref, b_ref, zp_ref, st_ref):
    i = pl.program_id(0)

    @pl.when(i == 0)
    def _():
        st_ref[...] = jnp.zeros_like(st_ref)

    z1 = jnp.maximum(z0_ref[...][:, :HID] * s1_ref[...] + t1_ref[...], 0.0)
    zp = jnp.dot(z1, w_ref[...], preferred_element_type=jnp.float32) + b_ref[...]
    zp_ref[...] = zp
    st_ref[0:1, :] += jnp.sum(zp, axis=0, keepdims=True)
    st_ref[1:2, :] += jnp.sum(zp * zp, axis=0, keepdims=True)


def _ztrans(z0, s1, t1, w, b):
    return pl.pallas_call(
        _ztrans_body,
        grid=(GRID_E,),
        in_specs=[
            pl.BlockSpec((BE, 128), lambda i: (i, 0)),
            pl.BlockSpec((1, HID), lambda i: (0, 0)),
            pl.BlockSpec((1, HID), lambda i: (0, 0)),
            pl.BlockSpec((HID, HID), lambda i: (0, 0)),
            pl.BlockSpec((1, HID), lambda i: (0, 0)),
        ],
        out_specs=[
            pl.BlockSpec((BE, HID), lambda i: (i, 0)),
            pl.BlockSpec((8, HID), lambda i: (0, 0)),
        ],
        out_shape=[
            jax.ShapeDtypeStruct((E, HID), jnp.float32),
            jax.ShapeDtypeStruct((8, HID), jnp.float32),
        ],
    )(z0, s1, t1, w, b)


def _msg_body(zp_ref, hs_ref, s2_ref, t2_ref, wle_ref, ble_ref, m_ref):
    z2 = jnp.maximum(zp_ref[...] * s2_ref[...] + t2_ref[...], 0.0)
    e = jnp.dot(z2, wle_ref[...], preferred_element_type=jnp.float32) + ble_ref[...]
    m = jnp.maximum(hs_ref[...][:, :HID] + e, 0.0)
    m_ref[...] = jnp.concatenate([m, jnp.zeros_like(m)], axis=1)


def _msg(z2pre, hs, s2, t2, wle, ble):
    return pl.pallas_call(
        _msg_body,
        grid=(GRID_E,),
        in_specs=[
            pl.BlockSpec((BE, HID), lambda i: (i, 0)),
            pl.BlockSpec((BE, 128), lambda i: (i, 0)),
            pl.BlockSpec((1, HID), lambda i: (0, 0)),
            pl.BlockSpec((1, HID), lambda i: (0, 0)),
            pl.BlockSpec((HID, HID), lambda i: (0, 0)),
            pl.BlockSpec((1, HID), lambda i: (0, 0)),
        ],
        out_specs=pl.BlockSpec((BE, 128), lambda i: (i, 0)),
        out_shape=jax.ShapeDtypeStruct((E, 128), jnp.float32),
    )(z2pre, hs, s2, t2, wle, ble)


LK = 80
NLB = (E // NWORK) // LK


def _layer_body(e_hbm, h_hbm, src_hbm, dst_hbm, zeros_hbm, out_hbm,
                sv0, sv1, dv0, dv1, eb0, eb1, gb0, gb1, accum, sem, semg):
    cid = lax.axis_index("c")
    sid = lax.axis_index("s")
    wid = cid * 16 + sid
    tbase = wid * (E // NWORK)
    iota16 = lax.iota(jnp.int32, 16)

    pltpu.sync_copy(zeros_hbm.at[pl.ds(0, 640)], accum.at[pl.ds(sid * 640, 640)])
    plsc.subcore_barrier()

    def fire3(base, sv, dv, eb):
        c1 = pltpu.async_copy(src_hbm.at[pl.ds(base, LK)], sv, sem)
        c2 = pltpu.async_copy(dst_hbm.at[pl.ds(base, LK)], dv, sem)
        c3 = pltpu.async_copy(e_hbm.at[pl.ds(base, LK)], eb, sem)
        return (c1, c2, c3)

    def drain3(cs):
        cs[2].wait()
        cs[1].wait()
        cs[0].wait()

    def compute_m(gb, eb):
        @plsc.parallel_loop(0, LK, unroll=4)
        def _(r):
            r16 = jnp.full((16,), r, jnp.int32)
            for j in range(4):
                col = iota16 + 16 * j
                hv = plsc.load_gather(gb, [r16, col])
                ev = plsc.load_gather(eb, [r16, col])
                plsc.store_scatter(gb, [r16, col],
                                   jnp.maximum(hv + ev, 0.0))

    @pl.loop(0, NLB // 2)
    def _(i):
        b0 = pl.multiple_of(tbase + (2 * i) * LK, 8)
        b1 = pl.multiple_of(tbase + (2 * i + 1) * LK, 8)
        cs0 = fire3(b0, sv0, dv0, eb0)
        drain3(cs0)
        g0 = pltpu.async_copy(h_hbm.at[sv0], gb0, semg)
        cs1 = fire3(b1, sv1, dv1, eb1)
        g0.wait()
        compute_m(gb0, eb0)
        drain3(cs1)
        g1 = pltpu.async_copy(h_hbm.at[sv1], gb1, semg)
        pltpu.sync_copy(gb0, accum.at[dv0], add=True)
        g1.wait()
        compute_m(gb1, eb1)
        pltpu.sync_copy(gb1, accum.at[dv1], add=True)

    b0 = pl.multiple_of(tbase + (NLB - 1) * LK, 8)
    cs0 = fire3(b0, sv0, dv0, eb0)
    drain3(cs0)
    pltpu.async_copy(h_hbm.at[sv0], gb0, semg).wait()
    compute_m(gb0, eb0)
    pltpu.sync_copy(gb0, accum.at[dv0], add=True)

    plsc.subcore_barrier()
    pltpu.sync_copy(accum.at[pl.ds(sid * 640, 640)],
                    out_hbm.at[cid].at[pl.ds(sid * 640, 640)])


@jax.jit
def _layer_sc(e, h128, src, dst):
    zeros = jnp.zeros((640, 128), jnp.float32)
    mesh = plsc.VectorSubcoreMesh(core_axis_name="c", subcore_axis_name="s",
                                  num_cores=2, num_subcores=16)
    kern = pl.kernel(
        _layer_body,
        out_type=jax.ShapeDtypeStruct((2, NP, 128), jnp.float32),
        mesh=mesh,
        scratch_types=[
            pltpu.VMEM((LK,), jnp.int32),
            pltpu.VMEM((LK,), jnp.int32),
            pltpu.VMEM((LK,), jnp.int32),
            pltpu.VMEM((LK,), jnp.int32),
            pltpu.VMEM((LK, 128), jnp.float32),
            pltpu.VMEM((LK, 128), jnp.float32),
            pltpu.VMEM((LK, 128), jnp.float32),
            pltpu.VMEM((LK, 128), jnp.float32),
            pltpu.VMEM_SHARED((NP, 128), jnp.float32),
            pltpu.SemaphoreType.DMA,
            pltpu.SemaphoreType.DMA,
        ],
        compiler_params=_sc_compiler_params(),
    )
    return kern(e, h128, src.astype(jnp.int32), dst.astype(jnp.int32), zeros)


def _edge_e_body(zp_ref, s2_ref, t2_ref, wle_ref, ble_ref, e_ref):
    z2 = jnp.maximum(zp_ref[...] * s2_ref[...] + t2_ref[...], 0.0)
    e = jnp.dot(z2, wle_ref[...], preferred_element_type=jnp.float32) + ble_ref[...]
    e_ref[...] = jnp.concatenate([e, jnp.zeros_like(e)], axis=1)


def _edge_e(z2pre, s2, t2, wle, ble):
    return pl.pallas_call(
        _edge_e_body,
        grid=(GRID_E,),
        in_specs=[
            pl.BlockSpec((BE, HID), lambda i: (i, 0)),
            pl.BlockSpec((1, HID), lambda i: (0, 0)),
            pl.BlockSpec((1, HID), lambda i: (0, 0)),
            pl.BlockSpec((HID, HID), lambda i: (0, 0)),
            pl.BlockSpec((1, HID), lambda i: (0, 0)),
        ],
        out_specs=pl.BlockSpec((BE, 128), lambda i: (i, 0)),
        out_shape=jax.ShapeDtypeStruct((E, 128), jnp.float32),
    )(z2pre, s2, t2, wle, ble)


def _bn_in(x, g, b):
    m = jnp.mean(x, axis=0, keepdims=True)
    v = jnp.mean(x * x, axis=0, keepdims=True) - m * m
    return (x - m) / jnp.sqrt(v + 1e-5) * g + b


def _mlp_in(h, w1, b1, g1, t1, w2, b2, g2, t2):
    h = jnp.dot(h, w1, preferred_element_type=jnp.float32) + b1
    h = jnp.maximum(_bn_in(h, g1, t1), 0.0)
    h = jnp.dot(h, w2, preferred_element_type=jnp.float32) + b2
    return jnp.maximum(_bn_in(h, g2, t2), 0.0)


def _xemb_body(x_ref, w1, b1, g1, t1, w2, b2, g2, t2, o_ref):
    o_ref[...] = _mlp_in(x_ref[...], w1[...], b1[...], g1[...], t1[...],
                         w2[...], b2[...], g2[...], t2[...])


def _xemb(x, p):
    return pl.pallas_call(
        _xemb_body,
        out_shape=jax.ShapeDtypeStruct((N, HID), jnp.float32),
    )(x, p["xemb_l1_W"], p["xemb_l1_b"][None, :], p["xemb_bn1_g"][None, :],
      p["xemb_bn1_bt"][None, :], p["xemb_l2_W"], p["xemb_l2_b"][None, :],
      p["xemb_bn2_g"][None, :], p["xemb_bn2_bt"][None, :])


def _node_body(parts_ref, h_ref, sc_ref, w1, b1, g1, t1, w2, b2, g2, t2, o_ref):
    agg = (parts_ref[0] + parts_ref[1])[:N, :HID]
    h = agg + sc_ref[...] * h_ref[...]
    o_ref[...] = _mlp_in(h, w1[...], b1[...], g1[...], t1[...],
                         w2[...], b2[...], g2[...], t2[...])


def _node_update(parts, h, eps, p, pre):
    return pl.pallas_call(
        _node_body,
        out_shape=jax.ShapeDtypeStruct((N, HID), jnp.float32),
    )(parts, h, (1.0 + eps).reshape(1, 1),
      p[pre + "_l1_W"], p[pre + "_l1_b"][None, :], p[pre + "_bn1_g"][None, :],
      p[pre + "_bn1_bt"][None, :], p[pre + "_l2_W"], p[pre + "_l2_b"][None, :],
      p[pre + "_bn2_g"][None, :], p[pre + "_bn2_bt"][None, :])


def _head_body(x0, x1, x2, x3, b_ref, w1, b1, g1, t1, w2, b2, o_ref):
    hc = jnp.concatenate([x0[...], x1[...], x2[...], x3[...]], axis=1)
    onehot = (b_ref[...] == lax.broadcasted_iota(jnp.int32, (1, NG), 1)
              ).astype(jnp.float32)
    pooled = lax.dot_general(onehot, hc, (((0,), (0,)), ((), ())),
                             preferred_element_type=jnp.float32)
    cnt = jnp.sum(onehot, axis=0)[:, None]
    pooled = pooled / jnp.maximum(cnt, 1.0)
    o = jnp.dot(pooled, w1[...], preferred_element_type=jnp.float32) + b1[...]
    o = jnp.maximum(_bn_in(o, g1[...], t1[...]), 0.0)
    o = jnp.dot(o, w2[...], preferred_element_type=jnp.float32) + b2[...]
    o_ref[...] = jax.nn.log_softmax(o, axis=-1)


def _head(xs, batch, p):
    return pl.pallas_call(
        _head_body,
        out_shape=jax.ShapeDtypeStruct((NG, NC), jnp.float32),
    )(xs[0], xs[1], xs[2], xs[3], batch.astype(jnp.int32)[:, None],
      p["lin1_W"], p["lin1_b"][None, :], p["bn_lin1_g"][None, :],
      p["bn_lin1_bt"][None, :], p["lin2_W"], p["lin2_b"][None, :])


def kernel(x, edge_index, batch, pos_index, pos_enc, pos_batch, zinit_W, zemb_bn1_g, zemb_bn1_bt, zemb_lin_W, zemb_lin_b, zemb_bn2_g, zemb_bn2_bt, xemb_l1_W, xemb_l1_b, xemb_bn1_g, xemb_bn1_bt, xemb_l2_W, xemb_l2_b, xemb_bn2_g, xemb_bn2_bt, c0_le_W, c0_le_b, c0_eps, c0_l1_W, c0_l1_b, c0_bn1_g, c0_bn1_bt, c0_l2_W, c0_l2_b, c0_bn2_g, c0_bn2_bt, c1_le_W, c1_le_b, c1_eps, c1_l1_W, c1_l1_b, c1_bn1_g, c1_bn1_bt, c1_l2_W, c1_l2_b, c1_bn2_g, c1_bn2_bt, c2_le_W, c2_le_b, c2_eps, c2_l1_W, c2_l1_b, c2_bn1_g, c2_bn1_bt, c2_l2_W, c2_l2_b, c2_bn2_g, c2_bn2_bt, lin1_W, lin1_b, bn_lin1_g, bn_lin1_bt, lin2_W, lin2_b):
    p = dict(locals())
    src = edge_index[0]
    dst = edge_index[1]

    z0 = _z0_sc(zinit_W, pos_index, pos_enc, pos_batch)

    st1 = _col_stats(z0, BE)
    s1, t1 = _bn_coeffs(st1, float(E), zemb_bn1_g, zemb_bn1_bt)
    z2pre, st2 = _ztrans(z0, s1, t1, zemb_lin_W, zemb_lin_b[None, :])
    s2, t2 = _bn_coeffs(st2, float(E), zemb_bn2_g, zemb_bn2_bt)

    xs = [_xemb(x, p)]

    h = jnp.pad(x, ((0, 0), (0, HID - IN_DIM)))
    p["c0_le_W"] = jnp.pad(c0_le_W, ((0, 0), (0, HID - IN_DIM)))
    p["c0_le_b"] = jnp.pad(c0_le_b, (0, HID - IN_DIM))
    p["c0_l1_W"] = jnp.pad(c0_l1_W, ((0, HID - IN_DIM), (0, 0)))
    for i in range(NL):
        pre = "c%d" % i
        h128 = jnp.pad(h, ((0, 0), (0, 128 - HID)))
        e = _edge_e(z2pre, s2, t2, p[pre + "_le_W"], p[pre + "_le_b"][None, :])
        parts = _layer_sc(e, h128, src, dst)
        h = _node_update(parts, h, p[pre + "_eps"], p, pre)
        xs.append(h)

    return _head(xs, batch, p)


# speedup vs baseline: 4.8606x; 1.0257x over previous
import dataclasses
import functools

import jax
import jax.numpy as jnp
from jax import lax
from jax.experimental import pallas as pl
from jax.experimental.pallas import tpu as pltpu
from jax.experimental.pallas import tpu_sc as plsc

N = 10000
E = 320000
P = 640000
HID = 64
NL = 3
NC = 10
NG = 256
ZIN = 1800
IN_DIM = 10

BE = 6400
GRID_E = E // BE

NWORK = 32
ES = E // NWORK
ZCH = 400
NZCH = ES // ZCH
ZK = 512


def _sc_compiler_params():
    cp = pltpu.CompilerParams()
    if "needs_layout_passes" in pltpu.CompilerParams.__dataclass_fields__:
        cp = dataclasses.replace(cp, needs_layout_passes=False)
    return cp


def _z0_body(zinit_hbm, pi_hbm, pe_hbm, pb_hbm, btab_hbm, z0_hbm,
             bvec, idxv, encv, segv, gbuf, outbuf, sem):
    wid = lax.axis_index("s") * 2 + lax.axis_index("c")
    pltpu.sync_copy(btab_hbm.at[wid], bvec)
    iota16 = lax.iota(jnp.int32, 16)
    bv0 = bvec[pl.ds(0, 16)]
    bv1 = bvec[pl.ds(16, 16)]
    zeros16 = jnp.zeros((16,), jnp.float32)

    def bound(j):
        j16 = jnp.full((16,), j, jnp.int32)
        a = lax.reduce_max(jnp.where(iota16 == j16, bv0, -1), (0,))
        b = lax.reduce_max(jnp.where(iota16 == j16 - 16, bv1, -1), (0,))
        return lax.max(a, b)

    @pl.loop(0, NZCH)
    def _(c):
        lo = bound(c)
        hi = bound(c + 1)
        lo_al = lax.bitwise_and(lo, jnp.int32(~7))
        nblk = (hi - lo_al + (ZK - 1)) // ZK
        lo16 = jnp.full((16,), lo, jnp.int32)
        hi16 = jnp.full((16,), hi, jnp.int32)
        segbase16 = jnp.full((16,), wid * ES + c * ZCH, jnp.int32)

        @pl.loop(0, ZCH)
        def _(r):
            r16 = jnp.full((16,), r, jnp.int32)
            for j in range(4):
                plsc.store_scatter(outbuf, [r16, iota16 + 16 * j], zeros16)

        def load_block(base):
            c1 = pltpu.async_copy(pi_hbm.at[pl.ds(base, ZK)], idxv, sem)
            c2 = pltpu.async_copy(pe_hbm.at[pl.ds(base, ZK)], encv, sem)
            c3 = pltpu.async_copy(pb_hbm.at[pl.ds(base, ZK)], segv, sem)
            c3.wait()
            c2.wait()
            c1.wait()
            pltpu.async_copy(zinit_hbm.at[idxv], gbuf, sem).wait()

        def masked_block(b):
            base = pl.multiple_of(lo_al + b * ZK, 8)
            load_block(base)
            base16 = jnp.full((16,), base, jnp.int32)

            @plsc.parallel_loop(0, ZK, unroll=4)
            def _(k):
                k16 = jnp.full((16,), k, jnp.int32)
                seg16 = plsc.load_gather(segv, [k16])
                enc16 = plsc.load_gather(encv, [k16])
                gp16 = base16 + k16
                mask = (gp16 >= lo16) & (gp16 < hi16)
                enc_eff = jnp.where(mask, enc16, 0.0)
                row16 = jnp.where(mask, seg16 - segbase16, 0)
                for j in range(4):
                    col = iota16 + 16 * j
                    vals = plsc.load_gather(gbuf, [k16, col])
                    plsc.addupdate_scatter(outbuf, [row16, col], vals * enc_eff)

        masked_block(jnp.int32(0))

        @pl.loop(1, nblk - 1)
        def _(b):
            base = pl.multiple_of(lo_al + b * ZK, 8)
            load_block(base)

            @plsc.parallel_loop(0, ZK, unroll=4)
            def _(k):
                k16 = jnp.full((16,), k, jnp.int32)
                seg16 = plsc.load_gather(segv, [k16])
                enc16 = plsc.load_gather(encv, [k16])
                row16 = seg16 - segbase16
                for j in range(4):
                    col = iota16 + 16 * j
                    vals = plsc.load_gather(gbuf, [k16, col])
                    plsc.addupdate_scatter(outbuf, [row16, col], vals * enc16)

        @pl.when(nblk >= 2)
        def _():
            masked_block(nblk - 1)

        pltpu.sync_copy(outbuf, z0_hbm.at[pl.ds(pl.multiple_of(wid * ES + c * ZCH, 8), ZCH)])


@jax.jit
def _z0_sc(zinit_W, pos_index, pos_enc, pos_batch):
    bnd = jnp.searchsorted(pos_batch, jnp.arange(0, E + 1, ZCH)).astype(jnp.int32)
    bnd = jnp.pad(bnd, (0, 32))
    rows = jnp.arange(NWORK)[:, None] * NZCH + jnp.arange(32)[None, :]
    btab = bnd[rows]
    zpad = jnp.pad(zinit_W, ((0, 0), (0, 64)))
    pi = jnp.pad(pos_index.astype(jnp.int32), (0, ZK))
    pe = jnp.pad(pos_enc, (0, ZK))
    pb = jnp.pad(pos_batch.astype(jnp.int32), (0, ZK))

    mesh = plsc.VectorSubcoreMesh(core_axis_name="c", subcore_axis_name="s",
                                  num_cores=2, num_subcores=16)
    kern = pl.kernel(
        _z0_body,
        out_type=jax.ShapeDtypeStruct((E, 128), jnp.float32),
        mesh=mesh,
        scratch_types=[
            pltpu.VMEM((32,), jnp.int32),
            pltpu.VMEM((ZK,), jnp.int32),
            pltpu.VMEM((ZK,), jnp.float32),
            pltpu.VMEM((ZK,), jnp.int32),
            pltpu.VMEM((ZK, 128), jnp.float32),
            pltpu.VMEM((ZCH, 128), jnp.float32),
            pltpu.SemaphoreType.DMA,
        ],
        compiler_params=_sc_compiler_params(),
    )
    return kern(zpad, pi, pe, pb, btab)


NP = 10240
AK = 200
EPT = E // NWORK
NAB = EPT // AK


def _agg_body(m_hbm, dst_hbm, zeros_hbm, out_hbm, idxv, mbuf, accum, sem):
    cid = lax.axis_index("c")
    sid = lax.axis_index("s")
    wid = cid * 16 + sid

    pltpu.sync_copy(zeros_hbm.at[pl.ds(0, 640)], accum.at[pl.ds(sid * 640, 640)])
    plsc.subcore_barrier()

    @pl.loop(0, NAB)
    def _(b):
        base = pl.multiple_of(wid * EPT + b * AK, 8)
        c1 = pltpu.async_copy(dst_hbm.at[pl.ds(base, AK)], idxv, sem)
        c2 = pltpu.async_copy(m_hbm.at[pl.ds(base, AK)], mbuf, sem)
        c2.wait()
        c1.wait()
        pltpu.sync_copy(mbuf, accum.at[idxv], add=True)

    plsc.subcore_barrier()
    pltpu.sync_copy(accum.at[pl.ds(sid * 640, 640)],
                    out_hbm.at[cid].at[pl.ds(sid * 640, 640)])


@jax.jit
def _agg_sc(m, dst):
    zeros = jnp.zeros((640, 128), jnp.float32)
    mesh = plsc.VectorSubcoreMesh(core_axis_name="c", subcore_axis_name="s",
                                  num_cores=2, num_subcores=16)
    kern = pl.kernel(
        _agg_body,
        out_type=jax.ShapeDtypeStruct((2, NP, 128), jnp.float32),
        mesh=mesh,
        scratch_types=[
            pltpu.VMEM((AK,), jnp.int32),
            pltpu.VMEM((AK, 128), jnp.float32),
            pltpu.VMEM_SHARED((NP, 128), jnp.float32),
            pltpu.SemaphoreType.DMA,
        ],
        compiler_params=_sc_compiler_params(),
    )
    return kern(m, dst.astype(jnp.int32), zeros)


GK = 400
NGB = (E // NWORK) // GK


def _gat_body(h_hbm, src_hbm, hs_hbm, idx0, idx1, gb0, gb1, sem, semw):
    wid = lax.axis_index("s") * 2 + lax.axis_index("c")
    tbase = wid * (E // NWORK)

    @pl.loop(0, NGB // 2)
    def _(i):
        b0 = pl.multiple_of(tbase + (2 * i) * GK, 8)
        b1 = pl.multiple_of(tbase + (2 * i + 1) * GK, 8)
        pltpu.sync_copy(src_hbm.at[pl.ds(b0, GK)], idx0)
        g0 = pltpu.async_copy(h_hbm.at[idx0], gb0, sem)
        pltpu.sync_copy(src_hbm.at[pl.ds(b1, GK)], idx1)
        g0.wait()
        w0 = pltpu.async_copy(gb0, hs_hbm.at[pl.ds(b0, GK)], semw)
        g1 = pltpu.async_copy(h_hbm.at[idx1], gb1, sem)
        g1.wait()
        w1 = pltpu.async_copy(gb1, hs_hbm.at[pl.ds(b1, GK)], semw)
        w0.wait()
        w1.wait()

    b0 = pl.multiple_of(tbase + (NGB - 1) * GK, 8)
    pltpu.sync_copy(src_hbm.at[pl.ds(b0, GK)], idx0)
    pltpu.async_copy(h_hbm.at[idx0], gb0, sem).wait()
    pltpu.sync_copy(gb0, hs_hbm.at[pl.ds(b0, GK)])


@jax.jit
def _gather_sc(h128, src):
    mesh = plsc.VectorSubcoreMesh(core_axis_name="c", subcore_axis_name="s",
                                  num_cores=2, num_subcores=16)
    kern = pl.kernel(
        _gat_body,
        out_type=jax.ShapeDtypeStruct((E, 128), jnp.float32),
        mesh=mesh,
        scratch_types=[
            pltpu.VMEM((GK,), jnp.int32),
            pltpu.VMEM((GK,), jnp.int32),
            pltpu.VMEM((GK, 128), jnp.float32),
            pltpu.VMEM((GK, 128), jnp.float32),
            pltpu.SemaphoreType.DMA,
            pltpu.SemaphoreType.DMA,
        ],
        compiler_params=_sc_compiler_params(),
    )
    return kern(h128, src.astype(jnp.int32))


def _stats_body(x_ref, o_ref):
    i = pl.program_id(0)

    @pl.when(i == 0)
    def _():
        o_ref[...] = jnp.zeros_like(o_ref)

    blk = x_ref[...][:, :HID]
    s1 = jnp.sum(blk, axis=0, keepdims=True)
    s2 = jnp.sum(blk * blk, axis=0, keepdims=True)
    o_ref[0:1, :] += s1
    o_ref[1:2, :] += s2


def _col_stats(x, bd):
    d = x.shape[1]
    return pl.pallas_call(
        _stats_body,
        grid=(x.shape[0] // bd,),
        in_specs=[pl.BlockSpec((bd, d), lambda i: (i, 0))],
        out_specs=pl.BlockSpec((8, HID), lambda i: (0, 0)),
        out_shape=jax.ShapeDtypeStruct((8, HID), jnp.float32),
    )(x)


def _bn_coeffs(stats, n, g, b):
    m = stats[0] / n
    v = stats[1] / n - m * m
    s = g / jnp.sqrt(v + 1e-5)
    t = b - m * s
    return s[None, :], t[None, :]


def _ztrans_body(z0_ref, s1_ref, t1_ref, w_ref, b_ref, zp_ref, st_ref):
    i = pl.program_id(0)

    @pl.when(i == 0)
    def _():
        st_ref[...] = jnp.zeros_like(st_ref)

    z1 = jnp.maximum(z0_ref[...][:, :HID] * s1_ref[...] + t1_ref[...], 0.0)
    zp = jnp.dot(z1, w_ref[...], preferred_element_type=jnp.float32) + b_ref[...]
    zp_ref[...] = zp
    st_ref[0:1, :] += jnp.sum(zp, axis=0, keepdims=True)
    st_ref[1:2, :] += jnp.sum(zp * zp, axis=0, keepdims=True)


def _ztrans(z0, s1, t1, w, b):
    return pl.pallas_call(
        _ztrans_body,
        grid=(GRID_E,),
        in_specs=[
            pl.BlockSpec((BE, 128), lambda i: (i, 0)),
            pl.BlockSpec((1, HID), lambda i: (0, 0)),
            pl.BlockSpec((1, HID), lambda i: (0, 0)),
            pl.BlockSpec((HID, HID), lambda i: (0, 0)),
            pl.BlockSpec((1, HID), lambda i: (0, 0)),
        ],
        out_specs=[
            pl.BlockSpec((BE, HID), lambda i: (i, 0)),
            pl.BlockSpec((8, HID), lambda i: (0, 0)),
        ],
        out_shape=[
            jax.ShapeDtypeStruct((E, HID), jnp.float32),
            jax.ShapeDtypeStruct((8, HID), jnp.float32),
        ],
    )(z0, s1, t1, w, b)


def _msg_body(zp_ref, hs_ref, s2_ref, t2_ref, wle_ref, ble_ref, m_ref):
    z2 = jnp.maximum(zp_ref[...] * s2_ref[...] + t2_ref[...], 0.0)
    e = jnp.dot(z2, wle_ref[...], preferred_element_type=jnp.float32) + ble_ref[...]
    m = jnp.maximum(hs_ref[...][:, :HID] + e, 0.0)
    m_ref[...] = jnp.concatenate([m, jnp.zeros_like(m)], axis=1)


def _msg(z2pre, hs, s2, t2, wle, ble):
    return pl.pallas_call(
        _msg_body,
        grid=(GRID_E,),
        in_specs=[
            pl.BlockSpec((BE, HID), lambda i: (i, 0)),
            pl.BlockSpec((BE, 128), lambda i: (i, 0)),
            pl.BlockSpec((1, HID), lambda i: (0, 0)),
            pl.BlockSpec((1, HID), lambda i: (0, 0)),
            pl.BlockSpec((HID, HID), lambda i: (0, 0)),
            pl.BlockSpec((1, HID), lambda i: (0, 0)),
        ],
        out_specs=pl.BlockSpec((BE, 128), lambda i: (i, 0)),
        out_shape=jax.ShapeDtypeStruct((E, 128), jnp.float32),
    )(z2pre, hs, s2, t2, wle, ble)


LK = 80
NLB = (E // NWORK) // LK


def _layer_body(e_hbm, h_hbm, src_hbm, dst_hbm, zeros_hbm, out_hbm,
                sv0, sv1, dv0, dv1, eb0, eb1, gb0, gb1, accum, sem, semg):
    cid = lax.axis_index("c")
    sid = lax.axis_index("s")
    wid = cid * 16 + sid
    tbase = wid * (E // NWORK)
    iota16 = lax.iota(jnp.int32, 16)

    pltpu.sync_copy(zeros_hbm.at[pl.ds(0, 640)], accum.at[pl.ds(sid * 640, 640)])
    plsc.subcore_barrier()

    def fire3(base, sv, dv, eb):
        c1 = pltpu.async_copy(src_hbm.at[pl.ds(base, LK)], sv, sem)
        c2 = pltpu.async_copy(dst_hbm.at[pl.ds(base, LK)], dv, sem)
        c3 = pltpu.async_copy(e_hbm.at[pl.ds(base, LK)], eb, sem)
        return (c1, c2, c3)

    def drain3(cs):
        cs[2].wait()
        cs[1].wait()
        cs[0].wait()

    def compute_m(gb, eb):
        @plsc.parallel_loop(0, LK, unroll=4)
        def _(r):
            r16 = jnp.full((16,), r, jnp.int32)
            for j in range(4):
                col = iota16 + 16 * j
                hv = plsc.load_gather(gb, [r16, col])
                ev = plsc.load_gather(eb, [r16, col])
                plsc.store_scatter(gb, [r16, col],
                                   jnp.maximum(hv + ev, 0.0))

    @pl.loop(0, NLB // 2)
    def _(i):
        b0 = pl.multiple_of(tbase + (2 * i) * LK, 8)
        b1 = pl.multiple_of(tbase + (2 * i + 1) * LK, 8)

        @pl.when(i > 0)
        def _():
            pltpu.make_async_copy(gb0, accum.at[dv0], semg).wait()
            pltpu.make_async_copy(gb1, accum.at[dv1], semg).wait()

        cs0 = fire3(b0, sv0, dv0, eb0)
        drain3(cs0)
        g0 = pltpu.async_copy(h_hbm.at[sv0], gb0, sem)
        cs1 = fire3(b1, sv1, dv1, eb1)
        g0.wait()
        compute_m(gb0, eb0)
        pltpu.async_copy(gb0, accum.at[dv0], semg, add=True)
        drain3(cs1)
        g1 = pltpu.async_copy(h_hbm.at[sv1], gb1, sem)
        g1.wait()
        compute_m(gb1, eb1)
        pltpu.async_copy(gb1, accum.at[dv1], semg, add=True)

    pltpu.make_async_copy(gb0, accum.at[dv0], semg).wait()
    pltpu.make_async_copy(gb1, accum.at[dv1], semg).wait()

    b0 = pl.multiple_of(tbase + (NLB - 1) * LK, 8)
    cs0 = fire3(b0, sv0, dv0, eb0)
    drain3(cs0)
    pltpu.async_copy(h_hbm.at[sv0], gb0, sem).wait()
    compute_m(gb0, eb0)
    pltpu.sync_copy(gb0, accum.at[dv0], add=True)

    plsc.subcore_barrier()
    pltpu.sync_copy(accum.at[pl.ds(sid * 640, 640)],
                    out_hbm.at[cid].at[pl.ds(sid * 640, 640)])


@jax.jit
def _layer_sc(e, h128, src, dst):
    zeros = jnp.zeros((640, 128), jnp.float32)
    mesh = plsc.VectorSubcoreMesh(core_axis_name="c", subcore_axis_name="s",
                                  num_cores=2, num_subcores=16)
    kern = pl.kernel(
        _layer_body,
        out_type=jax.ShapeDtypeStruct((2, NP, 128), jnp.float32),
        mesh=mesh,
        scratch_types=[
            pltpu.VMEM((LK,), jnp.int32),
            pltpu.VMEM((LK,), jnp.int32),
            pltpu.VMEM((LK,), jnp.int32),
            pltpu.VMEM((LK,), jnp.int32),
            pltpu.VMEM((LK, 128), jnp.float32),
            pltpu.VMEM((LK, 128), jnp.float32),
            pltpu.VMEM((LK, 128), jnp.float32),
            pltpu.VMEM((LK, 128), jnp.float32),
            pltpu.VMEM_SHARED((NP, 128), jnp.float32),
            pltpu.SemaphoreType.DMA,
            pltpu.SemaphoreType.DMA,
        ],
        compiler_params=_sc_compiler_params(),
    )
    return kern(e, h128, src.astype(jnp.int32), dst.astype(jnp.int32), zeros)


def _edge_e_body(zp_ref, s2_ref, t2_ref, wle_ref, ble_ref, e_ref):
    z2 = jnp.maximum(zp_ref[...] * s2_ref[...] + t2_ref[...], 0.0)
    e = jnp.dot(z2, wle_ref[...], preferred_element_type=jnp.float32) + ble_ref[...]
    e_ref[...] = jnp.concatenate([e, jnp.zeros_like(e)], axis=1)


def _edge_e(z2pre, s2, t2, wle, ble):
    return pl.pallas_call(
        _edge_e_body,
        grid=(GRID_E,),
        in_specs=[
            pl.BlockSpec((BE, HID), lambda i: (i, 0)),
            pl.BlockSpec((1, HID), lambda i: (0, 0)),
            pl.BlockSpec((1, HID), lambda i: (0, 0)),
            pl.BlockSpec((HID, HID), lambda i: (0, 0)),
            pl.BlockSpec((1, HID), lambda i: (0, 0)),
        ],
        out_specs=pl.BlockSpec((BE, 128), lambda i: (i, 0)),
        out_shape=jax.ShapeDtypeStruct((E, 128), jnp.float32),
    )(z2pre, s2, t2, wle, ble)


def _bn_in(x, g, b):
    m = jnp.mean(x, axis=0, keepdims=True)
    v = jnp.mean(x * x, axis=0, keepdims=True) - m * m
    return (x - m) / jnp.sqrt(v + 1e-5) * g + b


def _mlp_in(h, w1, b1, g1, t1, w2, b2, g2, t2):
    h = jnp.dot(h, w1, preferred_element_type=jnp.float32) + b1
    h = jnp.maximum(_bn_in(h, g1, t1), 0.0)
    h = jnp.dot(h, w2, preferred_element_type=jnp.float32) + b2
    return jnp.maximum(_bn_in(h, g2, t2), 0.0)


def _xemb_body(x_ref, w1, b1, g1, t1, w2, b2, g2, t2, o_ref):
    o_ref[...] = _mlp_in(x_ref[...], w1[...], b1[...], g1[...], t1[...],
                         w2[...], b2[...], g2[...], t2[...])


def _xemb(x, p):
    return pl.pallas_call(
        _xemb_body,
        out_shape=jax.ShapeDtypeStruct((N, HID), jnp.float32),
    )(x, p["xemb_l1_W"], p["xemb_l1_b"][None, :], p["xemb_bn1_g"][None, :],
      p["xemb_bn1_bt"][None, :], p["xemb_l2_W"], p["xemb_l2_b"][None, :],
      p["xemb_bn2_g"][None, :], p["xemb_bn2_bt"][None, :])


def _node_body(parts_ref, h_ref, sc_ref, w1, b1, g1, t1, w2, b2, g2, t2, o_ref):
    agg = (parts_ref[0] + parts_ref[1])[:N, :HID]
    h = agg + sc_ref[...] * h_ref[...]
    o_ref[...] = _mlp_in(h, w1[...], b1[...], g1[...], t1[...],
                         w2[...], b2[...], g2[...], t2[...])


def _node_update(parts, h, eps, p, pre):
    return pl.pallas_call(
        _node_body,
        out_shape=jax.ShapeDtypeStruct((N, HID), jnp.float32),
    )(parts, h, (1.0 + eps).reshape(1, 1),
      p[pre + "_l1_W"], p[pre + "_l1_b"][None, :], p[pre + "_bn1_g"][None, :],
      p[pre + "_bn1_bt"][None, :], p[pre + "_l2_W"], p[pre + "_l2_b"][None, :],
      p[pre + "_bn2_g"][None, :], p[pre + "_bn2_bt"][None, :])


def _head_body(x0, x1, x2, x3, b_ref, w1, b1, g1, t1, w2, b2, o_ref):
    hc = jnp.concatenate([x0[...], x1[...], x2[...], x3[...]], axis=1)
    onehot = (b_ref[...] == lax.broadcasted_iota(jnp.int32, (1, NG), 1)
              ).astype(jnp.float32)
    pooled = lax.dot_general(onehot, hc, (((0,), (0,)), ((), ())),
                             preferred_element_type=jnp.float32)
    cnt = jnp.sum(onehot, axis=0)[:, None]
    pooled = pooled / jnp.maximum(cnt, 1.0)
    o = jnp.dot(pooled, w1[...], preferred_element_type=jnp.float32) + b1[...]
    o = jnp.maximum(_bn_in(o, g1[...], t1[...]), 0.0)
    o = jnp.dot(o, w2[...], preferred_element_type=jnp.float32) + b2[...]
    o_ref[...] = jax.nn.log_softmax(o, axis=-1)


def _head(xs, batch, p):
    return pl.pallas_call(
        _head_body,
        out_shape=jax.ShapeDtypeStruct((NG, NC), jnp.float32),
    )(xs[0], xs[1], xs[2], xs[3], batch.astype(jnp.int32)[:, None],
      p["lin1_W"], p["lin1_b"][None, :], p["bn_lin1_g"][None, :],
      p["bn_lin1_bt"][None, :], p["lin2_W"], p["lin2_b"][None, :])


def kernel(x, edge_index, batch, pos_index, pos_enc, pos_batch, zinit_W, zemb_bn1_g, zemb_bn1_bt, zemb_lin_W, zemb_lin_b, zemb_bn2_g, zemb_bn2_bt, xemb_l1_W, xemb_l1_b, xemb_bn1_g, xemb_bn1_bt, xemb_l2_W, xemb_l2_b, xemb_bn2_g, xemb_bn2_bt, c0_le_W, c0_le_b, c0_eps, c0_l1_W, c0_l1_b, c0_bn1_g, c0_bn1_bt, c0_l2_W, c0_l2_b, c0_bn2_g, c0_bn2_bt, c1_le_W, c1_le_b, c1_eps, c1_l1_W, c1_l1_b, c1_bn1_g, c1_bn1_bt, c1_l2_W, c1_l2_b, c1_bn2_g, c1_bn2_bt, c2_le_W, c2_le_b, c2_eps, c2_l1_W, c2_l1_b, c2_bn1_g, c2_bn1_bt, c2_l2_W, c2_l2_b, c2_bn2_g, c2_bn2_bt, lin1_W, lin1_b, bn_lin1_g, bn_lin1_bt, lin2_W, lin2_b):
    p = dict(locals())
    src = edge_index[0]
    dst = edge_index[1]

    z0 = _z0_sc(zinit_W, pos_index, pos_enc, pos_batch)

    st1 = _col_stats(z0, BE)
    s1, t1 = _bn_coeffs(st1, float(E), zemb_bn1_g, zemb_bn1_bt)
    z2pre, st2 = _ztrans(z0, s1, t1, zemb_lin_W, zemb_lin_b[None, :])
    s2, t2 = _bn_coeffs(st2, float(E), zemb_bn2_g, zemb_bn2_bt)

    xs = [_xemb(x, p)]

    h = jnp.pad(x, ((0, 0), (0, HID - IN_DIM)))
    p["c0_le_W"] = jnp.pad(c0_le_W, ((0, 0), (0, HID - IN_DIM)))
    p["c0_le_b"] = jnp.pad(c0_le_b, (0, HID - IN_DIM))
    p["c0_l1_W"] = jnp.pad(c0_l1_W, ((0, HID - IN_DIM), (0, 0)))
    for i in range(NL):
        pre = "c%d" % i
        h128 = jnp.pad(h, ((0, 0), (0, 128 - HID)))
        e = _edge_e(z2pre, s2, t2, p[pre + "_le_W"], p[pre + "_le_b"][None, :])
        parts = _layer_sc(e, h128, src, dst)
        h = _node_update(parts, h, p[pre + "_eps"], p, pre)
        xs.append(h)

    return _head(xs, batch, p)
